# Initial kernel scaffold; baseline (speedup 1.0000x reference)
import jax
import jax.numpy as jnp
from jax.experimental import pallas as pl


def kernel(xyz, params):
    raise NotImplementedError("write your pallas kernel here")


# speedup vs baseline: 1.4336x; 1.4336x over previous
import functools

import jax
import jax.numpy as jnp
import numpy as np
from jax.experimental import pallas as pl
from jax.experimental.pallas import tpu as pltpu

_BN_S = 1.0 / np.sqrt(1.0 + 1e-5)


def _fold_layers(layers):
    out = []
    for (W, b, g, be) in layers:
        s = g * _BN_S
        out.append((W * s[:, None], (b * s + be)[:, None]))
    return out


def _mlp_body(x_ref, *refs, nlayers, relus, pool):
    out_ref = refs[-1]
    x = x_ref[0]
    for i in range(nlayers):
        W = refs[2 * i][...]
        b = refs[2 * i + 1][...]
        x = jnp.dot(W, x, preferred_element_type=jnp.float32) + b
        if relus[i]:
            x = jnp.maximum(x, 0.0)
    if pool:
        out_ref[0] = jnp.max(x, axis=-1, keepdims=True)
    else:
        out_ref[0] = x


def _mlp(x, layers, relus=None, pool=False):
    B, Cin, S = x.shape
    nlayers = len(layers)
    if relus is None:
        relus = (True,) * nlayers
    Cout = layers[-1][0].shape[0]
    Sout = 1 if pool else S
    wargs = []
    wspecs = []
    for (W, b) in layers:
        wargs += [W, b]
        wspecs += [pl.BlockSpec(W.shape, lambda b_: (0, 0)),
                   pl.BlockSpec(b.shape, lambda b_: (0, 0))]
    return pl.pallas_call(
        functools.partial(_mlp_body, nlayers=nlayers, relus=tuple(relus),
                          pool=pool),
        grid=(B,),
        in_specs=[pl.BlockSpec((1, Cin, S), lambda b_: (b_, 0, 0))] + wspecs,
        out_specs=pl.BlockSpec((1, Cout, Sout), lambda b_: (b_, 0, 0)),
        out_shape=jax.ShapeDtypeStruct((B, Cout, Sout), jnp.float32),
    )(x, *wargs)


def _sa1_body(gx_ref, gy_ref, gz_ref, v_ref, c_ref, *refs, nlayers, K):
    out_ref = refs[-1]
    c3 = c_ref[0]
    cx, cy, cz = c3[0:1], c3[1:2], c3[2:3]
    Cout = out_ref.shape[1]
    S = out_ref.shape[2]

    def body(k, m):
        xk = gx_ref[0, pl.ds(k, 1), :]
        yk = gy_ref[0, pl.ds(k, 1), :]
        zk = gz_ref[0, pl.ds(k, 1), :]
        vk = v_ref[0, pl.ds(k, 1), :]
        x = jnp.concatenate([xk - cx, yk - cy, zk - cz, xk, yk, zk], axis=0)
        for i in range(nlayers):
            W = refs[2 * i][...]
            b = refs[2 * i + 1][...]
            x = jnp.maximum(jnp.dot(W, x, preferred_element_type=jnp.float32)
                            + b, 0.0)
        return jnp.maximum(m, x * vk)

    out_ref[0] = jax.lax.fori_loop(0, K, body, jnp.zeros((Cout, S),
                                                         jnp.float32))


def _sa1_branch(gx, gy, gz, valid, centers, layers):
    B, K, S = gx.shape
    nlayers = len(layers)
    Cout = layers[-1][0].shape[0]
    wargs = []
    wspecs = []
    for (W, b) in layers:
        wargs += [W, b]
        wspecs += [pl.BlockSpec(W.shape, lambda b_: (0, 0)),
                   pl.BlockSpec(b.shape, lambda b_: (0, 0))]
    gspec = pl.BlockSpec((1, K, S), lambda b_: (b_, 0, 0))
    return pl.pallas_call(
        functools.partial(_sa1_body, nlayers=nlayers, K=K),
        grid=(B,),
        in_specs=[gspec, gspec, gspec, gspec,
                  pl.BlockSpec((1, 3, S), lambda b_: (b_, 0, 0))] + wspecs,
        out_specs=pl.BlockSpec((1, Cout, S), lambda b_: (b_, 0, 0)),
        out_shape=jax.ShapeDtypeStruct((B, Cout, S), jnp.float32),
    )(gx, gy, gz, valid, centers, *wargs)


def _sa2_body(a_ref, v_ref, c_ref, w1x_ref, *refs, nlayers, K):
    out_ref = refs[-1]
    corr = jnp.dot(w1x_ref[...], c_ref[0],
                   preferred_element_type=jnp.float32)
    Cout = out_ref.shape[1]
    S = out_ref.shape[2]

    def body(k, m):
        ak = a_ref[0, pl.ds(k, 1)][0]
        vk = v_ref[0, pl.ds(k, 1), :]
        x = jnp.maximum(ak - corr, 0.0)
        for i in range(nlayers):
            W = refs[2 * i][...]
            b = refs[2 * i + 1][...]
            x = jnp.maximum(jnp.dot(W, x, preferred_element_type=jnp.float32)
                            + b, 0.0)
        return jnp.maximum(m, x * vk)

    out_ref[0] = jax.lax.fori_loop(0, K, body, jnp.zeros((Cout, S),
                                                         jnp.float32))


def _sa2_branch(A4, valid, centers, W1x, layers):
    B, K, C1, S = A4.shape
    nlayers = len(layers)
    Cout = layers[-1][0].shape[0]
    wargs = []
    wspecs = []
    for (W, b) in layers:
        wargs += [W, b]
        wspecs += [pl.BlockSpec(W.shape, lambda b_: (0, 0)),
                   pl.BlockSpec(b.shape, lambda b_: (0, 0))]
    return pl.pallas_call(
        functools.partial(_sa2_body, nlayers=nlayers, K=K),
        grid=(B,),
        in_specs=[pl.BlockSpec((1, K, C1, S), lambda b_: (b_, 0, 0, 0)),
                  pl.BlockSpec((1, K, S), lambda b_: (b_, 0, 0)),
                  pl.BlockSpec((1, 3, S), lambda b_: (b_, 0, 0)),
                  pl.BlockSpec(W1x.shape, lambda b_: (0, 0))] + wspecs,
        out_specs=pl.BlockSpec((1, Cout, S), lambda b_: (b_, 0, 0)),
        out_shape=jax.ShapeDtypeStruct((B, Cout, S), jnp.float32),
    )(A4, valid, centers, W1x, *wargs)


def _head_body(x_ref, w1_ref, b1_ref, w2_ref, b2_ref, seg_ref, obj_ref,
               bck_ref):
    x = x_ref[0]
    h = jnp.maximum(jnp.dot(w1_ref[...], x,
                            preferred_element_type=jnp.float32)
                    + b1_ref[...], 0.0)
    z = jnp.dot(w2_ref[...], h, preferred_element_type=jnp.float32) \
        + b2_ref[...]
    s = 1.0 / (1.0 + jnp.exp(-z))
    seg_ref[0] = s
    obj_ref[0] = jnp.max(s * x, axis=-1, keepdims=True)
    bck_ref[0] = jnp.max((1.0 - s) * x, axis=-1, keepdims=True)


def _head(x, W1, b1, W2, b2):
    B, C, S = x.shape
    return pl.pallas_call(
        _head_body,
        grid=(B,),
        in_specs=[pl.BlockSpec((1, C, S), lambda b_: (b_, 0, 0)),
                  pl.BlockSpec(W1.shape, lambda b_: (0, 0)),
                  pl.BlockSpec(b1.shape, lambda b_: (0, 0)),
                  pl.BlockSpec(W2.shape, lambda b_: (0, 0)),
                  pl.BlockSpec(b2.shape, lambda b_: (0, 0))],
        out_specs=[pl.BlockSpec((1, 1, S), lambda b_: (b_, 0, 0)),
                   pl.BlockSpec((1, C, 1), lambda b_: (b_, 0, 0)),
                   pl.BlockSpec((1, C, 1), lambda b_: (b_, 0, 0))],
        out_shape=[jax.ShapeDtypeStruct((B, 1, S), jnp.float32),
                   jax.ShapeDtypeStruct((B, C, 1), jnp.float32),
                   jax.ShapeDtypeStruct((B, C, 1), jnp.float32)],
    )(x, W1, b1, W2, b2)


def _fps_level(x, y, z, S, cx_ref, cy_ref, cz_ref):
    B, N = x.shape
    iota_n = jax.lax.broadcasted_iota(jnp.int32, (B, N), 1)
    iota_s = jax.lax.broadcasted_iota(jnp.int32, (B, S), 1)
    cx_ref[...] = jnp.zeros((B, S), jnp.float32)
    cy_ref[...] = jnp.zeros((B, S), jnp.float32)
    cz_ref[...] = jnp.zeros((B, S), jnp.float32)

    def body(t, carry):
        dist, far = carry
        sel = (iota_n == far).astype(jnp.float32)
        cx = jnp.sum(x * sel, -1, keepdims=True)
        cy = jnp.sum(y * sel, -1, keepdims=True)
        cz = jnp.sum(z * sel, -1, keepdims=True)
        oh = (iota_s == t).astype(jnp.float32)
        cx_ref[...] += cx * oh
        cy_ref[...] += cy * oh
        cz_ref[...] += cz * oh
        d = (x - cx) ** 2 + (y - cy) ** 2 + (z - cz) ** 2
        dist = jnp.minimum(dist, d)
        m = jnp.max(dist, -1, keepdims=True)
        far = jnp.min(jnp.where(dist == m, iota_n, N), -1, keepdims=True)
        return dist, far

    jax.lax.fori_loop(
        0, S, body,
        (jnp.full((B, N), 1e10, jnp.float32),
         jnp.zeros((B, 1), jnp.int32)))


def _fps_body(x_ref, y_ref, z_ref,
              c1x_ref, c1y_ref, c1z_ref, c2x_ref, c2y_ref, c2z_ref):
    _fps_level(x_ref[...], y_ref[...], z_ref[...], 512,
               c1x_ref, c1y_ref, c1z_ref)
    _fps_level(c1x_ref[...], c1y_ref[...], c1z_ref[...], 128,
               c2x_ref, c2y_ref, c2z_ref)


def _fps(xyz):
    B, N, _ = xyz.shape
    x = xyz[:, :, 0]
    y = xyz[:, :, 1]
    z = xyz[:, :, 2]
    full = lambda s: pl.BlockSpec(s, lambda: tuple(0 for _ in s))
    return pl.pallas_call(
        _fps_body,
        in_specs=[full((B, N))] * 3,
        out_specs=[full((B, 512))] * 3 + [full((B, 128))] * 3,
        out_shape=[jax.ShapeDtypeStruct((B, 512), jnp.float32)] * 3
        + [jax.ShapeDtypeStruct((B, 128), jnp.float32)] * 3,
    )(x, y, z)


def _interp3_body(kx_ref, ky_ref, kz_ref, ux_ref, uy_ref, uz_ref, f_ref,
                  out_ref, *, unit_w):
    kx = kx_ref[0]
    ky = ky_ref[0]
    kz = kz_ref[0]
    ux = ux_ref[0]
    uy = uy_ref[0]
    uz = uz_ref[0]
    Sk = kx.shape[0]
    Su = ux.shape[1]
    d2 = (kx - ux) ** 2 + (ky - uy) ** 2 + (kz - uz) ** 2
    iota_k = jax.lax.broadcasted_iota(jnp.int32, (Sk, Su), 0)
    E = jnp.zeros((Sk, Su), jnp.float32)
    ws = []
    idxs = []
    for _ in range(3):
        m = jnp.min(d2, axis=0, keepdims=True)
        i = jnp.min(jnp.where(d2 == m, iota_k, Sk), axis=0, keepdims=True)
        idxs.append(i)
        ws.append(1.0 / (jnp.sqrt(jnp.maximum(m, 0.0)) + 1e-8))
        d2 = jnp.where(iota_k == i, jnp.float32(3.0e38), d2)
    if unit_w:
        for i in idxs:
            E += (iota_k == i).astype(jnp.float32)
    else:
        wsum = ws[0] + ws[1] + ws[2]
        for w, i in zip(ws, idxs):
            E += jnp.where(iota_k == i, w / wsum, 0.0)
    out_ref[0] = jnp.dot(f_ref[0], E, preferred_element_type=jnp.float32)


def _interp3(kx, ky, kz, ux, uy, uz, feats, unit_w=False):
    B, Sk = kx.shape
    Su = ux.shape[1]
    C = feats.shape[1]
    kcol = lambda a: a.reshape(B, Sk, 1)
    urow = lambda a: a.reshape(B, 1, Su)
    return pl.pallas_call(
        functools.partial(_interp3_body, unit_w=unit_w),
        grid=(B,),
        in_specs=[pl.BlockSpec((1, Sk, 1), lambda b_: (b_, 0, 0))] * 3
        + [pl.BlockSpec((1, 1, Su), lambda b_: (b_, 0, 0))] * 3
        + [pl.BlockSpec((1, C, Sk), lambda b_: (b_, 0, 0))],
        out_specs=pl.BlockSpec((1, C, Su), lambda b_: (b_, 0, 0)),
        out_shape=jax.ShapeDtypeStruct((B, C, Su), jnp.float32),
    )(kcol(kx), kcol(ky), kcol(kz), urow(ux), urow(uy), urow(uz), feats)


def _fps_single(pts, npoint):
    N = pts.shape[0]

    def body(carry, _):
        dist, far = carry
        centroid = pts[far]
        d = jnp.sum((pts - centroid) ** 2, axis=-1)
        dist = jnp.minimum(dist, d)
        new_far = jnp.argmax(dist).astype(jnp.int32)
        return (dist, new_far), far

    _, idxs = jax.lax.scan(
        body,
        (jnp.full((N,), 1e10, dtype=pts.dtype), jnp.array(0, jnp.int32)),
        None, length=npoint)
    return idxs


def _ball_query(radius, nsample, xyz, new_xyz):
    N = xyz.shape[1]
    d2 = jnp.sum((new_xyz[:, :, None, :] - xyz[:, None, :, :]) ** 2, axis=-1)
    mask = d2 <= radius * radius
    ar = jnp.arange(N)
    key = jnp.where(mask, ar[None, None, :], N + ar[None, None, :])
    order = jnp.argsort(key, axis=-1)[..., :nsample]
    cnt = jnp.sum(mask, axis=-1, keepdims=True)
    valid = jnp.arange(nsample)[None, None, :] < cnt
    first = order[..., :1]
    return jnp.where(valid, order, first)


def _three_nn(unknown, known):
    d2 = jnp.sum((unknown[:, :, None, :] - known[:, None, :, :]) ** 2,
                 axis=-1)
    neg, idx = jax.lax.top_k(-d2, 3)
    dist = jnp.sqrt(jnp.maximum(-neg, 0.0))
    return dist, idx


def _gather_points(pts, idx):
    return jax.vmap(lambda p, i: p[i])(pts, idx)


def _group_channels(xyz, idx):
    g = _gather_points(xyz, idx)
    g = jnp.transpose(g, (0, 2, 1, 3))
    return g[..., 0], g[..., 1], g[..., 2]


def kernel(xyz, params):
    B, N, _ = xyz.shape
    f32 = jnp.float32

    sa1_layers = [_fold_layers(ls) for ls in params['sa1']]
    sa2_layers = [_fold_layers(ls) for ls in params['sa2']]
    sa3_layers = _fold_layers(params['sa3'])
    fp3_layers = _fold_layers(params['fp3'])
    fp2_layers = _fold_layers(params['fp2'])

    fps1 = jax.vmap(lambda p: _fps_single(p, 512))(xyz)
    l1_xyz = _gather_points(xyz, fps1)
    c1 = jnp.transpose(l1_xyz, (0, 2, 1))
    outs1 = []
    for r, ns, layers in zip([0.1, 0.2, 0.4], [32, 64, 128], sa1_layers):
        idx = _ball_query(r, ns, xyz, l1_xyz)
        gx, gy, gz = _group_channels(xyz, idx)
        valid = jnp.ones((B, ns, 512), f32)
        outs1.append(_sa1_branch(gx, gy, gz, valid, c1, layers))
    l1_points = jnp.concatenate(outs1, axis=1)

    fps2 = jax.vmap(lambda p: _fps_single(p, 128))(l1_xyz)
    l2_xyz = _gather_points(l1_xyz, fps2)
    c2 = jnp.transpose(l2_xyz, (0, 2, 1))
    src2 = jnp.concatenate([c1, l1_points], axis=1)
    outs2 = []
    for r, ns, layers in zip([0.4, 0.8], [64, 128], sa2_layers):
        (W1, b1) = layers[0]
        A = _mlp(src2, [(W1, b1)], relus=(False,))
        idx = _ball_query(r, ns, l1_xyz, l2_xyz)
        Ag = jax.vmap(lambda a, i: a[:, i])(A, idx)
        A4 = jnp.transpose(Ag, (0, 3, 1, 2))
        valid = jnp.ones((B, ns, 128), f32)
        W1x = W1[:, :3]
        outs2.append(_sa2_branch(A4, valid, c2, W1x, layers[1:]))
    l2_points = jnp.concatenate(outs2, axis=1)

    g3 = jnp.concatenate([c2, l2_points], axis=1)
    l3 = _mlp(g3, sa3_layers, pool=True)

    interp3 = jnp.broadcast_to(l3, (B, 1024, 128))
    f3 = jnp.concatenate([interp3, l2_points], axis=1)
    l2f = _mlp(f3, fp3_layers)

    dist, idx = _three_nn(l1_xyz, l2_xyz)
    rec = 1.0 / (dist + 1e-8)
    w = rec / jnp.sum(rec, axis=-1, keepdims=True)
    g = jax.vmap(lambda f, i: f[:, i])(l2f, idx)
    interp2 = jnp.sum(g * w[:, None, :, :], axis=-1)
    f2 = jnp.concatenate([interp2, l1_points], axis=1)
    l1f = _mlp(f2, fp2_layers)

    p = params['conv1']
    s1 = p['g1'] * _BN_S
    W1 = p['W1'] * s1[:, None]
    b1 = (p['b1'] * s1 + p['be1'])[:, None]
    W2 = p['W2']
    b2 = p['b2'][:, None]
    seg, obj, bck = _head(l1f, W1, b1, W2, b2)

    dist2, idx2 = _three_nn(xyz, l1_xyz)
    g = jax.vmap(lambda f, i: f[:, i])(seg, idx2)
    final_seg = jnp.sum(g, axis=-1)

    return (seg, l1f, jnp.squeeze(obj, -1), jnp.squeeze(bck, -1), final_seg)


# speedup vs baseline: 20.1442x; 14.0516x over previous
import functools

import jax
import jax.numpy as jnp
import numpy as np
from jax import lax
from jax.experimental import pallas as pl
from jax.experimental.pallas import tpu as pltpu
from jax.experimental.pallas import tpu_sc as plsc

_BN_S = 1.0 / np.sqrt(1.0 + 1e-5)


def _fold_layers(layers):
    out = []
    for (W, b, g, be) in layers:
        s = g * _BN_S
        out.append((W * s[:, None], (b * s + be)[:, None]))
    return out


def _mlp_body(x_ref, *refs, nlayers, relus, pool):
    out_ref = refs[-1]
    x = x_ref[0]
    for i in range(nlayers):
        W = refs[2 * i][...]
        b = refs[2 * i + 1][...]
        x = jnp.dot(W, x, preferred_element_type=jnp.float32) + b
        if relus[i]:
            x = jnp.maximum(x, 0.0)
    if pool:
        out_ref[0] = jnp.max(x, axis=-1, keepdims=True)
    else:
        out_ref[0] = x


def _mlp(x, layers, relus=None, pool=False):
    B, Cin, S = x.shape
    nlayers = len(layers)
    if relus is None:
        relus = (True,) * nlayers
    Cout = layers[-1][0].shape[0]
    Sout = 1 if pool else S
    wargs = []
    wspecs = []
    for (W, b) in layers:
        wargs += [W, b]
        wspecs += [pl.BlockSpec(W.shape, lambda b_: (0, 0)),
                   pl.BlockSpec(b.shape, lambda b_: (0, 0))]
    return pl.pallas_call(
        functools.partial(_mlp_body, nlayers=nlayers, relus=tuple(relus),
                          pool=pool),
        grid=(B,),
        in_specs=[pl.BlockSpec((1, Cin, S), lambda b_: (b_, 0, 0))] + wspecs,
        out_specs=pl.BlockSpec((1, Cout, Sout), lambda b_: (b_, 0, 0)),
        out_shape=jax.ShapeDtypeStruct((B, Cout, Sout), jnp.float32),
    )(x, *wargs)


def _sa1_body(gx_ref, gy_ref, gz_ref, v_ref, c_ref, *refs, nlayers, K):
    out_ref = refs[-1]
    c3 = c_ref[0]
    cx, cy, cz = c3[0:1], c3[1:2], c3[2:3]
    Cout = out_ref.shape[1]
    S = out_ref.shape[2]

    def body(k, m):
        xk = gx_ref[0, pl.ds(k, 1), :]
        yk = gy_ref[0, pl.ds(k, 1), :]
        zk = gz_ref[0, pl.ds(k, 1), :]
        vk = v_ref[0, pl.ds(k, 1), :]
        x = jnp.concatenate([xk - cx, yk - cy, zk - cz, xk, yk, zk], axis=0)
        for i in range(nlayers):
            W = refs[2 * i][...]
            b = refs[2 * i + 1][...]
            x = jnp.maximum(jnp.dot(W, x, preferred_element_type=jnp.float32)
                            + b, 0.0)
        return jnp.maximum(m, x * vk)

    out_ref[0] = jax.lax.fori_loop(0, K, body, jnp.zeros((Cout, S),
                                                         jnp.float32))


def _sa1_branch(gx, gy, gz, valid, centers, layers):
    B, K, S = gx.shape
    nlayers = len(layers)
    Cout = layers[-1][0].shape[0]
    wargs = []
    wspecs = []
    for (W, b) in layers:
        wargs += [W, b]
        wspecs += [pl.BlockSpec(W.shape, lambda b_: (0, 0)),
                   pl.BlockSpec(b.shape, lambda b_: (0, 0))]
    gspec = pl.BlockSpec((1, K, S), lambda b_: (b_, 0, 0))
    return pl.pallas_call(
        functools.partial(_sa1_body, nlayers=nlayers, K=K),
        grid=(B,),
        in_specs=[gspec, gspec, gspec, gspec,
                  pl.BlockSpec((1, 3, S), lambda b_: (b_, 0, 0))] + wspecs,
        out_specs=pl.BlockSpec((1, Cout, S), lambda b_: (b_, 0, 0)),
        out_shape=jax.ShapeDtypeStruct((B, Cout, S), jnp.float32),
    )(gx, gy, gz, valid, centers, *wargs)


def _head_body(x_ref, w1_ref, b1_ref, w2_ref, b2_ref, seg_ref, obj_ref,
               bck_ref):
    x = x_ref[0]
    h = jnp.maximum(jnp.dot(w1_ref[...], x,
                            preferred_element_type=jnp.float32)
                    + b1_ref[...], 0.0)
    z = jnp.dot(w2_ref[...], h, preferred_element_type=jnp.float32) \
        + b2_ref[...]
    s = 1.0 / (1.0 + jnp.exp(-z))
    seg_ref[0] = s
    obj_ref[0] = jnp.max(s * x, axis=-1, keepdims=True)
    bck_ref[0] = jnp.max((1.0 - s) * x, axis=-1, keepdims=True)


def _head(x, W1, b1, W2, b2):
    B, C, S = x.shape
    return pl.pallas_call(
        _head_body,
        grid=(B,),
        in_specs=[pl.BlockSpec((1, C, S), lambda b_: (b_, 0, 0)),
                  pl.BlockSpec(W1.shape, lambda b_: (0, 0)),
                  pl.BlockSpec(b1.shape, lambda b_: (0, 0)),
                  pl.BlockSpec(W2.shape, lambda b_: (0, 0)),
                  pl.BlockSpec(b2.shape, lambda b_: (0, 0))],
        out_specs=[pl.BlockSpec((1, 1, S), lambda b_: (b_, 0, 0)),
                   pl.BlockSpec((1, C, 1), lambda b_: (b_, 0, 0)),
                   pl.BlockSpec((1, C, 1), lambda b_: (b_, 0, 0))],
        out_shape=[jax.ShapeDtypeStruct((B, 1, S), jnp.float32),
                   jax.ShapeDtypeStruct((B, C, 1), jnp.float32),
                   jax.ShapeDtypeStruct((B, C, 1), jnp.float32)],
    )(x, W1, b1, W2, b2)


def _fps_level(x, y, z, S, cx_ref, cy_ref, cz_ref):
    B, N = x.shape
    iota_n = jax.lax.broadcasted_iota(jnp.int32, (B, N), 1)
    iota_s = jax.lax.broadcasted_iota(jnp.int32, (B, S), 1)
    cx_ref[...] = jnp.zeros((B, S), jnp.float32)
    cy_ref[...] = jnp.zeros((B, S), jnp.float32)
    cz_ref[...] = jnp.zeros((B, S), jnp.float32)

    def body(t, carry):
        dist, far = carry
        sel = (iota_n == far).astype(jnp.float32)
        cx = jnp.sum(x * sel, -1, keepdims=True)
        cy = jnp.sum(y * sel, -1, keepdims=True)
        cz = jnp.sum(z * sel, -1, keepdims=True)
        oh = (iota_s == t).astype(jnp.float32)
        cx_ref[...] += cx * oh
        cy_ref[...] += cy * oh
        cz_ref[...] += cz * oh
        d = (x - cx) ** 2 + (y - cy) ** 2 + (z - cz) ** 2
        dist = jnp.minimum(dist, d)
        m = jnp.max(dist, -1, keepdims=True)
        far = jnp.min(jnp.where(dist == m, iota_n, N), -1, keepdims=True)
        return dist, far

    jax.lax.fori_loop(
        0, S, body,
        (jnp.full((B, N), 1e10, jnp.float32),
         jnp.zeros((B, 1), jnp.int32)))


def _fps_body(x_ref, y_ref, z_ref,
              c1x_ref, c1y_ref, c1z_ref, c2x_ref, c2y_ref, c2z_ref):
    _fps_level(x_ref[...], y_ref[...], z_ref[...], 512,
               c1x_ref, c1y_ref, c1z_ref)
    _fps_level(c1x_ref[...], c1y_ref[...], c1z_ref[...], 128,
               c2x_ref, c2y_ref, c2z_ref)


def _fps(xyz):
    B, N, _ = xyz.shape
    x = xyz[:, :, 0]
    y = xyz[:, :, 1]
    z = xyz[:, :, 2]
    full = lambda s: pl.BlockSpec(s, lambda: tuple(0 for _ in s))
    return pl.pallas_call(
        _fps_body,
        in_specs=[full((B, N))] * 3,
        out_specs=[full((B, 512))] * 3 + [full((B, 128))] * 3,
        out_shape=[jax.ShapeDtypeStruct((B, 512), jnp.float32)] * 3
        + [jax.ShapeDtypeStruct((B, 128), jnp.float32)] * 3,
    )(x, y, z)


def _interp3_body(kx_ref, ky_ref, kz_ref, ux_ref, uy_ref, uz_ref, f_ref,
                  out_ref, *, unit_w):
    kx = kx_ref[0]
    ky = ky_ref[0]
    kz = kz_ref[0]
    ux = ux_ref[0]
    uy = uy_ref[0]
    uz = uz_ref[0]
    Sk = kx.shape[0]
    Su = ux.shape[1]
    d2 = (kx - ux) ** 2 + (ky - uy) ** 2 + (kz - uz) ** 2
    iota_k = jax.lax.broadcasted_iota(jnp.int32, (Sk, Su), 0)
    E = jnp.zeros((Sk, Su), jnp.float32)
    ws = []
    idxs = []
    for _ in range(3):
        m = jnp.min(d2, axis=0, keepdims=True)
        i = jnp.min(jnp.where(d2 == m, iota_k, Sk), axis=0, keepdims=True)
        idxs.append(i)
        ws.append(1.0 / (jnp.sqrt(jnp.maximum(m, 0.0)) + 1e-8))
        d2 = jnp.where(iota_k == i, jnp.float32(3.0e38), d2)
    if unit_w:
        for i in idxs:
            E += (iota_k == i).astype(jnp.float32)
    else:
        wsum = ws[0] + ws[1] + ws[2]
        for w, i in zip(ws, idxs):
            E += jnp.where(iota_k == i, w / wsum, 0.0)
    out_ref[0] = jnp.dot(f_ref[0], E, preferred_element_type=jnp.float32)


def _interp3(kx, ky, kz, ux, uy, uz, feats, unit_w=False):
    B, Sk = kx.shape
    Su = ux.shape[1]
    C = feats.shape[1]
    kcol = lambda a: a.reshape(B, Sk, 1)
    urow = lambda a: a.reshape(B, 1, Su)
    return pl.pallas_call(
        functools.partial(_interp3_body, unit_w=unit_w),
        grid=(B,),
        in_specs=[pl.BlockSpec((1, Sk, 1), lambda b_: (b_, 0, 0))] * 3
        + [pl.BlockSpec((1, 1, Su), lambda b_: (b_, 0, 0))] * 3
        + [pl.BlockSpec((1, C, Sk), lambda b_: (b_, 0, 0))],
        out_specs=pl.BlockSpec((1, C, Su), lambda b_: (b_, 0, 0)),
        out_shape=jax.ShapeDtypeStruct((B, C, Su), jnp.float32),
    )(kcol(kx), kcol(ky), kcol(kz), urow(ux), urow(uy), urow(uz), feats)


_SA1_RAD = ((0.1, 32, 48), (0.2, 64, 80), (0.4, 128, 144))


def _bq_sa1(x, y, z, cx, cy, cz):
    B, N = x.shape
    S = cx.shape[1]
    NW = 32
    RPW = B * S // NW
    WPB = NW // B
    SPW = S // WPB
    NJ = N // 16
    mesh = plsc.VectorSubcoreMesh(core_axis_name="c", subcore_axis_name="s")
    out_type = []
    for (_, ns, kp) in _SA1_RAD:
        out_type += [jax.ShapeDtypeStruct((B * S * kp,), jnp.float32)] * 3
        out_type += [jax.ShapeDtypeStruct((B * S,), jnp.int32)]
    scratch = ([pltpu.VMEM((N,), jnp.float32)] * 3
               + [pltpu.VMEM((RPW,), jnp.float32)] * 3)
    for (_, ns, kp) in _SA1_RAD:
        scratch += [pltpu.VMEM((16 * kp,), jnp.float32)] * 3
        scratch += [pltpu.VMEM((16,), jnp.int32)]

    @functools.partial(
        pl.kernel, mesh=mesh, out_type=out_type, scratch_types=scratch,
        compiler_params=pltpu.CompilerParams(needs_layout_passes=False))
    def k(xh, yh, zh, cxh, cyh, czh, *refs):
        outs = refs[:12]
        xv, yv, zv, ccx, ccy, ccz = refs[12:18]
        bufs = refs[18:]
        wid = lax.axis_index("s") * 2 + lax.axis_index("c")
        b = wid // WPB
        s0 = pl.multiple_of((wid % WPB) * SPW, SPW)
        pltpu.sync_copy(xh.at[b], xv)
        pltpu.sync_copy(yh.at[b], yv)
        pltpu.sync_copy(zh.at[b], zv)
        pltpu.sync_copy(cxh.at[b, pl.ds(s0, SPW)], ccx)
        pltpu.sync_copy(cyh.at[b, pl.ds(s0, SPW)], ccy)
        pltpu.sync_copy(czh.at[b, pl.ds(s0, SPW)], ccz)
        lane = lax.iota(jnp.int32, 16)

        def group_body(g, _):
            g16 = pl.multiple_of(g * 16, 16)
            cxg = ccx[pl.ds(g16, 16)]
            cyg = ccy[pl.ds(g16, 16)]
            czg = ccz[pl.ds(g16, 16)]
            cntv = [jnp.zeros((16,), jnp.int32) for _ in _SA1_RAD]
            for ic in range(16):
                cxb = jnp.full((16,), cxg[ic], jnp.float32)
                cyb = jnp.full((16,), cyg[ic], jnp.float32)
                czb = jnp.full((16,), czg[ic], jnp.float32)

                def pt_body(j, cnts, ic=ic, cxb=cxb, cyb=cyb, czb=czb):
                    px = xv[pl.ds(j * 16, 16)]
                    py = yv[pl.ds(j * 16, 16)]
                    pz = zv[pl.ds(j * 16, 16)]
                    dx = px - cxb
                    dy = py - cyb
                    dz = pz - czb
                    d2 = dx * dx + dy * dy + dz * dz
                    new = []
                    for ri, (r, ns, kp) in enumerate(_SA1_RAD):
                        cnt = cnts[ri]
                        m = d2 <= r * r
                        mi = m.astype(jnp.int32)
                        incl = jnp.cumsum(mi)
                        tgt = (incl - mi) + (ic * kp + jnp.minimum(cnt, ns))
                        plsc.store_scatter(bufs[4 * ri + 0], [tgt], px,
                                           mask=m)
                        plsc.store_scatter(bufs[4 * ri + 1], [tgt], py,
                                           mask=m)
                        plsc.store_scatter(bufs[4 * ri + 2], [tgt], pz,
                                           mask=m)
                        new.append(cnt + incl[15])
                    return tuple(new)

                cnts = lax.fori_loop(0, NJ, pt_body, (0, 0, 0))
                for ri, (r, ns, kp) in enumerate(_SA1_RAD):
                    cntv[ri] = cntv[ri] + jnp.where(
                        lane == ic,
                        jnp.full((16,), jnp.minimum(cnts[ri], ns),
                                 jnp.int32),
                        jnp.zeros((16,), jnp.int32))

            row0 = pl.multiple_of(wid * RPW + g16, 16)
            for ri, (r, ns, kp) in enumerate(_SA1_RAD):
                cb = bufs[4 * ri + 3]
                cb[...] = cntv[ri]
                for ch in range(3):
                    pltpu.sync_copy(
                        bufs[4 * ri + ch],
                        outs[4 * ri + ch].at[
                            pl.ds(pl.multiple_of(row0 * kp, 16 * kp),
                                  16 * kp)])
                pltpu.sync_copy(cb, outs[4 * ri + 3].at[pl.ds(row0, 16)])
            return 0

        lax.fori_loop(0, RPW // 16, group_body, 0)

    res = k(x, y, z, cx, cy, cz)
    groups = []
    for ri, (r, ns, kp) in enumerate(_SA1_RAD):
        ox, oy, oz, cnt = res[4 * ri:4 * ri + 4]
        groups.append((ox.reshape(B, S, kp), oy.reshape(B, S, kp),
                       oz.reshape(B, S, kp), cnt.reshape(B, 1, S)))
    return groups


_SA2_RAD = ((0.4, 64, 80), (0.8, 128, 144))


def _bq_sa2(x, y, z, cx, cy, cz):
    B, N = x.shape
    S = cx.shape[1]
    NW = 32
    RPW = B * S // NW
    WPB = NW // B
    SPW = S // WPB
    NJ = N // 16
    mesh = plsc.VectorSubcoreMesh(core_axis_name="c", subcore_axis_name="s")
    out_type = []
    for (_, ns, kp) in _SA2_RAD:
        out_type += [jax.ShapeDtypeStruct((B * S * kp,), jnp.int32),
                     jax.ShapeDtypeStruct((B * S,), jnp.int32)]
    scratch = ([pltpu.VMEM((N,), jnp.float32)] * 3
               + [pltpu.VMEM((SPW,), jnp.float32)] * 3)
    for (_, ns, kp) in _SA2_RAD:
        scratch += [pltpu.VMEM((16 * kp,), jnp.int32),
                    pltpu.VMEM((16,), jnp.int32)]

    @functools.partial(
        pl.kernel, mesh=mesh, out_type=out_type, scratch_types=scratch,
        compiler_params=pltpu.CompilerParams(needs_layout_passes=False))
    def k(xh, yh, zh, cxh, cyh, czh, *refs):
        outs = refs[:4]
        xv, yv, zv, ccx, ccy, ccz = refs[4:10]
        bufs = refs[10:]
        wid = lax.axis_index("s") * 2 + lax.axis_index("c")
        b = wid // WPB
        s0 = pl.multiple_of((wid % WPB) * SPW, SPW)
        pltpu.sync_copy(xh.at[b], xv)
        pltpu.sync_copy(yh.at[b], yv)
        pltpu.sync_copy(zh.at[b], zv)
        pltpu.sync_copy(cxh.at[b, pl.ds(s0, SPW)], ccx)
        pltpu.sync_copy(cyh.at[b, pl.ds(s0, SPW)], ccy)
        pltpu.sync_copy(czh.at[b, pl.ds(s0, SPW)], ccz)
        lane = lax.iota(jnp.int32, 16)

        def group_body(g, _):
            g16 = pl.multiple_of(g * 16, 16)
            cxg = ccx[pl.ds(g16, 16)]
            cyg = ccy[pl.ds(g16, 16)]
            czg = ccz[pl.ds(g16, 16)]
            cntv = [jnp.zeros((16,), jnp.int32) for _ in _SA2_RAD]
            for ic in range(16):
                cxb = jnp.full((16,), cxg[ic], jnp.float32)
                cyb = jnp.full((16,), cyg[ic], jnp.float32)
                czb = jnp.full((16,), czg[ic], jnp.float32)

                def pt_body(j, cnts, ic=ic, cxb=cxb, cyb=cyb, czb=czb):
                    px = xv[pl.ds(j * 16, 16)]
                    py = yv[pl.ds(j * 16, 16)]
                    pz = zv[pl.ds(j * 16, 16)]
                    dx = px - cxb
                    dy = py - cyb
                    dz = pz - czb
                    d2 = dx * dx + dy * dy + dz * dz
                    pidx = j * 16 + lane
                    new = []
                    for ri, (r, ns, kp) in enumerate(_SA2_RAD):
                        cnt = cnts[ri]
                        m = d2 <= r * r
                        mi = m.astype(jnp.int32)
                        incl = jnp.cumsum(mi)
                        tgt = (incl - mi) + (ic * kp
                                             + jnp.minimum(cnt, ns))
                        plsc.store_scatter(bufs[2 * ri], [tgt], pidx,
                                           mask=m)
                        new.append(cnt + incl[15])
                    return tuple(new)

                cnts = lax.fori_loop(0, NJ, pt_body, (0, 0))
                for ri, (r, ns, kp) in enumerate(_SA2_RAD):
                    cntv[ri] = cntv[ri] + jnp.where(
                        lane == ic,
                        jnp.full((16,), jnp.minimum(cnts[ri], ns),
                                 jnp.int32),
                        jnp.zeros((16,), jnp.int32))

            row0 = pl.multiple_of(wid * RPW + g16, 16)
            for ri, (r, ns, kp) in enumerate(_SA2_RAD):
                cb = bufs[2 * ri + 1]
                cb[...] = cntv[ri]
                pltpu.sync_copy(
                    bufs[2 * ri],
                    outs[2 * ri].at[
                        pl.ds(pl.multiple_of(row0 * kp, 16 * kp), 16 * kp)])
                pltpu.sync_copy(cb, outs[2 * ri + 1].at[pl.ds(row0, 16)])
            return 0

        lax.fori_loop(0, RPW // 16, group_body, 0)

    res = k(x, y, z, cx, cy, cz)
    groups = []
    for ri, (r, ns, kp) in enumerate(_SA2_RAD):
        idx, cnt = res[2 * ri:2 * ri + 2]
        groups.append((idx.reshape(B, S, kp), cnt.reshape(B, S)))
    return groups


def _sc_gather(A2, idx, cnt, ns, Np):
    BNp, C = A2.shape
    B, S, KP = idx.shape
    NW = 32
    WPB = 4
    KPW = ns // WPB
    NJ = S // 16
    mesh = plsc.VectorSubcoreMesh(core_axis_name="c", subcore_axis_name="s")
    idx_f = idx.reshape(B, S * KP)
    out_type = jax.ShapeDtypeStruct((B, ns, S, C), jnp.float32)
    scratch = [pltpu.VMEM((S * KP,), jnp.int32),
               pltpu.VMEM((S,), jnp.int32),
               pltpu.VMEM((S,), jnp.int32),
               pltpu.VMEM((S, C), jnp.float32),
               pltpu.SemaphoreType.DMA]

    @functools.partial(
        pl.kernel, mesh=mesh, out_type=out_type, scratch_types=scratch,
        compiler_params=pltpu.CompilerParams(needs_layout_passes=False))
    def k(ah, ih, ch, oh, iv, cv, gidx, rows, sem):
        wid = lax.axis_index("s") * 2 + lax.axis_index("c")
        b = wid // WPB
        k0 = (wid % WPB) * KPW
        pltpu.sync_copy(ih.at[b], iv)
        pltpu.sync_copy(ch.at[b], cv)
        lane = lax.iota(jnp.int32, 16)
        base = b * Np

        def k_body(kk, _):
            kq = k0 + kk
            for jj in range(NJ):
                offs = (jj * 16 + lane) * KP + kq
                raw = plsc.load_gather(iv, [offs])
                cchunk = cv[pl.ds(jj * 16, 16)]
                safe = jnp.minimum(jnp.maximum(raw, 0), Np - 1)
                sel = jnp.where(kq < cchunk, safe, 0) + base
                gidx[pl.ds(jj * 16, 16)] = sel
            pltpu.async_copy(ah.at[gidx], rows, sem).wait()
            pltpu.sync_copy(rows, oh.at[b, kq])
            return 0

        lax.fori_loop(0, KPW, k_body, 0)

    return k(A2, idx_f, cnt)


def _sa2s_body(a_ref, cnt_ref, c_ref, w1xt_ref, *refs, nlayers, K):
    out_ref = refs[-1]
    corr = jnp.dot(c_ref[0], w1xt_ref[...],
                   preferred_element_type=jnp.float32)
    cntcol = cnt_ref[0]
    S = out_ref.shape[2]
    Cout = out_ref.shape[1]

    def body(kq, m):
        ak = a_ref[0, pl.ds(kq, 1)][0]
        vk = (cntcol > kq).astype(jnp.float32)
        x = jnp.maximum(ak - corr, 0.0)
        for i in range(nlayers):
            Wt = refs[2 * i][...]
            bt = refs[2 * i + 1][...]
            x = jnp.maximum(jnp.dot(x, Wt,
                                    preferred_element_type=jnp.float32)
                            + bt, 0.0)
        return jnp.maximum(m, x * vk)

    m = jax.lax.fori_loop(0, K, body, jnp.zeros((S, Cout), jnp.float32))
    out_ref[0] = jnp.transpose(m)


def _sa2_branch_s(A4, cnt, centers_t, W1x, layers):
    B, K, S, C1 = A4.shape
    nlayers = len(layers)
    Cout = layers[-1][0].shape[0]
    wargs = []
    wspecs = []
    for (W, b) in layers:
        Wt = jnp.transpose(W)
        bt = jnp.transpose(b)
        wargs += [Wt, bt]
        wspecs += [pl.BlockSpec(Wt.shape, lambda b_: (0, 0)),
                   pl.BlockSpec(bt.shape, lambda b_: (0, 0))]
    W1xt = jnp.transpose(W1x)
    return pl.pallas_call(
        functools.partial(_sa2s_body, nlayers=nlayers, K=K),
        grid=(B,),
        in_specs=[pl.BlockSpec((1, K, S, C1), lambda b_: (b_, 0, 0, 0)),
                  pl.BlockSpec((1, S, 1), lambda b_: (b_, 0, 0)),
                  pl.BlockSpec((1, S, 3), lambda b_: (b_, 0, 0)),
                  pl.BlockSpec(W1xt.shape, lambda b_: (0, 0))] + wspecs,
        out_specs=pl.BlockSpec((1, Cout, S), lambda b_: (b_, 0, 0)),
        out_shape=jax.ShapeDtypeStruct((B, Cout, S), jnp.float32),
    )(A4, cnt, centers_t, W1xt, *wargs)


def _bq_fix_body(ox_ref, oy_ref, oz_ref, cnt_ref,
                 gx_ref, gy_ref, gz_ref, v_ref, *, K):
    S = cnt_ref.shape[2]
    cnt = cnt_ref[0]
    iota_k = lax.broadcasted_iota(jnp.int32, (K, S), 0)
    vm = iota_k < cnt
    for src, dst in ((ox_ref, gx_ref), (oy_ref, gy_ref), (oz_ref, gz_ref)):
        t = jnp.transpose(src[0])[:K]
        dst[0] = jnp.where(vm, t, 0.0)
    v_ref[0] = vm.astype(jnp.float32)


def _bq_fix(ox, oy, oz, cnt, K):
    B, S, KP = ox.shape
    ospec = pl.BlockSpec((1, S, KP), lambda b_: (b_, 0, 0))
    gspec = pl.BlockSpec((1, K, S), lambda b_: (b_, 0, 0))
    return pl.pallas_call(
        functools.partial(_bq_fix_body, K=K),
        grid=(B,),
        in_specs=[ospec] * 3 + [pl.BlockSpec((1, 1, S), lambda b_: (b_, 0, 0))],
        out_specs=[gspec] * 4,
        out_shape=[jax.ShapeDtypeStruct((B, K, S), jnp.float32)] * 4,
    )(ox, oy, oz, cnt)


def kernel(xyz, params):
    B, N, _ = xyz.shape

    sa1_layers = [_fold_layers(ls) for ls in params['sa1']]
    sa2_layers = [_fold_layers(ls) for ls in params['sa2']]
    sa3_layers = _fold_layers(params['sa3'])
    fp3_layers = _fold_layers(params['fp3'])
    fp2_layers = _fold_layers(params['fp2'])

    c1x, c1y, c1z, c2x, c2y, c2z = _fps(xyz)
    c1 = jnp.stack([c1x, c1y, c1z], axis=1)
    c2 = jnp.stack([c2x, c2y, c2z], axis=1)

    groups = _bq_sa1(xyz[:, :, 0], xyz[:, :, 1], xyz[:, :, 2],
                     c1x, c1y, c1z)
    outs1 = []
    for (ox, oy, oz, cnt), (r, ns, kp), layers in zip(groups, _SA1_RAD,
                                                      sa1_layers):
        gx, gy, gz, valid = _bq_fix(ox, oy, oz, cnt, ns)
        outs1.append(_sa1_branch(gx, gy, gz, valid, c1, layers))
    l1_points = jnp.concatenate(outs1, axis=1)

    src2 = jnp.concatenate([c1, l1_points], axis=1)
    groups2 = _bq_sa2(c1x, c1y, c1z, c2x, c2y, c2z)
    c2t = jnp.transpose(c2, (0, 2, 1))
    outs2 = []
    for (idx, cnt), (r, ns, kp), layers in zip(groups2, _SA2_RAD,
                                               sa2_layers):
        (W1, b1) = layers[0]
        A = _mlp(src2, [(W1, b1)], relus=(False,))
        A2 = jnp.transpose(A, (0, 2, 1)).reshape(B * 512, 128)
        A4 = _sc_gather(A2, idx, cnt, ns, 512)
        outs2.append(_sa2_branch_s(A4, cnt.reshape(B, 128, 1), c2t,
                                   W1[:, :3], layers[1:]))
    l2_points = jnp.concatenate(outs2, axis=1)

    g3 = jnp.concatenate([c2, l2_points], axis=1)
    l3 = _mlp(g3, sa3_layers, pool=True)

    interp3 = jnp.broadcast_to(l3, (B, 1024, 128))
    f3 = jnp.concatenate([interp3, l2_points], axis=1)
    l2f = _mlp(f3, fp3_layers)

    interp2 = _interp3(c2x, c2y, c2z, c1x, c1y, c1z, l2f)
    f2 = jnp.concatenate([interp2, l1_points], axis=1)
    l1f = _mlp(f2, fp2_layers)

    p = params['conv1']
    s1 = p['g1'] * _BN_S
    W1 = p['W1'] * s1[:, None]
    b1 = (p['b1'] * s1 + p['be1'])[:, None]
    W2 = p['W2']
    b2 = p['b2'][:, None]
    seg, obj, bck = _head(l1f, W1, b1, W2, b2)

    final_seg = _interp3(c1x, c1y, c1z,
                         xyz[:, :, 0], xyz[:, :, 1], xyz[:, :, 2],
                         seg, unit_w=True)

    return (seg, l1f, jnp.squeeze(obj, -1), jnp.squeeze(bck, -1), final_seg)


# speedup vs baseline: 20.3984x; 1.0126x over previous
import functools

import jax
import jax.numpy as jnp
import numpy as np
from jax import lax
from jax.experimental import pallas as pl
from jax.experimental.pallas import tpu as pltpu
from jax.experimental.pallas import tpu_sc as plsc

_BN_S = 1.0 / np.sqrt(1.0 + 1e-5)


def _fold_layers(layers):
    out = []
    for (W, b, g, be) in layers:
        s = g * _BN_S
        out.append((W * s[:, None], (b * s + be)[:, None]))
    return out


def _mlp_body(x_ref, *refs, nlayers, relus, pool):
    out_ref = refs[-1]
    x = x_ref[0]
    for i in range(nlayers):
        W = refs[2 * i][...]
        b = refs[2 * i + 1][...]
        x = jnp.dot(W, x, preferred_element_type=jnp.float32) + b
        if relus[i]:
            x = jnp.maximum(x, 0.0)
    if pool:
        out_ref[0] = jnp.max(x, axis=-1, keepdims=True)
    else:
        out_ref[0] = x


def _mlp(x, layers, relus=None, pool=False):
    B, Cin, S = x.shape
    nlayers = len(layers)
    if relus is None:
        relus = (True,) * nlayers
    Cout = layers[-1][0].shape[0]
    Sout = 1 if pool else S
    wargs = []
    wspecs = []
    for (W, b) in layers:
        wargs += [W, b]
        wspecs += [pl.BlockSpec(W.shape, lambda b_: (0, 0)),
                   pl.BlockSpec(b.shape, lambda b_: (0, 0))]
    return pl.pallas_call(
        functools.partial(_mlp_body, nlayers=nlayers, relus=tuple(relus),
                          pool=pool),
        grid=(B,),
        in_specs=[pl.BlockSpec((1, Cin, S), lambda b_: (b_, 0, 0))] + wspecs,
        out_specs=pl.BlockSpec((1, Cout, Sout), lambda b_: (b_, 0, 0)),
        out_shape=jax.ShapeDtypeStruct((B, Cout, Sout), jnp.float32),
    )(x, *wargs)


def _sa1_body(gx_ref, gy_ref, gz_ref, v_ref, c_ref, *refs, nlayers, K):
    out_ref = refs[-1]
    c3 = c_ref[0]
    cx, cy, cz = c3[0:1], c3[1:2], c3[2:3]
    Cout = out_ref.shape[1]
    S = out_ref.shape[2]

    def body(k, m):
        xk = gx_ref[0, pl.ds(k, 1), :]
        yk = gy_ref[0, pl.ds(k, 1), :]
        zk = gz_ref[0, pl.ds(k, 1), :]
        vk = v_ref[0, pl.ds(k, 1), :]
        x = jnp.concatenate([xk - cx, yk - cy, zk - cz, xk, yk, zk], axis=0)
        for i in range(nlayers):
            W = refs[2 * i][...]
            b = refs[2 * i + 1][...]
            x = jnp.maximum(jnp.dot(W, x, preferred_element_type=jnp.float32)
                            + b, 0.0)
        return jnp.maximum(m, x * vk)

    out_ref[0] = jax.lax.fori_loop(0, K, body, jnp.zeros((Cout, S),
                                                         jnp.float32))


def _sa1_branch(gx, gy, gz, valid, centers, layers):
    B, K, S = gx.shape
    nlayers = len(layers)
    Cout = layers[-1][0].shape[0]
    wargs = []
    wspecs = []
    for (W, b) in layers:
        wargs += [W, b]
        wspecs += [pl.BlockSpec(W.shape, lambda b_: (0, 0)),
                   pl.BlockSpec(b.shape, lambda b_: (0, 0))]
    gspec = pl.BlockSpec((1, K, S), lambda b_: (b_, 0, 0))
    return pl.pallas_call(
        functools.partial(_sa1_body, nlayers=nlayers, K=K),
        grid=(B,),
        in_specs=[gspec, gspec, gspec, gspec,
                  pl.BlockSpec((1, 3, S), lambda b_: (b_, 0, 0))] + wspecs,
        out_specs=pl.BlockSpec((1, Cout, S), lambda b_: (b_, 0, 0)),
        out_shape=jax.ShapeDtypeStruct((B, Cout, S), jnp.float32),
    )(gx, gy, gz, valid, centers, *wargs)


def _head_body(x_ref, w1_ref, b1_ref, w2_ref, b2_ref, seg_ref, obj_ref,
               bck_ref):
    x = x_ref[0]
    h = jnp.maximum(jnp.dot(w1_ref[...], x,
                            preferred_element_type=jnp.float32)
                    + b1_ref[...], 0.0)
    z = jnp.dot(w2_ref[...], h, preferred_element_type=jnp.float32) \
        + b2_ref[...]
    s = 1.0 / (1.0 + jnp.exp(-z))
    seg_ref[0] = s
    obj_ref[0] = jnp.max(s * x, axis=-1, keepdims=True)
    bck_ref[0] = jnp.max((1.0 - s) * x, axis=-1, keepdims=True)


def _head(x, W1, b1, W2, b2):
    B, C, S = x.shape
    return pl.pallas_call(
        _head_body,
        grid=(B,),
        in_specs=[pl.BlockSpec((1, C, S), lambda b_: (b_, 0, 0)),
                  pl.BlockSpec(W1.shape, lambda b_: (0, 0)),
                  pl.BlockSpec(b1.shape, lambda b_: (0, 0)),
                  pl.BlockSpec(W2.shape, lambda b_: (0, 0)),
                  pl.BlockSpec(b2.shape, lambda b_: (0, 0))],
        out_specs=[pl.BlockSpec((1, 1, S), lambda b_: (b_, 0, 0)),
                   pl.BlockSpec((1, C, 1), lambda b_: (b_, 0, 0)),
                   pl.BlockSpec((1, C, 1), lambda b_: (b_, 0, 0))],
        out_shape=[jax.ShapeDtypeStruct((B, 1, S), jnp.float32),
                   jax.ShapeDtypeStruct((B, C, 1), jnp.float32),
                   jax.ShapeDtypeStruct((B, C, 1), jnp.float32)],
    )(x, W1, b1, W2, b2)


def _fps_level(x, y, z, S, cx_ref, cy_ref, cz_ref):
    B, N = x.shape
    iota_n = jax.lax.broadcasted_iota(jnp.int32, (B, N), 1)
    iota_s = jax.lax.broadcasted_iota(jnp.int32, (B, S), 1)
    cx_ref[...] = jnp.zeros((B, S), jnp.float32)
    cy_ref[...] = jnp.zeros((B, S), jnp.float32)
    cz_ref[...] = jnp.zeros((B, S), jnp.float32)

    def body(t, carry):
        dist, far = carry
        sel = (iota_n == far).astype(jnp.float32)
        cx = jnp.sum(x * sel, -1, keepdims=True)
        cy = jnp.sum(y * sel, -1, keepdims=True)
        cz = jnp.sum(z * sel, -1, keepdims=True)
        oh = (iota_s == t).astype(jnp.float32)
        cx_ref[...] += cx * oh
        cy_ref[...] += cy * oh
        cz_ref[...] += cz * oh
        d = (x - cx) ** 2 + (y - cy) ** 2 + (z - cz) ** 2
        dist = jnp.minimum(dist, d)
        m = jnp.max(dist, -1, keepdims=True)
        far = jnp.min(jnp.where(dist == m, iota_n, N), -1, keepdims=True)
        return dist, far

    jax.lax.fori_loop(
        0, S, body,
        (jnp.full((B, N), 1e10, jnp.float32),
         jnp.zeros((B, 1), jnp.int32)))


def _fps_body(x_ref, y_ref, z_ref,
              c1x_ref, c1y_ref, c1z_ref, c2x_ref, c2y_ref, c2z_ref):
    _fps_level(x_ref[...], y_ref[...], z_ref[...], 512,
               c1x_ref, c1y_ref, c1z_ref)
    _fps_level(c1x_ref[...], c1y_ref[...], c1z_ref[...], 128,
               c2x_ref, c2y_ref, c2z_ref)


def _fps(xyz):
    B, N, _ = xyz.shape
    x = xyz[:, :, 0]
    y = xyz[:, :, 1]
    z = xyz[:, :, 2]
    full = lambda s: pl.BlockSpec(s, lambda: tuple(0 for _ in s))
    return pl.pallas_call(
        _fps_body,
        in_specs=[full((B, N))] * 3,
        out_specs=[full((B, 512))] * 3 + [full((B, 128))] * 3,
        out_shape=[jax.ShapeDtypeStruct((B, 512), jnp.float32)] * 3
        + [jax.ShapeDtypeStruct((B, 128), jnp.float32)] * 3,
    )(x, y, z)


def _interp3_body(kx_ref, ky_ref, kz_ref, ux_ref, uy_ref, uz_ref, f_ref,
                  out_ref, *, unit_w):
    kx = kx_ref[0]
    ky = ky_ref[0]
    kz = kz_ref[0]
    ux = ux_ref[0]
    uy = uy_ref[0]
    uz = uz_ref[0]
    Sk = kx.shape[0]
    Su = ux.shape[1]
    d2 = (kx - ux) ** 2 + (ky - uy) ** 2 + (kz - uz) ** 2
    iota_k = jax.lax.broadcasted_iota(jnp.int32, (Sk, Su), 0)
    E = jnp.zeros((Sk, Su), jnp.float32)
    ws = []
    idxs = []
    for _ in range(3):
        m = jnp.min(d2, axis=0, keepdims=True)
        i = jnp.min(jnp.where(d2 == m, iota_k, Sk), axis=0, keepdims=True)
        idxs.append(i)
        ws.append(1.0 / (jnp.sqrt(jnp.maximum(m, 0.0)) + 1e-8))
        d2 = jnp.where(iota_k == i, jnp.float32(3.0e38), d2)
    if unit_w:
        for i in idxs:
            E += (iota_k == i).astype(jnp.float32)
    else:
        wsum = ws[0] + ws[1] + ws[2]
        for w, i in zip(ws, idxs):
            E += jnp.where(iota_k == i, w / wsum, 0.0)
    out_ref[0] = jnp.dot(f_ref[0], E, preferred_element_type=jnp.float32)


def _interp3(kx, ky, kz, ux, uy, uz, feats, unit_w=False):
    B, Sk = kx.shape
    Su = ux.shape[1]
    C = feats.shape[1]
    kcol = lambda a: a.reshape(B, Sk, 1)
    urow = lambda a: a.reshape(B, 1, Su)
    return pl.pallas_call(
        functools.partial(_interp3_body, unit_w=unit_w),
        grid=(B,),
        in_specs=[pl.BlockSpec((1, Sk, 1), lambda b_: (b_, 0, 0))] * 3
        + [pl.BlockSpec((1, 1, Su), lambda b_: (b_, 0, 0))] * 3
        + [pl.BlockSpec((1, C, Sk), lambda b_: (b_, 0, 0))],
        out_specs=pl.BlockSpec((1, C, Su), lambda b_: (b_, 0, 0)),
        out_shape=jax.ShapeDtypeStruct((B, C, Su), jnp.float32),
    )(kcol(kx), kcol(ky), kcol(kz), urow(ux), urow(uy), urow(uz), feats)


_SA1_RAD = ((0.1, 32, 48), (0.2, 64, 80), (0.4, 128, 144))


def _bq_sa1(x, y, z, cx, cy, cz):
    B, N = x.shape
    S = cx.shape[1]
    NW = 32
    RPW = B * S // NW
    WPB = NW // B
    SPW = S // WPB
    NJ = N // 16
    mesh = plsc.VectorSubcoreMesh(core_axis_name="c", subcore_axis_name="s")
    out_type = []
    for (_, ns, kp) in _SA1_RAD:
        out_type += [jax.ShapeDtypeStruct((B * S * kp,), jnp.float32)] * 3
        out_type += [jax.ShapeDtypeStruct((B * S,), jnp.int32)]
    scratch = ([pltpu.VMEM((N,), jnp.float32)] * 3
               + [pltpu.VMEM((RPW,), jnp.float32)] * 3)
    for (_, ns, kp) in _SA1_RAD:
        scratch += [pltpu.VMEM((16 * kp,), jnp.float32)] * 3
        scratch += [pltpu.VMEM((16,), jnp.int32)]

    @functools.partial(
        pl.kernel, mesh=mesh, out_type=out_type, scratch_types=scratch,
        compiler_params=pltpu.CompilerParams(needs_layout_passes=False))
    def k(xh, yh, zh, cxh, cyh, czh, *refs):
        outs = refs[:12]
        xv, yv, zv, ccx, ccy, ccz = refs[12:18]
        bufs = refs[18:]
        wid = lax.axis_index("s") * 2 + lax.axis_index("c")
        b = wid // WPB
        s0 = pl.multiple_of((wid % WPB) * SPW, SPW)
        pltpu.sync_copy(xh.at[b], xv)
        pltpu.sync_copy(yh.at[b], yv)
        pltpu.sync_copy(zh.at[b], zv)
        pltpu.sync_copy(cxh.at[b, pl.ds(s0, SPW)], ccx)
        pltpu.sync_copy(cyh.at[b, pl.ds(s0, SPW)], ccy)
        pltpu.sync_copy(czh.at[b, pl.ds(s0, SPW)], ccz)
        lane = lax.iota(jnp.int32, 16)

        def group_body(g, _):
            g16 = pl.multiple_of(g * 16, 16)
            cxg = ccx[pl.ds(g16, 16)]
            cyg = ccy[pl.ds(g16, 16)]
            czg = ccz[pl.ds(g16, 16)]
            cntv = [jnp.zeros((16,), jnp.int32) for _ in _SA1_RAD]
            for ic in range(16):
                cxb = jnp.full((16,), cxg[ic], jnp.float32)
                cyb = jnp.full((16,), cyg[ic], jnp.float32)
                czb = jnp.full((16,), czg[ic], jnp.float32)

                def pt_body(j, cnts, ic=ic, cxb=cxb, cyb=cyb, czb=czb):
                    px = xv[pl.ds(j * 16, 16)]
                    py = yv[pl.ds(j * 16, 16)]
                    pz = zv[pl.ds(j * 16, 16)]
                    dx = px - cxb
                    dy = py - cyb
                    dz = pz - czb
                    d2 = dx * dx + dy * dy + dz * dz
                    new = []
                    for ri, (r, ns, kp) in enumerate(_SA1_RAD):
                        cnt = cnts[ri]
                        m = d2 <= r * r
                        mi = m.astype(jnp.int32)
                        incl = jnp.cumsum(mi)
                        tgt = (incl - mi) + (ic * kp + jnp.minimum(cnt, ns))
                        plsc.store_scatter(bufs[4 * ri + 0], [tgt], px,
                                           mask=m)
                        plsc.store_scatter(bufs[4 * ri + 1], [tgt], py,
                                           mask=m)
                        plsc.store_scatter(bufs[4 * ri + 2], [tgt], pz,
                                           mask=m)
                        new.append(cnt + plsc.all_reduce_population_count(m))
                    return tuple(new)

                zero16 = jnp.zeros((16,), jnp.int32)
                cnts = lax.fori_loop(0, NJ, pt_body, (zero16, zero16,
                                                      zero16), unroll=2)
                for ri, (r, ns, kp) in enumerate(_SA1_RAD):
                    cntv[ri] = cntv[ri] + jnp.where(
                        lane == ic, jnp.minimum(cnts[ri], ns), 0)

            row0 = pl.multiple_of(wid * RPW + g16, 16)
            for ri, (r, ns, kp) in enumerate(_SA1_RAD):
                cb = bufs[4 * ri + 3]
                cb[...] = cntv[ri]
                for ch in range(3):
                    pltpu.sync_copy(
                        bufs[4 * ri + ch],
                        outs[4 * ri + ch].at[
                            pl.ds(pl.multiple_of(row0 * kp, 16 * kp),
                                  16 * kp)])
                pltpu.sync_copy(cb, outs[4 * ri + 3].at[pl.ds(row0, 16)])
            return 0

        lax.fori_loop(0, RPW // 16, group_body, 0)

    res = k(x, y, z, cx, cy, cz)
    groups = []
    for ri, (r, ns, kp) in enumerate(_SA1_RAD):
        ox, oy, oz, cnt = res[4 * ri:4 * ri + 4]
        groups.append((ox.reshape(B, S, kp), oy.reshape(B, S, kp),
                       oz.reshape(B, S, kp), cnt.reshape(B, 1, S)))
    return groups


_SA2_RAD = ((0.4, 64, 80), (0.8, 128, 144))


def _bq_sa2(x, y, z, cx, cy, cz):
    B, N = x.shape
    S = cx.shape[1]
    NW = 32
    RPW = B * S // NW
    WPB = NW // B
    SPW = S // WPB
    NJ = N // 16
    mesh = plsc.VectorSubcoreMesh(core_axis_name="c", subcore_axis_name="s")
    out_type = []
    for (_, ns, kp) in _SA2_RAD:
        out_type += [jax.ShapeDtypeStruct((B * S * kp,), jnp.int32),
                     jax.ShapeDtypeStruct((B * S,), jnp.int32)]
    scratch = ([pltpu.VMEM((N,), jnp.float32)] * 3
               + [pltpu.VMEM((SPW,), jnp.float32)] * 3)
    for (_, ns, kp) in _SA2_RAD:
        scratch += [pltpu.VMEM((16 * kp,), jnp.int32),
                    pltpu.VMEM((16,), jnp.int32)]

    @functools.partial(
        pl.kernel, mesh=mesh, out_type=out_type, scratch_types=scratch,
        compiler_params=pltpu.CompilerParams(needs_layout_passes=False))
    def k(xh, yh, zh, cxh, cyh, czh, *refs):
        outs = refs[:4]
        xv, yv, zv, ccx, ccy, ccz = refs[4:10]
        bufs = refs[10:]
        wid = lax.axis_index("s") * 2 + lax.axis_index("c")
        b = wid // WPB
        s0 = pl.multiple_of((wid % WPB) * SPW, SPW)
        pltpu.sync_copy(xh.at[b], xv)
        pltpu.sync_copy(yh.at[b], yv)
        pltpu.sync_copy(zh.at[b], zv)
        pltpu.sync_copy(cxh.at[b, pl.ds(s0, SPW)], ccx)
        pltpu.sync_copy(cyh.at[b, pl.ds(s0, SPW)], ccy)
        pltpu.sync_copy(czh.at[b, pl.ds(s0, SPW)], ccz)
        lane = lax.iota(jnp.int32, 16)

        def group_body(g, _):
            g16 = pl.multiple_of(g * 16, 16)
            cxg = ccx[pl.ds(g16, 16)]
            cyg = ccy[pl.ds(g16, 16)]
            czg = ccz[pl.ds(g16, 16)]
            cntv = [jnp.zeros((16,), jnp.int32) for _ in _SA2_RAD]
            for ic in range(16):
                cxb = jnp.full((16,), cxg[ic], jnp.float32)
                cyb = jnp.full((16,), cyg[ic], jnp.float32)
                czb = jnp.full((16,), czg[ic], jnp.float32)

                def pt_body(j, cnts, ic=ic, cxb=cxb, cyb=cyb, czb=czb):
                    px = xv[pl.ds(j * 16, 16)]
                    py = yv[pl.ds(j * 16, 16)]
                    pz = zv[pl.ds(j * 16, 16)]
                    dx = px - cxb
                    dy = py - cyb
                    dz = pz - czb
                    d2 = dx * dx + dy * dy + dz * dz
                    pidx = j * 16 + lane
                    new = []
                    for ri, (r, ns, kp) in enumerate(_SA2_RAD):
                        cnt = cnts[ri]
                        m = d2 <= r * r
                        mi = m.astype(jnp.int32)
                        incl = jnp.cumsum(mi)
                        tgt = (incl - mi) + (ic * kp
                                             + jnp.minimum(cnt, ns))
                        plsc.store_scatter(bufs[2 * ri], [tgt], pidx,
                                           mask=m)
                        new.append(cnt + plsc.all_reduce_population_count(m))
                    return tuple(new)

                zero16 = jnp.zeros((16,), jnp.int32)
                cnts = lax.fori_loop(0, NJ, pt_body, (zero16, zero16),
                                     unroll=2)
                for ri, (r, ns, kp) in enumerate(_SA2_RAD):
                    cntv[ri] = cntv[ri] + jnp.where(
                        lane == ic, jnp.minimum(cnts[ri], ns), 0)

            row0 = pl.multiple_of(wid * RPW + g16, 16)
            for ri, (r, ns, kp) in enumerate(_SA2_RAD):
                cb = bufs[2 * ri + 1]
                cb[...] = cntv[ri]
                pltpu.sync_copy(
                    bufs[2 * ri],
                    outs[2 * ri].at[
                        pl.ds(pl.multiple_of(row0 * kp, 16 * kp), 16 * kp)])
                pltpu.sync_copy(cb, outs[2 * ri + 1].at[pl.ds(row0, 16)])
            return 0

        lax.fori_loop(0, RPW // 16, group_body, 0)

    res = k(x, y, z, cx, cy, cz)
    groups = []
    for ri, (r, ns, kp) in enumerate(_SA2_RAD):
        idx, cnt = res[2 * ri:2 * ri + 2]
        groups.append((idx.reshape(B, S, kp), cnt.reshape(B, S)))
    return groups


def _sc_gather(A2, idx, cnt, ns, Np):
    BNp, C = A2.shape
    B, S, KP = idx.shape
    NW = 32
    WPB = 4
    KPW = ns // WPB
    NJ = S // 16
    mesh = plsc.VectorSubcoreMesh(core_axis_name="c", subcore_axis_name="s")
    idx_f = idx.reshape(B, S * KP)
    out_type = jax.ShapeDtypeStruct((B, ns, S, C), jnp.float32)
    scratch = [pltpu.VMEM((S * KP,), jnp.int32),
               pltpu.VMEM((S,), jnp.int32),
               pltpu.VMEM((S,), jnp.int32),
               pltpu.VMEM((S, C), jnp.float32),
               pltpu.SemaphoreType.DMA]

    @functools.partial(
        pl.kernel, mesh=mesh, out_type=out_type, scratch_types=scratch,
        compiler_params=pltpu.CompilerParams(needs_layout_passes=False))
    def k(ah, ih, ch, oh, iv, cv, gidx, rows, sem):
        wid = lax.axis_index("s") * 2 + lax.axis_index("c")
        b = wid // WPB
        k0 = (wid % WPB) * KPW
        pltpu.sync_copy(ih.at[b], iv)
        pltpu.sync_copy(ch.at[b], cv)
        lane = lax.iota(jnp.int32, 16)
        base = b * Np

        def k_body(kk, _):
            kq = k0 + kk
            for jj in range(NJ):
                offs = (jj * 16 + lane) * KP + kq
                raw = plsc.load_gather(iv, [offs])
                cchunk = cv[pl.ds(jj * 16, 16)]
                safe = jnp.minimum(jnp.maximum(raw, 0), Np - 1)
                sel = jnp.where(kq < cchunk, safe, 0) + base
                gidx[pl.ds(jj * 16, 16)] = sel
            pltpu.async_copy(ah.at[gidx], rows, sem).wait()
            pltpu.sync_copy(rows, oh.at[b, kq])
            return 0

        lax.fori_loop(0, KPW, k_body, 0)

    return k(A2, idx_f, cnt)


def _sa2s_body(a_ref, cnt_ref, c_ref, w1xt_ref, *refs, nlayers, K):
    out_ref = refs[-1]
    corr = jnp.dot(c_ref[0], w1xt_ref[...],
                   preferred_element_type=jnp.float32)
    cntcol = cnt_ref[0]
    S = out_ref.shape[2]
    Cout = out_ref.shape[1]

    def body(kq, m):
        ak = a_ref[0, pl.ds(kq, 1)][0]
        vk = (cntcol > kq).astype(jnp.float32)
        x = jnp.maximum(ak - corr, 0.0)
        for i in range(nlayers):
            Wt = refs[2 * i][...]
            bt = refs[2 * i + 1][...]
            x = jnp.maximum(jnp.dot(x, Wt,
                                    preferred_element_type=jnp.float32)
                            + bt, 0.0)
        return jnp.maximum(m, x * vk)

    m = jax.lax.fori_loop(0, K, body, jnp.zeros((S, Cout), jnp.float32))
    out_ref[0] = jnp.transpose(m)


def _sa2_branch_s(A4, cnt, centers_t, W1x, layers):
    B, K, S, C1 = A4.shape
    nlayers = len(layers)
    Cout = layers[-1][0].shape[0]
    wargs = []
    wspecs = []
    for (W, b) in layers:
        Wt = jnp.transpose(W)
        bt = jnp.transpose(b)
        wargs += [Wt, bt]
        wspecs += [pl.BlockSpec(Wt.shape, lambda b_: (0, 0)),
                   pl.BlockSpec(bt.shape, lambda b_: (0, 0))]
    W1xt = jnp.transpose(W1x)
    return pl.pallas_call(
        functools.partial(_sa2s_body, nlayers=nlayers, K=K),
        grid=(B,),
        in_specs=[pl.BlockSpec((1, K, S, C1), lambda b_: (b_, 0, 0, 0)),
                  pl.BlockSpec((1, S, 1), lambda b_: (b_, 0, 0)),
                  pl.BlockSpec((1, S, 3), lambda b_: (b_, 0, 0)),
                  pl.BlockSpec(W1xt.shape, lambda b_: (0, 0))] + wspecs,
        out_specs=pl.BlockSpec((1, Cout, S), lambda b_: (b_, 0, 0)),
        out_shape=jax.ShapeDtypeStruct((B, Cout, S), jnp.float32),
    )(A4, cnt, centers_t, W1xt, *wargs)


def _bq_fix_body(ox_ref, oy_ref, oz_ref, cnt_ref,
                 gx_ref, gy_ref, gz_ref, v_ref, *, K):
    S = cnt_ref.shape[2]
    cnt = cnt_ref[0]
    iota_k = lax.broadcasted_iota(jnp.int32, (K, S), 0)
    vm = iota_k < cnt
    for src, dst in ((ox_ref, gx_ref), (oy_ref, gy_ref), (oz_ref, gz_ref)):
        t = jnp.transpose(src[0])[:K]
        dst[0] = jnp.where(vm, t, 0.0)
    v_ref[0] = vm.astype(jnp.float32)


def _bq_fix(ox, oy, oz, cnt, K):
    B, S, KP = ox.shape
    ospec = pl.BlockSpec((1, S, KP), lambda b_: (b_, 0, 0))
    gspec = pl.BlockSpec((1, K, S), lambda b_: (b_, 0, 0))
    return pl.pallas_call(
        functools.partial(_bq_fix_body, K=K),
        grid=(B,),
        in_specs=[ospec] * 3 + [pl.BlockSpec((1, 1, S), lambda b_: (b_, 0, 0))],
        out_specs=[gspec] * 4,
        out_shape=[jax.ShapeDtypeStruct((B, K, S), jnp.float32)] * 4,
    )(ox, oy, oz, cnt)


def kernel(xyz, params):
    B, N, _ = xyz.shape

    sa1_layers = [_fold_layers(ls) for ls in params['sa1']]
    sa2_layers = [_fold_layers(ls) for ls in params['sa2']]
    sa3_layers = _fold_layers(params['sa3'])
    fp3_layers = _fold_layers(params['fp3'])
    fp2_layers = _fold_layers(params['fp2'])

    c1x, c1y, c1z, c2x, c2y, c2z = _fps(xyz)
    c1 = jnp.stack([c1x, c1y, c1z], axis=1)
    c2 = jnp.stack([c2x, c2y, c2z], axis=1)

    groups = _bq_sa1(xyz[:, :, 0], xyz[:, :, 1], xyz[:, :, 2],
                     c1x, c1y, c1z)
    outs1 = []
    for (ox, oy, oz, cnt), (r, ns, kp), layers in zip(groups, _SA1_RAD,
                                                      sa1_layers):
        gx, gy, gz, valid = _bq_fix(ox, oy, oz, cnt, ns)
        outs1.append(_sa1_branch(gx, gy, gz, valid, c1, layers))
    l1_points = jnp.concatenate(outs1, axis=1)

    src2 = jnp.concatenate([c1, l1_points], axis=1)
    groups2 = _bq_sa2(c1x, c1y, c1z, c2x, c2y, c2z)
    c2t = jnp.transpose(c2, (0, 2, 1))
    outs2 = []
    for (idx, cnt), (r, ns, kp), layers in zip(groups2, _SA2_RAD,
                                               sa2_layers):
        (W1, b1) = layers[0]
        A = _mlp(src2, [(W1, b1)], relus=(False,))
        A2 = jnp.transpose(A, (0, 2, 1)).reshape(B * 512, 128)
        A4 = _sc_gather(A2, idx, cnt, ns, 512)
        outs2.append(_sa2_branch_s(A4, cnt.reshape(B, 128, 1), c2t,
                                   W1[:, :3], layers[1:]))
    l2_points = jnp.concatenate(outs2, axis=1)

    g3 = jnp.concatenate([c2, l2_points], axis=1)
    l3 = _mlp(g3, sa3_layers, pool=True)

    interp3 = jnp.broadcast_to(l3, (B, 1024, 128))
    f3 = jnp.concatenate([interp3, l2_points], axis=1)
    l2f = _mlp(f3, fp3_layers)

    interp2 = _interp3(c2x, c2y, c2z, c1x, c1y, c1z, l2f)
    f2 = jnp.concatenate([interp2, l1_points], axis=1)
    l1f = _mlp(f2, fp2_layers)

    p = params['conv1']
    s1 = p['g1'] * _BN_S
    W1 = p['W1'] * s1[:, None]
    b1 = (p['b1'] * s1 + p['be1'])[:, None]
    W2 = p['W2']
    b2 = p['b2'][:, None]
    seg, obj, bck = _head(l1f, W1, b1, W2, b2)

    final_seg = _interp3(c1x, c1y, c1z,
                         xyz[:, :, 0], xyz[:, :, 1], xyz[:, :, 2],
                         seg, unit_w=True)

    return (seg, l1f, jnp.squeeze(obj, -1), jnp.squeeze(bck, -1), final_seg)


# speedup vs baseline: 22.5059x; 1.1033x over previous
import functools

import jax
import jax.numpy as jnp
import numpy as np
from jax import lax
from jax.experimental import pallas as pl
from jax.experimental.pallas import tpu as pltpu
from jax.experimental.pallas import tpu_sc as plsc

_BN_S = 1.0 / np.sqrt(1.0 + 1e-5)


def _fold_layers(layers):
    out = []
    for (W, b, g, be) in layers:
        s = g * _BN_S
        out.append((W * s[:, None], (b * s + be)[:, None]))
    return out


def _mlp_body(x_ref, *refs, nlayers, relus, pool):
    out_ref = refs[-1]
    x = x_ref[0]
    for i in range(nlayers):
        W = refs[2 * i][...]
        b = refs[2 * i + 1][...]
        x = jnp.dot(W, x, preferred_element_type=jnp.float32) + b
        if relus[i]:
            x = jnp.maximum(x, 0.0)
    if pool:
        out_ref[0] = jnp.max(x, axis=-1, keepdims=True)
    else:
        out_ref[0] = x


def _mlp(x, layers, relus=None, pool=False):
    B, Cin, S = x.shape
    nlayers = len(layers)
    if relus is None:
        relus = (True,) * nlayers
    Cout = layers[-1][0].shape[0]
    Sout = 1 if pool else S
    wargs = []
    wspecs = []
    for (W, b) in layers:
        wargs += [W, b]
        wspecs += [pl.BlockSpec(W.shape, lambda b_: (0, 0)),
                   pl.BlockSpec(b.shape, lambda b_: (0, 0))]
    return pl.pallas_call(
        functools.partial(_mlp_body, nlayers=nlayers, relus=tuple(relus),
                          pool=pool),
        grid=(B,),
        in_specs=[pl.BlockSpec((1, Cin, S), lambda b_: (b_, 0, 0))] + wspecs,
        out_specs=pl.BlockSpec((1, Cout, Sout), lambda b_: (b_, 0, 0)),
        out_shape=jax.ShapeDtypeStruct((B, Cout, Sout), jnp.float32),
    )(x, *wargs)


def _sa1_body(gx_ref, gy_ref, gz_ref, v_ref, c_ref, *refs, nlayers, K):
    out_ref = refs[-1]
    c3 = c_ref[0]
    cx, cy, cz = c3[0:1], c3[1:2], c3[2:3]
    Cout = out_ref.shape[1]
    S = out_ref.shape[2]

    def body(k, m):
        xk = gx_ref[0, pl.ds(k, 1), :]
        yk = gy_ref[0, pl.ds(k, 1), :]
        zk = gz_ref[0, pl.ds(k, 1), :]
        vk = v_ref[0, pl.ds(k, 1), :]
        x = jnp.concatenate([xk - cx, yk - cy, zk - cz, xk, yk, zk], axis=0)
        for i in range(nlayers):
            W = refs[2 * i][...]
            b = refs[2 * i + 1][...]
            x = jnp.maximum(jnp.dot(W, x, preferred_element_type=jnp.float32)
                            + b, 0.0)
        return jnp.maximum(m, x * vk)

    out_ref[0] = jax.lax.fori_loop(0, K, body, jnp.zeros((Cout, S),
                                                         jnp.float32),
                                   unroll=2)


def _sa1_branch(gx, gy, gz, valid, centers, layers):
    B, K, S = gx.shape
    nlayers = len(layers)
    Cout = layers[-1][0].shape[0]
    wargs = []
    wspecs = []
    for (W, b) in layers:
        wargs += [W, b]
        wspecs += [pl.BlockSpec(W.shape, lambda b_: (0, 0)),
                   pl.BlockSpec(b.shape, lambda b_: (0, 0))]
    gspec = pl.BlockSpec((1, K, S), lambda b_: (b_, 0, 0))
    return pl.pallas_call(
        functools.partial(_sa1_body, nlayers=nlayers, K=K),
        grid=(B,),
        in_specs=[gspec, gspec, gspec, gspec,
                  pl.BlockSpec((1, 3, S), lambda b_: (b_, 0, 0))] + wspecs,
        out_specs=pl.BlockSpec((1, Cout, S), lambda b_: (b_, 0, 0)),
        out_shape=jax.ShapeDtypeStruct((B, Cout, S), jnp.float32),
    )(gx, gy, gz, valid, centers, *wargs)


def _head_body(x_ref, w1_ref, b1_ref, w2_ref, b2_ref, seg_ref, obj_ref,
               bck_ref):
    x = x_ref[0]
    h = jnp.maximum(jnp.dot(w1_ref[...], x,
                            preferred_element_type=jnp.float32)
                    + b1_ref[...], 0.0)
    z = jnp.dot(w2_ref[...], h, preferred_element_type=jnp.float32) \
        + b2_ref[...]
    s = 1.0 / (1.0 + jnp.exp(-z))
    seg_ref[0] = s
    obj_ref[0] = jnp.max(s * x, axis=-1, keepdims=True)
    bck_ref[0] = jnp.max((1.0 - s) * x, axis=-1, keepdims=True)


def _head(x, W1, b1, W2, b2):
    B, C, S = x.shape
    return pl.pallas_call(
        _head_body,
        grid=(B,),
        in_specs=[pl.BlockSpec((1, C, S), lambda b_: (b_, 0, 0)),
                  pl.BlockSpec(W1.shape, lambda b_: (0, 0)),
                  pl.BlockSpec(b1.shape, lambda b_: (0, 0)),
                  pl.BlockSpec(W2.shape, lambda b_: (0, 0)),
                  pl.BlockSpec(b2.shape, lambda b_: (0, 0))],
        out_specs=[pl.BlockSpec((1, 1, S), lambda b_: (b_, 0, 0)),
                   pl.BlockSpec((1, C, 1), lambda b_: (b_, 0, 0)),
                   pl.BlockSpec((1, C, 1), lambda b_: (b_, 0, 0))],
        out_shape=[jax.ShapeDtypeStruct((B, 1, S), jnp.float32),
                   jax.ShapeDtypeStruct((B, C, 1), jnp.float32),
                   jax.ShapeDtypeStruct((B, C, 1), jnp.float32)],
    )(x, W1, b1, W2, b2)


def _fps_level(x, y, z, S, cx_ref, cy_ref, cz_ref):
    B, N = x.shape
    iota_n = jax.lax.broadcasted_iota(jnp.int32, (B, N), 1)
    iota_s = jax.lax.broadcasted_iota(jnp.int32, (B, S), 1)
    cx_ref[...] = jnp.zeros((B, S), jnp.float32)
    cy_ref[...] = jnp.zeros((B, S), jnp.float32)
    cz_ref[...] = jnp.zeros((B, S), jnp.float32)

    def body(t, carry):
        dist, far = carry
        sel = (iota_n == far).astype(jnp.float32)
        cx = jnp.sum(x * sel, -1, keepdims=True)
        cy = jnp.sum(y * sel, -1, keepdims=True)
        cz = jnp.sum(z * sel, -1, keepdims=True)
        oh = (iota_s == t).astype(jnp.float32)
        cx_ref[...] += cx * oh
        cy_ref[...] += cy * oh
        cz_ref[...] += cz * oh
        d = (x - cx) ** 2 + (y - cy) ** 2 + (z - cz) ** 2
        dist = jnp.minimum(dist, d)
        m = jnp.max(dist, -1, keepdims=True)
        far = jnp.min(jnp.where(dist == m, iota_n, N), -1, keepdims=True)
        return dist, far

    jax.lax.fori_loop(
        0, S, body,
        (jnp.full((B, N), 1e10, jnp.float32),
         jnp.zeros((B, 1), jnp.int32)))


def _fps_body(x_ref, y_ref, z_ref,
              c1x_ref, c1y_ref, c1z_ref, c2x_ref, c2y_ref, c2z_ref):
    _fps_level(x_ref[...], y_ref[...], z_ref[...], 512,
               c1x_ref, c1y_ref, c1z_ref)
    _fps_level(c1x_ref[...], c1y_ref[...], c1z_ref[...], 128,
               c2x_ref, c2y_ref, c2z_ref)


def _fps(xyz):
    B, N, _ = xyz.shape
    x = xyz[:, :, 0]
    y = xyz[:, :, 1]
    z = xyz[:, :, 2]
    full = lambda s: pl.BlockSpec(s, lambda: tuple(0 for _ in s))
    return pl.pallas_call(
        _fps_body,
        in_specs=[full((B, N))] * 3,
        out_specs=[full((B, 512))] * 3 + [full((B, 128))] * 3,
        out_shape=[jax.ShapeDtypeStruct((B, 512), jnp.float32)] * 3
        + [jax.ShapeDtypeStruct((B, 128), jnp.float32)] * 3,
    )(x, y, z)


def _interp3_body(kx_ref, ky_ref, kz_ref, ux_ref, uy_ref, uz_ref, f_ref,
                  out_ref, *, unit_w):
    kx = kx_ref[0]
    ky = ky_ref[0]
    kz = kz_ref[0]
    ux = ux_ref[0]
    uy = uy_ref[0]
    uz = uz_ref[0]
    Sk = kx.shape[0]
    Su = ux.shape[1]
    d2 = (kx - ux) ** 2 + (ky - uy) ** 2 + (kz - uz) ** 2
    iota_k = jax.lax.broadcasted_iota(jnp.int32, (Sk, Su), 0)
    E = jnp.zeros((Sk, Su), jnp.float32)
    ws = []
    idxs = []
    for _ in range(3):
        m = jnp.min(d2, axis=0, keepdims=True)
        i = jnp.min(jnp.where(d2 == m, iota_k, Sk), axis=0, keepdims=True)
        idxs.append(i)
        ws.append(1.0 / (jnp.sqrt(jnp.maximum(m, 0.0)) + 1e-8))
        d2 = jnp.where(iota_k == i, jnp.float32(3.0e38), d2)
    if unit_w:
        for i in idxs:
            E += (iota_k == i).astype(jnp.float32)
    else:
        wsum = ws[0] + ws[1] + ws[2]
        for w, i in zip(ws, idxs):
            E += jnp.where(iota_k == i, w / wsum, 0.0)
    out_ref[0] = jnp.dot(f_ref[0], E, preferred_element_type=jnp.float32)


def _interp3(kx, ky, kz, ux, uy, uz, feats, unit_w=False):
    B, Sk = kx.shape
    Su = ux.shape[1]
    C = feats.shape[1]
    kcol = lambda a: a.reshape(B, Sk, 1)
    urow = lambda a: a.reshape(B, 1, Su)
    return pl.pallas_call(
        functools.partial(_interp3_body, unit_w=unit_w),
        grid=(B,),
        in_specs=[pl.BlockSpec((1, Sk, 1), lambda b_: (b_, 0, 0))] * 3
        + [pl.BlockSpec((1, 1, Su), lambda b_: (b_, 0, 0))] * 3
        + [pl.BlockSpec((1, C, Sk), lambda b_: (b_, 0, 0))],
        out_specs=pl.BlockSpec((1, C, Su), lambda b_: (b_, 0, 0)),
        out_shape=jax.ShapeDtypeStruct((B, C, Su), jnp.float32),
    )(kcol(kx), kcol(ky), kcol(kz), urow(ux), urow(uy), urow(uz), feats)


_SA1_RAD = ((0.1, 32, 48), (0.2, 64, 80), (0.4, 128, 144))


def _bq_sa1(x, y, z, cx, cy, cz):
    B, N = x.shape
    S = cx.shape[1]
    NW = 32
    RPW = B * S // NW
    WPB = NW // B
    SPW = S // WPB
    NJ = N // 16
    mesh = plsc.VectorSubcoreMesh(core_axis_name="c", subcore_axis_name="s")
    out_type = []
    for (_, ns, kp) in _SA1_RAD:
        out_type += [jax.ShapeDtypeStruct((B * S * kp,), jnp.float32)] * 3
        out_type += [jax.ShapeDtypeStruct((B * S,), jnp.int32)]
    scratch = ([pltpu.VMEM((N,), jnp.float32)] * 3
               + [pltpu.VMEM((RPW,), jnp.float32)] * 3)
    for (_, ns, kp) in _SA1_RAD:
        scratch += [pltpu.VMEM((16 * kp,), jnp.float32)] * 3
        scratch += [pltpu.VMEM((16,), jnp.int32)]

    @functools.partial(
        pl.kernel, mesh=mesh, out_type=out_type, scratch_types=scratch,
        compiler_params=pltpu.CompilerParams(needs_layout_passes=False))
    def k(xh, yh, zh, cxh, cyh, czh, *refs):
        outs = refs[:12]
        xv, yv, zv, ccx, ccy, ccz = refs[12:18]
        bufs = refs[18:]
        wid = lax.axis_index("s") * 2 + lax.axis_index("c")
        b = wid // WPB
        s0 = pl.multiple_of((wid % WPB) * SPW, SPW)
        pltpu.sync_copy(xh.at[b], xv)
        pltpu.sync_copy(yh.at[b], yv)
        pltpu.sync_copy(zh.at[b], zv)
        pltpu.sync_copy(cxh.at[b, pl.ds(s0, SPW)], ccx)
        pltpu.sync_copy(cyh.at[b, pl.ds(s0, SPW)], ccy)
        pltpu.sync_copy(czh.at[b, pl.ds(s0, SPW)], ccz)
        lane = lax.iota(jnp.int32, 16)

        def group_body(g, _):
            g16 = pl.multiple_of(g * 16, 16)
            cxg = ccx[pl.ds(g16, 16)]
            cyg = ccy[pl.ds(g16, 16)]
            czg = ccz[pl.ds(g16, 16)]
            cntv = [jnp.zeros((16,), jnp.int32) for _ in _SA1_RAD]
            for ic in range(16):
                cxb = jnp.full((16,), cxg[ic], jnp.float32)
                cyb = jnp.full((16,), cyg[ic], jnp.float32)
                czb = jnp.full((16,), czg[ic], jnp.float32)

                def pt_body(j, cnts, ic=ic, cxb=cxb, cyb=cyb, czb=czb):
                    px = xv[pl.ds(j * 16, 16)]
                    py = yv[pl.ds(j * 16, 16)]
                    pz = zv[pl.ds(j * 16, 16)]
                    dx = px - cxb
                    dy = py - cyb
                    dz = pz - czb
                    d2 = dx * dx + dy * dy + dz * dz
                    new = []
                    for ri, (r, ns, kp) in enumerate(_SA1_RAD):
                        cnt = cnts[ri]
                        m = d2 <= r * r
                        mi = m.astype(jnp.int32)
                        incl = jnp.cumsum(mi)
                        tgt = (incl - mi) + (ic * kp + jnp.minimum(cnt, ns))
                        plsc.store_scatter(bufs[4 * ri + 0], [tgt], px,
                                           mask=m)
                        plsc.store_scatter(bufs[4 * ri + 1], [tgt], py,
                                           mask=m)
                        plsc.store_scatter(bufs[4 * ri + 2], [tgt], pz,
                                           mask=m)
                        new.append(cnt + plsc.all_reduce_population_count(m))
                    return tuple(new)

                zero16 = jnp.zeros((16,), jnp.int32)
                cnts = lax.fori_loop(0, NJ, pt_body, (zero16, zero16,
                                                      zero16), unroll=4)
                for ri, (r, ns, kp) in enumerate(_SA1_RAD):
                    cntv[ri] = cntv[ri] + jnp.where(
                        lane == ic, jnp.minimum(cnts[ri], ns), 0)

            row0 = pl.multiple_of(wid * RPW + g16, 16)
            for ri, (r, ns, kp) in enumerate(_SA1_RAD):
                cb = bufs[4 * ri + 3]
                cb[...] = cntv[ri]
                for ch in range(3):
                    pltpu.sync_copy(
                        bufs[4 * ri + ch],
                        outs[4 * ri + ch].at[
                            pl.ds(pl.multiple_of(row0 * kp, 16 * kp),
                                  16 * kp)])
                pltpu.sync_copy(cb, outs[4 * ri + 3].at[pl.ds(row0, 16)])
            return 0

        lax.fori_loop(0, RPW // 16, group_body, 0)

    res = k(x, y, z, cx, cy, cz)
    groups = []
    for ri, (r, ns, kp) in enumerate(_SA1_RAD):
        ox, oy, oz, cnt = res[4 * ri:4 * ri + 4]
        groups.append((ox.reshape(B, S, kp), oy.reshape(B, S, kp),
                       oz.reshape(B, S, kp), cnt.reshape(B, 1, S)))
    return groups


_SA2_RAD = ((0.4, 64, 80), (0.8, 128, 144))


def _bq_sa2(x, y, z, cx, cy, cz):
    B, N = x.shape
    S = cx.shape[1]
    NW = 32
    RPW = B * S // NW
    WPB = NW // B
    SPW = S // WPB
    NJ = N // 16
    mesh = plsc.VectorSubcoreMesh(core_axis_name="c", subcore_axis_name="s")
    out_type = []
    for (_, ns, kp) in _SA2_RAD:
        out_type += [jax.ShapeDtypeStruct((B * S * kp,), jnp.int32),
                     jax.ShapeDtypeStruct((B * S,), jnp.int32)]
    scratch = ([pltpu.VMEM((N,), jnp.float32)] * 3
               + [pltpu.VMEM((SPW,), jnp.float32)] * 3)
    for (_, ns, kp) in _SA2_RAD:
        scratch += [pltpu.VMEM((16 * kp,), jnp.int32),
                    pltpu.VMEM((16,), jnp.int32)]

    @functools.partial(
        pl.kernel, mesh=mesh, out_type=out_type, scratch_types=scratch,
        compiler_params=pltpu.CompilerParams(needs_layout_passes=False))
    def k(xh, yh, zh, cxh, cyh, czh, *refs):
        outs = refs[:4]
        xv, yv, zv, ccx, ccy, ccz = refs[4:10]
        bufs = refs[10:]
        wid = lax.axis_index("s") * 2 + lax.axis_index("c")
        b = wid // WPB
        s0 = pl.multiple_of((wid % WPB) * SPW, SPW)
        pltpu.sync_copy(xh.at[b], xv)
        pltpu.sync_copy(yh.at[b], yv)
        pltpu.sync_copy(zh.at[b], zv)
        pltpu.sync_copy(cxh.at[b, pl.ds(s0, SPW)], ccx)
        pltpu.sync_copy(cyh.at[b, pl.ds(s0, SPW)], ccy)
        pltpu.sync_copy(czh.at[b, pl.ds(s0, SPW)], ccz)
        lane = lax.iota(jnp.int32, 16)

        def group_body(g, _):
            g16 = pl.multiple_of(g * 16, 16)
            cxg = ccx[pl.ds(g16, 16)]
            cyg = ccy[pl.ds(g16, 16)]
            czg = ccz[pl.ds(g16, 16)]
            cntv = [jnp.zeros((16,), jnp.int32) for _ in _SA2_RAD]
            for ic in range(16):
                cxb = jnp.full((16,), cxg[ic], jnp.float32)
                cyb = jnp.full((16,), cyg[ic], jnp.float32)
                czb = jnp.full((16,), czg[ic], jnp.float32)

                def pt_body(j, cnts, ic=ic, cxb=cxb, cyb=cyb, czb=czb):
                    px = xv[pl.ds(j * 16, 16)]
                    py = yv[pl.ds(j * 16, 16)]
                    pz = zv[pl.ds(j * 16, 16)]
                    dx = px - cxb
                    dy = py - cyb
                    dz = pz - czb
                    d2 = dx * dx + dy * dy + dz * dz
                    pidx = j * 16 + lane
                    new = []
                    for ri, (r, ns, kp) in enumerate(_SA2_RAD):
                        cnt = cnts[ri]
                        m = d2 <= r * r
                        mi = m.astype(jnp.int32)
                        incl = jnp.cumsum(mi)
                        tgt = (incl - mi) + (ic * kp
                                             + jnp.minimum(cnt, ns))
                        plsc.store_scatter(bufs[2 * ri], [tgt], pidx,
                                           mask=m)
                        new.append(cnt + plsc.all_reduce_population_count(m))
                    return tuple(new)

                zero16 = jnp.zeros((16,), jnp.int32)
                cnts = lax.fori_loop(0, NJ, pt_body, (zero16, zero16),
                                     unroll=2)
                for ri, (r, ns, kp) in enumerate(_SA2_RAD):
                    cntv[ri] = cntv[ri] + jnp.where(
                        lane == ic, jnp.minimum(cnts[ri], ns), 0)

            row0 = pl.multiple_of(wid * RPW + g16, 16)
            for ri, (r, ns, kp) in enumerate(_SA2_RAD):
                cb = bufs[2 * ri + 1]
                cb[...] = cntv[ri]
                pltpu.sync_copy(
                    bufs[2 * ri],
                    outs[2 * ri].at[
                        pl.ds(pl.multiple_of(row0 * kp, 16 * kp), 16 * kp)])
                pltpu.sync_copy(cb, outs[2 * ri + 1].at[pl.ds(row0, 16)])
            return 0

        lax.fori_loop(0, RPW // 16, group_body, 0)

    res = k(x, y, z, cx, cy, cz)
    groups = []
    for ri, (r, ns, kp) in enumerate(_SA2_RAD):
        idx, cnt = res[2 * ri:2 * ri + 2]
        groups.append((idx.reshape(B, S, kp), cnt.reshape(B, S)))
    return groups


def _sc_gather(A2, idx, cnt, ns, Np):
    BNp, C = A2.shape
    B, S, KP = idx.shape
    NW = 32
    WPB = 4
    KPW = ns // WPB
    NJ = S // 16
    mesh = plsc.VectorSubcoreMesh(core_axis_name="c", subcore_axis_name="s")
    idx_f = idx.reshape(B, S * KP)
    out_type = jax.ShapeDtypeStruct((B, ns, S, C), jnp.float32)
    scratch = [pltpu.VMEM((S * KP,), jnp.int32),
               pltpu.VMEM((S,), jnp.int32),
               pltpu.VMEM((S,), jnp.int32),
               pltpu.VMEM((S, C), jnp.float32),
               pltpu.SemaphoreType.DMA]

    @functools.partial(
        pl.kernel, mesh=mesh, out_type=out_type, scratch_types=scratch,
        compiler_params=pltpu.CompilerParams(needs_layout_passes=False))
    def k(ah, ih, ch, oh, iv, cv, gidx, rows, sem):
        wid = lax.axis_index("s") * 2 + lax.axis_index("c")
        b = wid // WPB
        k0 = (wid % WPB) * KPW
        pltpu.sync_copy(ih.at[b], iv)
        pltpu.sync_copy(ch.at[b], cv)
        lane = lax.iota(jnp.int32, 16)
        base = b * Np

        def k_body(kk, _):
            kq = k0 + kk
            for jj in range(NJ):
                offs = (jj * 16 + lane) * KP + kq
                raw = plsc.load_gather(iv, [offs])
                cchunk = cv[pl.ds(jj * 16, 16)]
                safe = jnp.minimum(jnp.maximum(raw, 0), Np - 1)
                sel = jnp.where(kq < cchunk, safe, 0) + base
                gidx[pl.ds(jj * 16, 16)] = sel
            pltpu.async_copy(ah.at[gidx], rows, sem).wait()
            pltpu.sync_copy(rows, oh.at[b, kq])
            return 0

        lax.fori_loop(0, KPW, k_body, 0)

    return k(A2, idx_f, cnt)


def _sa2s_body(a_ref, cnt_ref, c_ref, w1xt_ref, *refs, nlayers, K):
    out_ref = refs[-1]
    corr = jnp.dot(c_ref[0], w1xt_ref[...],
                   preferred_element_type=jnp.float32)
    cntcol = cnt_ref[0]
    S = out_ref.shape[2]
    Cout = out_ref.shape[1]

    def body(kq, m):
        ak = a_ref[0, pl.ds(kq, 1)][0]
        vk = (cntcol > kq).astype(jnp.float32)
        x = jnp.maximum(ak - corr, 0.0)
        for i in range(nlayers):
            Wt = refs[2 * i][...]
            bt = refs[2 * i + 1][...]
            x = jnp.maximum(jnp.dot(x, Wt,
                                    preferred_element_type=jnp.float32)
                            + bt, 0.0)
        return jnp.maximum(m, x * vk)

    m = jax.lax.fori_loop(0, K, body, jnp.zeros((S, Cout), jnp.float32),
                          unroll=2)
    out_ref[0] = jnp.transpose(m)


def _sa2_branch_s(A4, cnt, centers_t, W1x, layers):
    B, K, S, C1 = A4.shape
    nlayers = len(layers)
    Cout = layers[-1][0].shape[0]
    wargs = []
    wspecs = []
    for (W, b) in layers:
        Wt = jnp.transpose(W)
        bt = jnp.transpose(b)
        wargs += [Wt, bt]
        wspecs += [pl.BlockSpec(Wt.shape, lambda b_: (0, 0)),
                   pl.BlockSpec(bt.shape, lambda b_: (0, 0))]
    W1xt = jnp.transpose(W1x)
    return pl.pallas_call(
        functools.partial(_sa2s_body, nlayers=nlayers, K=K),
        grid=(B,),
        in_specs=[pl.BlockSpec((1, K, S, C1), lambda b_: (b_, 0, 0, 0)),
                  pl.BlockSpec((1, S, 1), lambda b_: (b_, 0, 0)),
                  pl.BlockSpec((1, S, 3), lambda b_: (b_, 0, 0)),
                  pl.BlockSpec(W1xt.shape, lambda b_: (0, 0))] + wspecs,
        out_specs=pl.BlockSpec((1, Cout, S), lambda b_: (b_, 0, 0)),
        out_shape=jax.ShapeDtypeStruct((B, Cout, S), jnp.float32),
    )(A4, cnt, centers_t, W1xt, *wargs)


def _bq_fix_body(ox_ref, oy_ref, oz_ref, cnt_ref,
                 gx_ref, gy_ref, gz_ref, v_ref, *, K):
    S = cnt_ref.shape[2]
    cnt = cnt_ref[0]
    iota_k = lax.broadcasted_iota(jnp.int32, (K, S), 0)
    vm = iota_k < cnt
    for src, dst in ((ox_ref, gx_ref), (oy_ref, gy_ref), (oz_ref, gz_ref)):
        t = jnp.transpose(src[0])[:K]
        dst[0] = jnp.where(vm, t, 0.0)
    v_ref[0] = vm.astype(jnp.float32)


def _bq_fix(ox, oy, oz, cnt, K):
    B, S, KP = ox.shape
    ospec = pl.BlockSpec((1, S, KP), lambda b_: (b_, 0, 0))
    gspec = pl.BlockSpec((1, K, S), lambda b_: (b_, 0, 0))
    return pl.pallas_call(
        functools.partial(_bq_fix_body, K=K),
        grid=(B,),
        in_specs=[ospec] * 3 + [pl.BlockSpec((1, 1, S), lambda b_: (b_, 0, 0))],
        out_specs=[gspec] * 4,
        out_shape=[jax.ShapeDtypeStruct((B, K, S), jnp.float32)] * 4,
    )(ox, oy, oz, cnt)


def kernel(xyz, params):
    B, N, _ = xyz.shape

    sa1_layers = [_fold_layers(ls) for ls in params['sa1']]
    sa2_layers = [_fold_layers(ls) for ls in params['sa2']]
    sa3_layers = _fold_layers(params['sa3'])
    fp3_layers = _fold_layers(params['fp3'])
    fp2_layers = _fold_layers(params['fp2'])

    c1x, c1y, c1z, c2x, c2y, c2z = _fps(xyz)
    c1 = jnp.stack([c1x, c1y, c1z], axis=1)
    c2 = jnp.stack([c2x, c2y, c2z], axis=1)

    groups = _bq_sa1(xyz[:, :, 0], xyz[:, :, 1], xyz[:, :, 2],
                     c1x, c1y, c1z)
    outs1 = []
    for (ox, oy, oz, cnt), (r, ns, kp), layers in zip(groups, _SA1_RAD,
                                                      sa1_layers):
        gx, gy, gz, valid = _bq_fix(ox, oy, oz, cnt, ns)
        outs1.append(_sa1_branch(gx, gy, gz, valid, c1, layers))
    l1_points = jnp.concatenate(outs1, axis=1)

    src2 = jnp.concatenate([c1, l1_points], axis=1)
    groups2 = _bq_sa2(c1x, c1y, c1z, c2x, c2y, c2z)
    c2t = jnp.transpose(c2, (0, 2, 1))
    outs2 = []
    for (idx, cnt), (r, ns, kp), layers in zip(groups2, _SA2_RAD,
                                               sa2_layers):
        (W1, b1) = layers[0]
        A = _mlp(src2, [(W1, b1)], relus=(False,))
        A2 = jnp.transpose(A, (0, 2, 1)).reshape(B * 512, 128)
        A4 = _sc_gather(A2, idx, cnt, ns, 512)
        outs2.append(_sa2_branch_s(A4, cnt.reshape(B, 128, 1), c2t,
                                   W1[:, :3], layers[1:]))
    l2_points = jnp.concatenate(outs2, axis=1)

    g3 = jnp.concatenate([c2, l2_points], axis=1)
    l3 = _mlp(g3, sa3_layers, pool=True)

    interp3 = jnp.broadcast_to(l3, (B, 1024, 128))
    f3 = jnp.concatenate([interp3, l2_points], axis=1)
    l2f = _mlp(f3, fp3_layers)

    interp2 = _interp3(c2x, c2y, c2z, c1x, c1y, c1z, l2f)
    f2 = jnp.concatenate([interp2, l1_points], axis=1)
    l1f = _mlp(f2, fp2_layers)

    p = params['conv1']
    s1 = p['g1'] * _BN_S
    W1 = p['W1'] * s1[:, None]
    b1 = (p['b1'] * s1 + p['be1'])[:, None]
    W2 = p['W2']
    b2 = p['b2'][:, None]
    seg, obj, bck = _head(l1f, W1, b1, W2, b2)

    final_seg = _interp3(c1x, c1y, c1z,
                         xyz[:, :, 0], xyz[:, :, 1], xyz[:, :, 2],
                         seg, unit_w=True)

    return (seg, l1f, jnp.squeeze(obj, -1), jnp.squeeze(bck, -1), final_seg)


# speedup vs baseline: 24.2379x; 1.0770x over previous
import functools

import jax
import jax.numpy as jnp
import numpy as np
from jax import lax
from jax.experimental import pallas as pl
from jax.experimental.pallas import tpu as pltpu
from jax.experimental.pallas import tpu_sc as plsc

_BN_S = 1.0 / np.sqrt(1.0 + 1e-5)


def _fold_layers(layers):
    out = []
    for (W, b, g, be) in layers:
        s = g * _BN_S
        out.append((W * s[:, None], (b * s + be)[:, None]))
    return out


def _mlp_body(x_ref, *refs, nlayers, relus, pool):
    out_ref = refs[-1]
    x = x_ref[0]
    for i in range(nlayers):
        W = refs[2 * i][...]
        b = refs[2 * i + 1][...]
        x = jnp.dot(W, x, preferred_element_type=jnp.float32) + b
        if relus[i]:
            x = jnp.maximum(x, 0.0)
    if pool:
        out_ref[0] = jnp.max(x, axis=-1, keepdims=True)
    else:
        out_ref[0] = x


def _mlp(x, layers, relus=None, pool=False):
    B, Cin, S = x.shape
    nlayers = len(layers)
    if relus is None:
        relus = (True,) * nlayers
    Cout = layers[-1][0].shape[0]
    Sout = 1 if pool else S
    wargs = []
    wspecs = []
    for (W, b) in layers:
        wargs += [W, b]
        wspecs += [pl.BlockSpec(W.shape, lambda b_: (0, 0)),
                   pl.BlockSpec(b.shape, lambda b_: (0, 0))]
    return pl.pallas_call(
        functools.partial(_mlp_body, nlayers=nlayers, relus=tuple(relus),
                          pool=pool),
        grid=(B,),
        in_specs=[pl.BlockSpec((1, Cin, S), lambda b_: (b_, 0, 0))] + wspecs,
        out_specs=pl.BlockSpec((1, Cout, Sout), lambda b_: (b_, 0, 0)),
        out_shape=jax.ShapeDtypeStruct((B, Cout, Sout), jnp.float32),
    )(x, *wargs)


def _sa1_body(gx_ref, gy_ref, gz_ref, v_ref, c_ref, *refs, nlayers, K):
    out_ref = refs[-1]
    c3 = c_ref[0]
    cx, cy, cz = c3[0:1], c3[1:2], c3[2:3]
    Cout = out_ref.shape[1]
    S = out_ref.shape[2]

    def body(k, m):
        xk = gx_ref[0, pl.ds(k, 1), :]
        yk = gy_ref[0, pl.ds(k, 1), :]
        zk = gz_ref[0, pl.ds(k, 1), :]
        vk = v_ref[0, pl.ds(k, 1), :]
        x = jnp.concatenate([xk - cx, yk - cy, zk - cz, xk, yk, zk], axis=0)
        for i in range(nlayers):
            W = refs[2 * i][...]
            b = refs[2 * i + 1][...]
            x = jnp.maximum(jnp.dot(W, x, preferred_element_type=jnp.float32)
                            + b, 0.0)
        return jnp.maximum(m, x * vk)

    out_ref[0] = jax.lax.fori_loop(0, K, body, jnp.zeros((Cout, S),
                                                         jnp.float32),
                                   unroll=4)


def _sa1_branch(gx, gy, gz, valid, centers, layers):
    B, K, S = gx.shape
    nlayers = len(layers)
    Cout = layers[-1][0].shape[0]
    wargs = []
    wspecs = []
    for (W, b) in layers:
        wargs += [W, b]
        wspecs += [pl.BlockSpec(W.shape, lambda b_: (0, 0)),
                   pl.BlockSpec(b.shape, lambda b_: (0, 0))]
    gspec = pl.BlockSpec((1, K, S), lambda b_: (b_, 0, 0))
    return pl.pallas_call(
        functools.partial(_sa1_body, nlayers=nlayers, K=K),
        grid=(B,),
        in_specs=[gspec, gspec, gspec, gspec,
                  pl.BlockSpec((1, 3, S), lambda b_: (b_, 0, 0))] + wspecs,
        out_specs=pl.BlockSpec((1, Cout, S), lambda b_: (b_, 0, 0)),
        out_shape=jax.ShapeDtypeStruct((B, Cout, S), jnp.float32),
    )(gx, gy, gz, valid, centers, *wargs)


def _head_body(x_ref, w1_ref, b1_ref, w2_ref, b2_ref, seg_ref, obj_ref,
               bck_ref):
    x = x_ref[0]
    h = jnp.maximum(jnp.dot(w1_ref[...], x,
                            preferred_element_type=jnp.float32)
                    + b1_ref[...], 0.0)
    z = jnp.dot(w2_ref[...], h, preferred_element_type=jnp.float32) \
        + b2_ref[...]
    s = 1.0 / (1.0 + jnp.exp(-z))
    seg_ref[0] = s
    obj_ref[0] = jnp.max(s * x, axis=-1, keepdims=True)
    bck_ref[0] = jnp.max((1.0 - s) * x, axis=-1, keepdims=True)


def _head(x, W1, b1, W2, b2):
    B, C, S = x.shape
    return pl.pallas_call(
        _head_body,
        grid=(B,),
        in_specs=[pl.BlockSpec((1, C, S), lambda b_: (b_, 0, 0)),
                  pl.BlockSpec(W1.shape, lambda b_: (0, 0)),
                  pl.BlockSpec(b1.shape, lambda b_: (0, 0)),
                  pl.BlockSpec(W2.shape, lambda b_: (0, 0)),
                  pl.BlockSpec(b2.shape, lambda b_: (0, 0))],
        out_specs=[pl.BlockSpec((1, 1, S), lambda b_: (b_, 0, 0)),
                   pl.BlockSpec((1, C, 1), lambda b_: (b_, 0, 0)),
                   pl.BlockSpec((1, C, 1), lambda b_: (b_, 0, 0))],
        out_shape=[jax.ShapeDtypeStruct((B, 1, S), jnp.float32),
                   jax.ShapeDtypeStruct((B, C, 1), jnp.float32),
                   jax.ShapeDtypeStruct((B, C, 1), jnp.float32)],
    )(x, W1, b1, W2, b2)


def _fps_level(x, y, z, S, cx_ref, cy_ref, cz_ref):
    B, N = x.shape
    iota_n = jax.lax.broadcasted_iota(jnp.int32, (B, N), 1)
    iota_s = jax.lax.broadcasted_iota(jnp.int32, (B, S), 1)
    cx_ref[...] = jnp.zeros((B, S), jnp.float32)
    cy_ref[...] = jnp.zeros((B, S), jnp.float32)
    cz_ref[...] = jnp.zeros((B, S), jnp.float32)

    def body(t, carry):
        dist, far = carry
        sel = (iota_n == far).astype(jnp.float32)
        cx = jnp.sum(x * sel, -1, keepdims=True)
        cy = jnp.sum(y * sel, -1, keepdims=True)
        cz = jnp.sum(z * sel, -1, keepdims=True)
        oh = (iota_s == t).astype(jnp.float32)
        cx_ref[...] += cx * oh
        cy_ref[...] += cy * oh
        cz_ref[...] += cz * oh
        d = (x - cx) ** 2 + (y - cy) ** 2 + (z - cz) ** 2
        dist = jnp.minimum(dist, d)
        m = jnp.max(dist, -1, keepdims=True)
        far = jnp.min(jnp.where(dist == m, iota_n, N), -1, keepdims=True)
        return dist, far

    jax.lax.fori_loop(
        0, S, body,
        (jnp.full((B, N), 1e10, jnp.float32),
         jnp.zeros((B, 1), jnp.int32)))


def _fps_body(x_ref, y_ref, z_ref, cx_ref, cy_ref, cz_ref, *, S):
    _fps_level(x_ref[...], y_ref[...], z_ref[...], S,
               cx_ref, cy_ref, cz_ref)


def _fps_stage(x, y, z, S):
    B, N = x.shape
    full = lambda s: pl.BlockSpec(s, lambda: tuple(0 for _ in s))
    return pl.pallas_call(
        functools.partial(_fps_body, S=S),
        in_specs=[full((B, N))] * 3,
        out_specs=[full((B, S))] * 3,
        out_shape=[jax.ShapeDtypeStruct((B, S), jnp.float32)] * 3,
    )(x, y, z)


def _fps(xyz):
    x = xyz[:, :, 0]
    y = xyz[:, :, 1]
    z = xyz[:, :, 2]
    c1x, c1y, c1z = _fps_stage(x, y, z, 512)
    c2x, c2y, c2z = _fps_stage(c1x, c1y, c1z, 128)
    return c1x, c1y, c1z, c2x, c2y, c2z


def _interp3_body(kx_ref, ky_ref, kz_ref, ux_ref, uy_ref, uz_ref, f_ref,
                  out_ref, *, unit_w):
    kx = kx_ref[0]
    ky = ky_ref[0]
    kz = kz_ref[0]
    ux = ux_ref[0]
    uy = uy_ref[0]
    uz = uz_ref[0]
    Sk = kx.shape[0]
    Su = ux.shape[1]
    d2 = (kx - ux) ** 2 + (ky - uy) ** 2 + (kz - uz) ** 2
    iota_k = jax.lax.broadcasted_iota(jnp.int32, (Sk, Su), 0)
    E = jnp.zeros((Sk, Su), jnp.float32)
    ws = []
    idxs = []
    for _ in range(3):
        m = jnp.min(d2, axis=0, keepdims=True)
        i = jnp.min(jnp.where(d2 == m, iota_k, Sk), axis=0, keepdims=True)
        idxs.append(i)
        ws.append(1.0 / (jnp.sqrt(jnp.maximum(m, 0.0)) + 1e-8))
        d2 = jnp.where(iota_k == i, jnp.float32(3.0e38), d2)
    if unit_w:
        for i in idxs:
            E += (iota_k == i).astype(jnp.float32)
    else:
        wsum = ws[0] + ws[1] + ws[2]
        for w, i in zip(ws, idxs):
            E += jnp.where(iota_k == i, w / wsum, 0.0)
    out_ref[0] = jnp.dot(f_ref[0], E, preferred_element_type=jnp.float32)


def _interp3(kx, ky, kz, ux, uy, uz, feats, unit_w=False):
    B, Sk = kx.shape
    Su = ux.shape[1]
    C = feats.shape[1]
    kcol = lambda a: a.reshape(B, Sk, 1)
    urow = lambda a: a.reshape(B, 1, Su)
    return pl.pallas_call(
        functools.partial(_interp3_body, unit_w=unit_w),
        grid=(B,),
        in_specs=[pl.BlockSpec((1, Sk, 1), lambda b_: (b_, 0, 0))] * 3
        + [pl.BlockSpec((1, 1, Su), lambda b_: (b_, 0, 0))] * 3
        + [pl.BlockSpec((1, C, Sk), lambda b_: (b_, 0, 0))],
        out_specs=pl.BlockSpec((1, C, Su), lambda b_: (b_, 0, 0)),
        out_shape=jax.ShapeDtypeStruct((B, C, Su), jnp.float32),
    )(kcol(kx), kcol(ky), kcol(kz), urow(ux), urow(uy), urow(uz), feats)


_SA1_RAD = ((0.1, 32, 48), (0.2, 64, 80), (0.4, 128, 144))


def _bq_sa1(x, y, z, cx, cy, cz):
    B, N = x.shape
    S = cx.shape[1]
    NW = 32
    RPW = B * S // NW
    WPB = NW // B
    SPW = S // WPB
    NJ = N // 16
    mesh = plsc.VectorSubcoreMesh(core_axis_name="c", subcore_axis_name="s")
    out_type = []
    for (_, ns, kp) in _SA1_RAD:
        out_type += [jax.ShapeDtypeStruct((B * S * kp,), jnp.float32)] * 3
        out_type += [jax.ShapeDtypeStruct((B * S,), jnp.int32)]
    scratch = ([pltpu.VMEM((N,), jnp.float32)] * 3
               + [pltpu.VMEM((RPW,), jnp.float32)] * 3)
    for (_, ns, kp) in _SA1_RAD:
        scratch += [pltpu.VMEM((16 * kp,), jnp.float32)] * 3
        scratch += [pltpu.VMEM((16,), jnp.int32)]

    @functools.partial(
        pl.kernel, mesh=mesh, out_type=out_type, scratch_types=scratch,
        compiler_params=pltpu.CompilerParams(needs_layout_passes=False))
    def k(xh, yh, zh, cxh, cyh, czh, *refs):
        outs = refs[:12]
        xv, yv, zv, ccx, ccy, ccz = refs[12:18]
        bufs = refs[18:]
        wid = lax.axis_index("s") * 2 + lax.axis_index("c")
        b = wid // WPB
        s0 = pl.multiple_of((wid % WPB) * SPW, SPW)
        pltpu.sync_copy(xh.at[b], xv)
        pltpu.sync_copy(yh.at[b], yv)
        pltpu.sync_copy(zh.at[b], zv)
        pltpu.sync_copy(cxh.at[b, pl.ds(s0, SPW)], ccx)
        pltpu.sync_copy(cyh.at[b, pl.ds(s0, SPW)], ccy)
        pltpu.sync_copy(czh.at[b, pl.ds(s0, SPW)], ccz)
        lane = lax.iota(jnp.int32, 16)

        def group_body(g, _):
            g16 = pl.multiple_of(g * 16, 16)
            cxg = ccx[pl.ds(g16, 16)]
            cyg = ccy[pl.ds(g16, 16)]
            czg = ccz[pl.ds(g16, 16)]
            cntv = [jnp.zeros((16,), jnp.int32) for _ in _SA1_RAD]
            for ic in range(16):
                cxb = jnp.full((16,), cxg[ic], jnp.float32)
                cyb = jnp.full((16,), cyg[ic], jnp.float32)
                czb = jnp.full((16,), czg[ic], jnp.float32)

                def pt_body(j, cnts, ic=ic, cxb=cxb, cyb=cyb, czb=czb):
                    px = xv[pl.ds(j * 16, 16)]
                    py = yv[pl.ds(j * 16, 16)]
                    pz = zv[pl.ds(j * 16, 16)]
                    dx = px - cxb
                    dy = py - cyb
                    dz = pz - czb
                    d2 = dx * dx + dy * dy + dz * dz
                    new = []
                    for ri, (r, ns, kp) in enumerate(_SA1_RAD):
                        cnt = cnts[ri]
                        m = d2 <= r * r
                        mi = m.astype(jnp.int32)
                        incl = jnp.cumsum(mi)
                        tgt = (incl - mi) + (ic * kp + jnp.minimum(cnt, ns))
                        plsc.store_scatter(bufs[4 * ri + 0], [tgt], px,
                                           mask=m)
                        plsc.store_scatter(bufs[4 * ri + 1], [tgt], py,
                                           mask=m)
                        plsc.store_scatter(bufs[4 * ri + 2], [tgt], pz,
                                           mask=m)
                        new.append(cnt + plsc.all_reduce_population_count(m))
                    return tuple(new)

                zero16 = jnp.zeros((16,), jnp.int32)
                cnts = lax.fori_loop(0, NJ, pt_body, (zero16, zero16,
                                                      zero16), unroll=4)
                for ri, (r, ns, kp) in enumerate(_SA1_RAD):
                    cntv[ri] = cntv[ri] + jnp.where(
                        lane == ic, jnp.minimum(cnts[ri], ns), 0)

            row0 = pl.multiple_of(wid * RPW + g16, 16)
            for ri, (r, ns, kp) in enumerate(_SA1_RAD):
                cb = bufs[4 * ri + 3]
                cb[...] = cntv[ri]
                for ch in range(3):
                    pltpu.sync_copy(
                        bufs[4 * ri + ch],
                        outs[4 * ri + ch].at[
                            pl.ds(pl.multiple_of(row0 * kp, 16 * kp),
                                  16 * kp)])
                pltpu.sync_copy(cb, outs[4 * ri + 3].at[pl.ds(row0, 16)])
            return 0

        lax.fori_loop(0, RPW // 16, group_body, 0)

    res = k(x, y, z, cx, cy, cz)
    groups = []
    for ri, (r, ns, kp) in enumerate(_SA1_RAD):
        ox, oy, oz, cnt = res[4 * ri:4 * ri + 4]
        groups.append((ox.reshape(B, S, kp), oy.reshape(B, S, kp),
                       oz.reshape(B, S, kp), cnt.reshape(B, 1, S)))
    return groups


_SA2_RAD = ((0.4, 64, 80), (0.8, 128, 144))


def _bq_sa2(x, y, z, cx, cy, cz):
    B, N = x.shape
    S = cx.shape[1]
    NW = 32
    RPW = B * S // NW
    WPB = NW // B
    SPW = S // WPB
    NJ = N // 16
    mesh = plsc.VectorSubcoreMesh(core_axis_name="c", subcore_axis_name="s")
    out_type = []
    for (_, ns, kp) in _SA2_RAD:
        out_type += [jax.ShapeDtypeStruct((B * S * kp,), jnp.int32),
                     jax.ShapeDtypeStruct((B * S,), jnp.int32)]
    scratch = ([pltpu.VMEM((N,), jnp.float32)] * 3
               + [pltpu.VMEM((SPW,), jnp.float32)] * 3)
    for (_, ns, kp) in _SA2_RAD:
        scratch += [pltpu.VMEM((16 * kp,), jnp.int32),
                    pltpu.VMEM((16,), jnp.int32)]

    @functools.partial(
        pl.kernel, mesh=mesh, out_type=out_type, scratch_types=scratch,
        compiler_params=pltpu.CompilerParams(needs_layout_passes=False))
    def k(xh, yh, zh, cxh, cyh, czh, *refs):
        outs = refs[:4]
        xv, yv, zv, ccx, ccy, ccz = refs[4:10]
        bufs = refs[10:]
        wid = lax.axis_index("s") * 2 + lax.axis_index("c")
        b = wid // WPB
        s0 = pl.multiple_of((wid % WPB) * SPW, SPW)
        pltpu.sync_copy(xh.at[b], xv)
        pltpu.sync_copy(yh.at[b], yv)
        pltpu.sync_copy(zh.at[b], zv)
        pltpu.sync_copy(cxh.at[b, pl.ds(s0, SPW)], ccx)
        pltpu.sync_copy(cyh.at[b, pl.ds(s0, SPW)], ccy)
        pltpu.sync_copy(czh.at[b, pl.ds(s0, SPW)], ccz)
        lane = lax.iota(jnp.int32, 16)

        def group_body(g, _):
            g16 = pl.multiple_of(g * 16, 16)
            cxg = ccx[pl.ds(g16, 16)]
            cyg = ccy[pl.ds(g16, 16)]
            czg = ccz[pl.ds(g16, 16)]
            cntv = [jnp.zeros((16,), jnp.int32) for _ in _SA2_RAD]
            for ic in range(16):
                cxb = jnp.full((16,), cxg[ic], jnp.float32)
                cyb = jnp.full((16,), cyg[ic], jnp.float32)
                czb = jnp.full((16,), czg[ic], jnp.float32)

                def pt_body(j, cnts, ic=ic, cxb=cxb, cyb=cyb, czb=czb):
                    px = xv[pl.ds(j * 16, 16)]
                    py = yv[pl.ds(j * 16, 16)]
                    pz = zv[pl.ds(j * 16, 16)]
                    dx = px - cxb
                    dy = py - cyb
                    dz = pz - czb
                    d2 = dx * dx + dy * dy + dz * dz
                    pidx = j * 16 + lane
                    new = []
                    for ri, (r, ns, kp) in enumerate(_SA2_RAD):
                        cnt = cnts[ri]
                        m = d2 <= r * r
                        mi = m.astype(jnp.int32)
                        incl = jnp.cumsum(mi)
                        tgt = (incl - mi) + (ic * kp
                                             + jnp.minimum(cnt, ns))
                        plsc.store_scatter(bufs[2 * ri], [tgt], pidx,
                                           mask=m)
                        new.append(cnt + plsc.all_reduce_population_count(m))
                    return tuple(new)

                zero16 = jnp.zeros((16,), jnp.int32)
                cnts = lax.fori_loop(0, NJ, pt_body, (zero16, zero16),
                                     unroll=2)
                for ri, (r, ns, kp) in enumerate(_SA2_RAD):
                    cntv[ri] = cntv[ri] + jnp.where(
                        lane == ic, jnp.minimum(cnts[ri], ns), 0)

            row0 = pl.multiple_of(wid * RPW + g16, 16)
            for ri, (r, ns, kp) in enumerate(_SA2_RAD):
                cb = bufs[2 * ri + 1]
                cb[...] = cntv[ri]
                pltpu.sync_copy(
                    bufs[2 * ri],
                    outs[2 * ri].at[
                        pl.ds(pl.multiple_of(row0 * kp, 16 * kp), 16 * kp)])
                pltpu.sync_copy(cb, outs[2 * ri + 1].at[pl.ds(row0, 16)])
            return 0

        lax.fori_loop(0, RPW // 16, group_body, 0)

    res = k(x, y, z, cx, cy, cz)
    groups = []
    for ri, (r, ns, kp) in enumerate(_SA2_RAD):
        idx, cnt = res[2 * ri:2 * ri + 2]
        groups.append((idx.reshape(B, S, kp), cnt.reshape(B, S)))
    return groups


def _sc_gather(A2, idx, cnt, ns, Np):
    BNp, C = A2.shape
    B, S, KP = idx.shape
    NW = 32
    WPB = 4
    KPW = ns // WPB
    NJ = S // 16
    mesh = plsc.VectorSubcoreMesh(core_axis_name="c", subcore_axis_name="s")
    idx_f = idx.reshape(B, S * KP)
    out_type = jax.ShapeDtypeStruct((B, ns, S, C), jnp.float32)
    scratch = [pltpu.VMEM((S * KP,), jnp.int32),
               pltpu.VMEM((S,), jnp.int32),
               pltpu.VMEM((S,), jnp.int32),
               pltpu.VMEM((S, C), jnp.float32),
               pltpu.SemaphoreType.DMA]

    @functools.partial(
        pl.kernel, mesh=mesh, out_type=out_type, scratch_types=scratch,
        compiler_params=pltpu.CompilerParams(needs_layout_passes=False))
    def k(ah, ih, ch, oh, iv, cv, gidx, rows, sem):
        wid = lax.axis_index("s") * 2 + lax.axis_index("c")
        b = wid // WPB
        k0 = (wid % WPB) * KPW
        pltpu.sync_copy(ih.at[b], iv)
        pltpu.sync_copy(ch.at[b], cv)
        lane = lax.iota(jnp.int32, 16)
        base = b * Np

        def k_body(kk, _):
            kq = k0 + kk
            for jj in range(NJ):
                offs = (jj * 16 + lane) * KP + kq
                raw = plsc.load_gather(iv, [offs])
                cchunk = cv[pl.ds(jj * 16, 16)]
                safe = jnp.minimum(jnp.maximum(raw, 0), Np - 1)
                sel = jnp.where(kq < cchunk, safe, 0) + base
                gidx[pl.ds(jj * 16, 16)] = sel
            pltpu.async_copy(ah.at[gidx], rows, sem).wait()
            pltpu.sync_copy(rows, oh.at[b, kq])
            return 0

        lax.fori_loop(0, KPW, k_body, 0)

    return k(A2, idx_f, cnt)


def _sa2s_body(a_ref, cnt_ref, c_ref, w1xt_ref, *refs, nlayers, K):
    out_ref = refs[-1]
    corr = jnp.dot(c_ref[0], w1xt_ref[...],
                   preferred_element_type=jnp.float32)
    cntcol = cnt_ref[0]
    S = out_ref.shape[2]
    Cout = out_ref.shape[1]

    def body(kq, m):
        ak = a_ref[0, pl.ds(kq, 1)][0]
        vk = (cntcol > kq).astype(jnp.float32)
        x = jnp.maximum(ak - corr, 0.0)
        for i in range(nlayers):
            Wt = refs[2 * i][...]
            bt = refs[2 * i + 1][...]
            x = jnp.maximum(jnp.dot(x, Wt,
                                    preferred_element_type=jnp.float32)
                            + bt, 0.0)
        return jnp.maximum(m, x * vk)

    m = jax.lax.fori_loop(0, K, body, jnp.zeros((S, Cout), jnp.float32),
                          unroll=4)
    out_ref[0] = jnp.transpose(m)


def _sa2_branch_s(A4, cnt, centers_t, W1x, layers):
    B, K, S, C1 = A4.shape
    nlayers = len(layers)
    Cout = layers[-1][0].shape[0]
    wargs = []
    wspecs = []
    for (W, b) in layers:
        Wt = jnp.transpose(W)
        bt = jnp.transpose(b)
        wargs += [Wt, bt]
        wspecs += [pl.BlockSpec(Wt.shape, lambda b_: (0, 0)),
                   pl.BlockSpec(bt.shape, lambda b_: (0, 0))]
    W1xt = jnp.transpose(W1x)
    return pl.pallas_call(
        functools.partial(_sa2s_body, nlayers=nlayers, K=K),
        grid=(B,),
        in_specs=[pl.BlockSpec((1, K, S, C1), lambda b_: (b_, 0, 0, 0)),
                  pl.BlockSpec((1, S, 1), lambda b_: (b_, 0, 0)),
                  pl.BlockSpec((1, S, 3), lambda b_: (b_, 0, 0)),
                  pl.BlockSpec(W1xt.shape, lambda b_: (0, 0))] + wspecs,
        out_specs=pl.BlockSpec((1, Cout, S), lambda b_: (b_, 0, 0)),
        out_shape=jax.ShapeDtypeStruct((B, Cout, S), jnp.float32),
    )(A4, cnt, centers_t, W1xt, *wargs)


def _bq_fix_body(ox_ref, oy_ref, oz_ref, cnt_ref,
                 gx_ref, gy_ref, gz_ref, v_ref, *, K):
    S = cnt_ref.shape[2]
    cnt = cnt_ref[0]
    iota_k = lax.broadcasted_iota(jnp.int32, (K, S), 0)
    vm = iota_k < cnt
    for src, dst in ((ox_ref, gx_ref), (oy_ref, gy_ref), (oz_ref, gz_ref)):
        t = jnp.transpose(src[0])[:K]
        dst[0] = jnp.where(vm, t, 0.0)
    v_ref[0] = vm.astype(jnp.float32)


def _bq_fix(ox, oy, oz, cnt, K):
    B, S, KP = ox.shape
    ospec = pl.BlockSpec((1, S, KP), lambda b_: (b_, 0, 0))
    gspec = pl.BlockSpec((1, K, S), lambda b_: (b_, 0, 0))
    return pl.pallas_call(
        functools.partial(_bq_fix_body, K=K),
        grid=(B,),
        in_specs=[ospec] * 3 + [pl.BlockSpec((1, 1, S), lambda b_: (b_, 0, 0))],
        out_specs=[gspec] * 4,
        out_shape=[jax.ShapeDtypeStruct((B, K, S), jnp.float32)] * 4,
    )(ox, oy, oz, cnt)


def kernel(xyz, params):
    B, N, _ = xyz.shape

    sa1_layers = [_fold_layers(ls) for ls in params['sa1']]
    sa2_layers = [_fold_layers(ls) for ls in params['sa2']]
    sa3_layers = _fold_layers(params['sa3'])
    fp3_layers = _fold_layers(params['fp3'])
    fp2_layers = _fold_layers(params['fp2'])

    c1x, c1y, c1z, c2x, c2y, c2z = _fps(xyz)
    c1 = jnp.stack([c1x, c1y, c1z], axis=1)
    c2 = jnp.stack([c2x, c2y, c2z], axis=1)

    groups = _bq_sa1(xyz[:, :, 0], xyz[:, :, 1], xyz[:, :, 2],
                     c1x, c1y, c1z)
    outs1 = []
    for (ox, oy, oz, cnt), (r, ns, kp), layers in zip(groups, _SA1_RAD,
                                                      sa1_layers):
        gx, gy, gz, valid = _bq_fix(ox, oy, oz, cnt, ns)
        outs1.append(_sa1_branch(gx, gy, gz, valid, c1, layers))
    l1_points = jnp.concatenate(outs1, axis=1)

    src2 = jnp.concatenate([c1, l1_points], axis=1)
    groups2 = _bq_sa2(c1x, c1y, c1z, c2x, c2y, c2z)
    c2t = jnp.transpose(c2, (0, 2, 1))
    outs2 = []
    for (idx, cnt), (r, ns, kp), layers in zip(groups2, _SA2_RAD,
                                               sa2_layers):
        (W1, b1) = layers[0]
        A = _mlp(src2, [(W1, b1)], relus=(False,))
        A2 = jnp.transpose(A, (0, 2, 1)).reshape(B * 512, 128)
        A4 = _sc_gather(A2, idx, cnt, ns, 512)
        outs2.append(_sa2_branch_s(A4, cnt.reshape(B, 128, 1), c2t,
                                   W1[:, :3], layers[1:]))
    l2_points = jnp.concatenate(outs2, axis=1)

    g3 = jnp.concatenate([c2, l2_points], axis=1)
    l3 = _mlp(g3, sa3_layers, pool=True)

    interp3 = jnp.broadcast_to(l3, (B, 1024, 128))
    f3 = jnp.concatenate([interp3, l2_points], axis=1)
    l2f = _mlp(f3, fp3_layers)

    interp2 = _interp3(c2x, c2y, c2z, c1x, c1y, c1z, l2f)
    f2 = jnp.concatenate([interp2, l1_points], axis=1)
    l1f = _mlp(f2, fp2_layers)

    p = params['conv1']
    s1 = p['g1'] * _BN_S
    W1 = p['W1'] * s1[:, None]
    b1 = (p['b1'] * s1 + p['be1'])[:, None]
    W2 = p['W2']
    b2 = p['b2'][:, None]
    seg, obj, bck = _head(l1f, W1, b1, W2, b2)

    final_seg = _interp3(c1x, c1y, c1z,
                         xyz[:, :, 0], xyz[:, :, 1], xyz[:, :, 2],
                         seg, unit_w=True)

    return (seg, l1f, jnp.squeeze(obj, -1), jnp.squeeze(bck, -1), final_seg)


# speedup vs baseline: 29.0121x; 1.1970x over previous
import functools

import jax
import jax.numpy as jnp
import numpy as np
from jax import lax
from jax.experimental import pallas as pl
from jax.experimental.pallas import tpu as pltpu
from jax.experimental.pallas import tpu_sc as plsc

_BN_S = 1.0 / np.sqrt(1.0 + 1e-5)


def _fold_layers(layers):
    out = []
    for (W, b, g, be) in layers:
        s = g * _BN_S
        out.append((W * s[:, None], (b * s + be)[:, None]))
    return out


def _mlp_body(x_ref, *refs, nlayers, relus, pool):
    out_ref = refs[-1]
    x = x_ref[0]
    for i in range(nlayers):
        W = refs[2 * i][...]
        b = refs[2 * i + 1][...]
        x = jnp.dot(W, x, preferred_element_type=jnp.float32) + b
        if relus[i]:
            x = jnp.maximum(x, 0.0)
    if pool:
        out_ref[0] = jnp.max(x, axis=-1, keepdims=True)
    else:
        out_ref[0] = x


def _mlp(x, layers, relus=None, pool=False):
    B, Cin, S = x.shape
    nlayers = len(layers)
    if relus is None:
        relus = (True,) * nlayers
    Cout = layers[-1][0].shape[0]
    Sout = 1 if pool else S
    wargs = []
    wspecs = []
    for (W, b) in layers:
        wargs += [W, b]
        wspecs += [pl.BlockSpec(W.shape, lambda b_: (0, 0)),
                   pl.BlockSpec(b.shape, lambda b_: (0, 0))]
    return pl.pallas_call(
        functools.partial(_mlp_body, nlayers=nlayers, relus=tuple(relus),
                          pool=pool),
        grid=(B,),
        in_specs=[pl.BlockSpec((1, Cin, S), lambda b_: (b_, 0, 0))] + wspecs,
        out_specs=pl.BlockSpec((1, Cout, Sout), lambda b_: (b_, 0, 0)),
        out_shape=jax.ShapeDtypeStruct((B, Cout, Sout), jnp.float32),
    )(x, *wargs)


def _sa1_body(gx_ref, gy_ref, gz_ref, v_ref, c_ref, *refs, nlayers, K):
    out_ref = refs[-1]
    c3 = c_ref[0]
    cx, cy, cz = c3[0:1], c3[1:2], c3[2:3]
    Cout = out_ref.shape[1]
    S = out_ref.shape[2]

    def body(k, m):
        xk = gx_ref[0, pl.ds(k, 1), :]
        yk = gy_ref[0, pl.ds(k, 1), :]
        zk = gz_ref[0, pl.ds(k, 1), :]
        vk = v_ref[0, pl.ds(k, 1), :]
        x = jnp.concatenate([xk - cx, yk - cy, zk - cz, xk, yk, zk], axis=0)
        for i in range(nlayers):
            W = refs[2 * i][...]
            b = refs[2 * i + 1][...]
            x = jnp.maximum(jnp.dot(W, x, preferred_element_type=jnp.float32)
                            + b, 0.0)
        return jnp.maximum(m, x * vk)

    out_ref[0] = jax.lax.fori_loop(0, K, body, jnp.zeros((Cout, S),
                                                         jnp.float32),
                                   unroll=4)


def _sa1_branch(gx, gy, gz, valid, centers, layers):
    B, K, S = gx.shape
    nlayers = len(layers)
    Cout = layers[-1][0].shape[0]
    wargs = []
    wspecs = []
    for (W, b) in layers:
        wargs += [W, b]
        wspecs += [pl.BlockSpec(W.shape, lambda b_: (0, 0)),
                   pl.BlockSpec(b.shape, lambda b_: (0, 0))]
    gspec = pl.BlockSpec((1, K, S), lambda b_: (b_, 0, 0))
    return pl.pallas_call(
        functools.partial(_sa1_body, nlayers=nlayers, K=K),
        grid=(B,),
        in_specs=[gspec, gspec, gspec, gspec,
                  pl.BlockSpec((1, 3, S), lambda b_: (b_, 0, 0))] + wspecs,
        out_specs=pl.BlockSpec((1, Cout, S), lambda b_: (b_, 0, 0)),
        out_shape=jax.ShapeDtypeStruct((B, Cout, S), jnp.float32),
    )(gx, gy, gz, valid, centers, *wargs)


def _head_body(x_ref, w1_ref, b1_ref, w2_ref, b2_ref, seg_ref, obj_ref,
               bck_ref):
    x = x_ref[0]
    h = jnp.maximum(jnp.dot(w1_ref[...], x,
                            preferred_element_type=jnp.float32)
                    + b1_ref[...], 0.0)
    z = jnp.dot(w2_ref[...], h, preferred_element_type=jnp.float32) \
        + b2_ref[...]
    s = 1.0 / (1.0 + jnp.exp(-z))
    seg_ref[0] = s
    obj_ref[0] = jnp.max(s * x, axis=-1, keepdims=True)
    bck_ref[0] = jnp.max((1.0 - s) * x, axis=-1, keepdims=True)


def _head(x, W1, b1, W2, b2):
    B, C, S = x.shape
    return pl.pallas_call(
        _head_body,
        grid=(B,),
        in_specs=[pl.BlockSpec((1, C, S), lambda b_: (b_, 0, 0)),
                  pl.BlockSpec(W1.shape, lambda b_: (0, 0)),
                  pl.BlockSpec(b1.shape, lambda b_: (0, 0)),
                  pl.BlockSpec(W2.shape, lambda b_: (0, 0)),
                  pl.BlockSpec(b2.shape, lambda b_: (0, 0))],
        out_specs=[pl.BlockSpec((1, 1, S), lambda b_: (b_, 0, 0)),
                   pl.BlockSpec((1, C, 1), lambda b_: (b_, 0, 0)),
                   pl.BlockSpec((1, C, 1), lambda b_: (b_, 0, 0))],
        out_shape=[jax.ShapeDtypeStruct((B, 1, S), jnp.float32),
                   jax.ShapeDtypeStruct((B, C, 1), jnp.float32),
                   jax.ShapeDtypeStruct((B, C, 1), jnp.float32)],
    )(x, W1, b1, W2, b2)


def _fps_level(x, y, z, S, cx_ref, cy_ref, cz_ref):
    B, N = x.shape
    iota_n = jax.lax.broadcasted_iota(jnp.int32, (B, N), 1)
    iota_s = jax.lax.broadcasted_iota(jnp.int32, (B, S), 1)
    cx_ref[...] = jnp.zeros((B, S), jnp.float32)
    cy_ref[...] = jnp.zeros((B, S), jnp.float32)
    cz_ref[...] = jnp.zeros((B, S), jnp.float32)

    def body(t, carry):
        dist, far = carry
        sel = (iota_n == far).astype(jnp.float32)
        cx = jnp.sum(x * sel, -1, keepdims=True)
        cy = jnp.sum(y * sel, -1, keepdims=True)
        cz = jnp.sum(z * sel, -1, keepdims=True)
        oh = (iota_s == t).astype(jnp.float32)
        cx_ref[...] += cx * oh
        cy_ref[...] += cy * oh
        cz_ref[...] += cz * oh
        d = (x - cx) ** 2 + (y - cy) ** 2 + (z - cz) ** 2
        dist = jnp.minimum(dist, d)
        m = jnp.max(dist, -1, keepdims=True)
        far = jnp.min(jnp.where(dist == m, iota_n, N), -1, keepdims=True)
        return dist, far

    jax.lax.fori_loop(
        0, S, body,
        (jnp.full((B, N), 1e10, jnp.float32),
         jnp.zeros((B, 1), jnp.int32)))


def _fps_body(x_ref, y_ref, z_ref, cx_ref, cy_ref, cz_ref, *, S):
    _fps_level(x_ref[...], y_ref[...], z_ref[...], S,
               cx_ref, cy_ref, cz_ref)


def _fps_stage(x, y, z, S):
    B, N = x.shape
    full = lambda s: pl.BlockSpec(s, lambda: tuple(0 for _ in s))
    return pl.pallas_call(
        functools.partial(_fps_body, S=S),
        in_specs=[full((B, N))] * 3,
        out_specs=[full((B, S))] * 3,
        out_shape=[jax.ShapeDtypeStruct((B, S), jnp.float32)] * 3,
    )(x, y, z)


def _fps(xyz):
    x = xyz[:, :, 0]
    y = xyz[:, :, 1]
    z = xyz[:, :, 2]
    c1x, c1y, c1z = _fps_stage(x, y, z, 512)
    c2x, c2y, c2z = _fps_stage(c1x, c1y, c1z, 128)
    return c1x, c1y, c1z, c2x, c2y, c2z


def _interp3_body(kx_ref, ky_ref, kz_ref, ux_ref, uy_ref, uz_ref, f_ref,
                  out_ref, *, unit_w):
    kx = kx_ref[0]
    ky = ky_ref[0]
    kz = kz_ref[0]
    ux = ux_ref[0]
    uy = uy_ref[0]
    uz = uz_ref[0]
    Sk = kx.shape[0]
    Su = ux.shape[1]
    d2 = (kx - ux) ** 2 + (ky - uy) ** 2 + (kz - uz) ** 2
    iota_k = jax.lax.broadcasted_iota(jnp.int32, (Sk, Su), 0)
    E = jnp.zeros((Sk, Su), jnp.float32)
    ws = []
    idxs = []
    for _ in range(3):
        m = jnp.min(d2, axis=0, keepdims=True)
        i = jnp.min(jnp.where(d2 == m, iota_k, Sk), axis=0, keepdims=True)
        idxs.append(i)
        ws.append(1.0 / (jnp.sqrt(jnp.maximum(m, 0.0)) + 1e-8))
        d2 = jnp.where(iota_k == i, jnp.float32(3.0e38), d2)
    if unit_w:
        for i in idxs:
            E += (iota_k == i).astype(jnp.float32)
    else:
        wsum = ws[0] + ws[1] + ws[2]
        for w, i in zip(ws, idxs):
            E += jnp.where(iota_k == i, w / wsum, 0.0)
    out_ref[0] = jnp.dot(f_ref[0], E, preferred_element_type=jnp.float32)


def _interp3(kx, ky, kz, ux, uy, uz, feats, unit_w=False):
    B, Sk = kx.shape
    Su = ux.shape[1]
    C = feats.shape[1]
    kcol = lambda a: a.reshape(B, Sk, 1)
    urow = lambda a: a.reshape(B, 1, Su)
    return pl.pallas_call(
        functools.partial(_interp3_body, unit_w=unit_w),
        grid=(B,),
        in_specs=[pl.BlockSpec((1, Sk, 1), lambda b_: (b_, 0, 0))] * 3
        + [pl.BlockSpec((1, 1, Su), lambda b_: (b_, 0, 0))] * 3
        + [pl.BlockSpec((1, C, Sk), lambda b_: (b_, 0, 0))],
        out_specs=pl.BlockSpec((1, C, Su), lambda b_: (b_, 0, 0)),
        out_shape=jax.ShapeDtypeStruct((B, C, Su), jnp.float32),
    )(kcol(kx), kcol(ky), kcol(kz), urow(ux), urow(uy), urow(uz), feats)


_SA1_RAD = ((0.1, 32, 48), (0.2, 64, 80), (0.4, 128, 144))


def _bq_sa1(x, y, z, cx, cy, cz):
    B, N = x.shape
    S = cx.shape[1]
    NW = 32
    RPW = B * S // NW
    WPB = NW // B
    SPW = S // WPB
    NJ = N // 16
    mesh = plsc.VectorSubcoreMesh(core_axis_name="c", subcore_axis_name="s")
    out_type = []
    for (_, ns, kp) in _SA1_RAD:
        out_type += [jax.ShapeDtypeStruct((B * S * kp,), jnp.float32)] * 3
        out_type += [jax.ShapeDtypeStruct((B * S,), jnp.int32)]
    scratch = ([pltpu.VMEM((N,), jnp.float32)] * 3
               + [pltpu.VMEM((RPW,), jnp.float32)] * 3)
    for (_, ns, kp) in _SA1_RAD:
        scratch += [pltpu.VMEM((16 * kp,), jnp.int32)]
        scratch += [pltpu.VMEM((16 * kp,), jnp.float32)] * 3
        scratch += [pltpu.VMEM((16,), jnp.int32)]

    @functools.partial(
        pl.kernel, mesh=mesh, out_type=out_type, scratch_types=scratch,
        compiler_params=pltpu.CompilerParams(needs_layout_passes=False))
    def k(xh, yh, zh, cxh, cyh, czh, *refs):
        outs = refs[:12]
        xv, yv, zv, ccx, ccy, ccz = refs[12:18]
        bufs = refs[18:]
        wid = lax.axis_index("s") * 2 + lax.axis_index("c")
        b = wid // WPB
        s0 = pl.multiple_of((wid % WPB) * SPW, SPW)
        pltpu.sync_copy(xh.at[b], xv)
        pltpu.sync_copy(yh.at[b], yv)
        pltpu.sync_copy(zh.at[b], zv)
        pltpu.sync_copy(cxh.at[b, pl.ds(s0, SPW)], ccx)
        pltpu.sync_copy(cyh.at[b, pl.ds(s0, SPW)], ccy)
        pltpu.sync_copy(czh.at[b, pl.ds(s0, SPW)], ccz)
        lane = lax.iota(jnp.int32, 16)

        kp_base = [lane * kp for (_, ns, kp) in _SA1_RAD]
        zero16 = jnp.zeros((16,), jnp.int32)

        def group_body(g, _):
            g16 = pl.multiple_of(g * 16, 16)
            cxg = ccx[pl.ds(g16, 16)]
            cyg = ccy[pl.ds(g16, 16)]
            czg = ccz[pl.ds(g16, 16)]

            def pt_body(j, cnts):
                pxv = xv[pl.ds(j * 16, 16)]
                pyv = yv[pl.ds(j * 16, 16)]
                pzv = zv[pl.ds(j * 16, 16)]
                new = list(cnts)
                for l in range(16):
                    pxb = jnp.full((16,), pxv[l], jnp.float32)
                    pyb = jnp.full((16,), pyv[l], jnp.float32)
                    pzb = jnp.full((16,), pzv[l], jnp.float32)
                    dx = cxg - pxb
                    dy = cyg - pyb
                    dz = czg - pzb
                    d2 = dx * dx + dy * dy + dz * dz
                    pib = jnp.full((16,), j * 16 + l, jnp.int32)
                    for ri, (r, ns, kp) in enumerate(_SA1_RAD):
                        m = d2 <= r * r
                        tgt = kp_base[ri] + jnp.minimum(new[ri], ns)
                        plsc.store_scatter(bufs[5 * ri], [tgt], pib,
                                           mask=m)
                        new[ri] = new[ri] + m.astype(jnp.int32)
                return tuple(new)

            cnts = lax.fori_loop(0, NJ, pt_body, (zero16, zero16, zero16))

            row0 = pl.multiple_of(wid * RPW + g16, 16)
            for ri, (r, ns, kp) in enumerate(_SA1_RAD):
                ib = bufs[5 * ri]
                bx = bufs[5 * ri + 1]
                by = bufs[5 * ri + 2]
                bz = bufs[5 * ri + 3]
                cb = bufs[5 * ri + 4]

                def flush_body(w, _, ib=ib, bx=bx, by=by, bz=bz):
                    iv = ib[pl.ds(w * 16, 16)]
                    ivc = jnp.minimum(jnp.maximum(iv, 0), N - 1)
                    bx[pl.ds(w * 16, 16)] = plsc.load_gather(xv, [ivc])
                    by[pl.ds(w * 16, 16)] = plsc.load_gather(yv, [ivc])
                    bz[pl.ds(w * 16, 16)] = plsc.load_gather(zv, [ivc])
                    return 0

                lax.fori_loop(0, kp, flush_body, 0)
                cb[...] = jnp.minimum(cnts[ri], ns)
                for ch, buf in enumerate((bx, by, bz)):
                    pltpu.sync_copy(
                        buf,
                        outs[4 * ri + ch].at[
                            pl.ds(pl.multiple_of(row0 * kp, 16 * kp),
                                  16 * kp)])
                pltpu.sync_copy(cb, outs[4 * ri + 3].at[pl.ds(row0, 16)])
            return 0

        lax.fori_loop(0, RPW // 16, group_body, 0)

    res = k(x, y, z, cx, cy, cz)
    groups = []
    for ri, (r, ns, kp) in enumerate(_SA1_RAD):
        ox, oy, oz, cnt = res[4 * ri:4 * ri + 4]
        groups.append((ox.reshape(B, S, kp), oy.reshape(B, S, kp),
                       oz.reshape(B, S, kp), cnt.reshape(B, 1, S)))
    return groups


_SA2_RAD = ((0.4, 64, 80), (0.8, 128, 144))


def _bq_sa2(x, y, z, cx, cy, cz):
    B, N = x.shape
    S = cx.shape[1]
    NW = 32
    RPW = B * S // NW
    WPB = NW // B
    SPW = S // WPB
    NJ = N // 16
    mesh = plsc.VectorSubcoreMesh(core_axis_name="c", subcore_axis_name="s")
    out_type = []
    for (_, ns, kp) in _SA2_RAD:
        out_type += [jax.ShapeDtypeStruct((B * S * kp,), jnp.int32),
                     jax.ShapeDtypeStruct((B * S,), jnp.int32)]
    scratch = ([pltpu.VMEM((N,), jnp.float32)] * 3
               + [pltpu.VMEM((SPW,), jnp.float32)] * 3)
    for (_, ns, kp) in _SA2_RAD:
        scratch += [pltpu.VMEM((16 * kp,), jnp.int32),
                    pltpu.VMEM((16,), jnp.int32)]

    @functools.partial(
        pl.kernel, mesh=mesh, out_type=out_type, scratch_types=scratch,
        compiler_params=pltpu.CompilerParams(needs_layout_passes=False))
    def k(xh, yh, zh, cxh, cyh, czh, *refs):
        outs = refs[:4]
        xv, yv, zv, ccx, ccy, ccz = refs[4:10]
        bufs = refs[10:]
        wid = lax.axis_index("s") * 2 + lax.axis_index("c")
        b = wid // WPB
        s0 = pl.multiple_of((wid % WPB) * SPW, SPW)
        pltpu.sync_copy(xh.at[b], xv)
        pltpu.sync_copy(yh.at[b], yv)
        pltpu.sync_copy(zh.at[b], zv)
        pltpu.sync_copy(cxh.at[b, pl.ds(s0, SPW)], ccx)
        pltpu.sync_copy(cyh.at[b, pl.ds(s0, SPW)], ccy)
        pltpu.sync_copy(czh.at[b, pl.ds(s0, SPW)], ccz)
        lane = lax.iota(jnp.int32, 16)

        kp_base = [lane * kp for (_, ns, kp) in _SA2_RAD]
        zero16 = jnp.zeros((16,), jnp.int32)

        def group_body(g, _):
            g16 = pl.multiple_of(g * 16, 16)
            cxg = ccx[pl.ds(g16, 16)]
            cyg = ccy[pl.ds(g16, 16)]
            czg = ccz[pl.ds(g16, 16)]

            def pt_body(j, cnts):
                pxv = xv[pl.ds(j * 16, 16)]
                pyv = yv[pl.ds(j * 16, 16)]
                pzv = zv[pl.ds(j * 16, 16)]
                new = list(cnts)
                for l in range(16):
                    pxb = jnp.full((16,), pxv[l], jnp.float32)
                    pyb = jnp.full((16,), pyv[l], jnp.float32)
                    pzb = jnp.full((16,), pzv[l], jnp.float32)
                    dx = cxg - pxb
                    dy = cyg - pyb
                    dz = czg - pzb
                    d2 = dx * dx + dy * dy + dz * dz
                    pib = jnp.full((16,), j * 16 + l, jnp.int32)
                    for ri, (r, ns, kp) in enumerate(_SA2_RAD):
                        m = d2 <= r * r
                        tgt = kp_base[ri] + jnp.minimum(new[ri], ns)
                        plsc.store_scatter(bufs[2 * ri], [tgt], pib,
                                           mask=m)
                        new[ri] = new[ri] + m.astype(jnp.int32)
                return tuple(new)

            cnts = lax.fori_loop(0, NJ, pt_body, (zero16, zero16))

            row0 = pl.multiple_of(wid * RPW + g16, 16)
            for ri, (r, ns, kp) in enumerate(_SA2_RAD):
                cb = bufs[2 * ri + 1]
                cb[...] = jnp.minimum(cnts[ri], ns)
                pltpu.sync_copy(
                    bufs[2 * ri],
                    outs[2 * ri].at[
                        pl.ds(pl.multiple_of(row0 * kp, 16 * kp), 16 * kp)])
                pltpu.sync_copy(cb, outs[2 * ri + 1].at[pl.ds(row0, 16)])
            return 0

        lax.fori_loop(0, RPW // 16, group_body, 0)

    res = k(x, y, z, cx, cy, cz)
    groups = []
    for ri, (r, ns, kp) in enumerate(_SA2_RAD):
        idx, cnt = res[2 * ri:2 * ri + 2]
        groups.append((idx.reshape(B, S, kp), cnt.reshape(B, S)))
    return groups


def _sc_gather(A2, idx, cnt, ns, Np):
    BNp, C = A2.shape
    B, S, KP = idx.shape
    NW = 32
    WPB = 4
    KPW = ns // WPB
    NJ = S // 16
    mesh = plsc.VectorSubcoreMesh(core_axis_name="c", subcore_axis_name="s")
    idx_f = idx.reshape(B, S * KP)
    out_type = jax.ShapeDtypeStruct((B, ns, S, C), jnp.float32)
    scratch = [pltpu.VMEM((S * KP,), jnp.int32),
               pltpu.VMEM((S,), jnp.int32),
               pltpu.VMEM((S,), jnp.int32),
               pltpu.VMEM((S, C), jnp.float32),
               pltpu.SemaphoreType.DMA]

    @functools.partial(
        pl.kernel, mesh=mesh, out_type=out_type, scratch_types=scratch,
        compiler_params=pltpu.CompilerParams(needs_layout_passes=False))
    def k(ah, ih, ch, oh, iv, cv, gidx, rows, sem):
        wid = lax.axis_index("s") * 2 + lax.axis_index("c")
        b = wid // WPB
        k0 = (wid % WPB) * KPW
        pltpu.sync_copy(ih.at[b], iv)
        pltpu.sync_copy(ch.at[b], cv)
        lane = lax.iota(jnp.int32, 16)
        base = b * Np

        def k_body(kk, _):
            kq = k0 + kk
            for jj in range(NJ):
                offs = (jj * 16 + lane) * KP + kq
                raw = plsc.load_gather(iv, [offs])
                cchunk = cv[pl.ds(jj * 16, 16)]
                safe = jnp.minimum(jnp.maximum(raw, 0), Np - 1)
                sel = jnp.where(kq < cchunk, safe, 0) + base
                gidx[pl.ds(jj * 16, 16)] = sel
            pltpu.async_copy(ah.at[gidx], rows, sem).wait()
            pltpu.sync_copy(rows, oh.at[b, kq])
            return 0

        lax.fori_loop(0, KPW, k_body, 0)

    return k(A2, idx_f, cnt)


def _sa2s_body(a_ref, cnt_ref, c_ref, w1xt_ref, *refs, nlayers, K):
    out_ref = refs[-1]
    corr = jnp.dot(c_ref[0], w1xt_ref[...],
                   preferred_element_type=jnp.float32)
    cntcol = cnt_ref[0]
    S = out_ref.shape[2]
    Cout = out_ref.shape[1]

    def body(kq, m):
        ak = a_ref[0, pl.ds(kq, 1)][0]
        vk = (cntcol > kq).astype(jnp.float32)
        x = jnp.maximum(ak - corr, 0.0)
        for i in range(nlayers):
            Wt = refs[2 * i][...]
            bt = refs[2 * i + 1][...]
            x = jnp.maximum(jnp.dot(x, Wt,
                                    preferred_element_type=jnp.float32)
                            + bt, 0.0)
        return jnp.maximum(m, x * vk)

    m = jax.lax.fori_loop(0, K, body, jnp.zeros((S, Cout), jnp.float32),
                          unroll=4)
    out_ref[0] = jnp.transpose(m)


def _sa2_branch_s(A4, cnt, centers_t, W1x, layers):
    B, K, S, C1 = A4.shape
    nlayers = len(layers)
    Cout = layers[-1][0].shape[0]
    wargs = []
    wspecs = []
    for (W, b) in layers:
        Wt = jnp.transpose(W)
        bt = jnp.transpose(b)
        wargs += [Wt, bt]
        wspecs += [pl.BlockSpec(Wt.shape, lambda b_: (0, 0)),
                   pl.BlockSpec(bt.shape, lambda b_: (0, 0))]
    W1xt = jnp.transpose(W1x)
    return pl.pallas_call(
        functools.partial(_sa2s_body, nlayers=nlayers, K=K),
        grid=(B,),
        in_specs=[pl.BlockSpec((1, K, S, C1), lambda b_: (b_, 0, 0, 0)),
                  pl.BlockSpec((1, S, 1), lambda b_: (b_, 0, 0)),
                  pl.BlockSpec((1, S, 3), lambda b_: (b_, 0, 0)),
                  pl.BlockSpec(W1xt.shape, lambda b_: (0, 0))] + wspecs,
        out_specs=pl.BlockSpec((1, Cout, S), lambda b_: (b_, 0, 0)),
        out_shape=jax.ShapeDtypeStruct((B, Cout, S), jnp.float32),
    )(A4, cnt, centers_t, W1xt, *wargs)


def _bq_fix_body(ox_ref, oy_ref, oz_ref, cnt_ref,
                 gx_ref, gy_ref, gz_ref, v_ref, *, K):
    S = cnt_ref.shape[2]
    cnt = cnt_ref[0]
    iota_k = lax.broadcasted_iota(jnp.int32, (K, S), 0)
    vm = iota_k < cnt
    for src, dst in ((ox_ref, gx_ref), (oy_ref, gy_ref), (oz_ref, gz_ref)):
        t = jnp.transpose(src[0])[:K]
        dst[0] = jnp.where(vm, t, 0.0)
    v_ref[0] = vm.astype(jnp.float32)


def _bq_fix(ox, oy, oz, cnt, K):
    B, S, KP = ox.shape
    ospec = pl.BlockSpec((1, S, KP), lambda b_: (b_, 0, 0))
    gspec = pl.BlockSpec((1, K, S), lambda b_: (b_, 0, 0))
    return pl.pallas_call(
        functools.partial(_bq_fix_body, K=K),
        grid=(B,),
        in_specs=[ospec] * 3 + [pl.BlockSpec((1, 1, S), lambda b_: (b_, 0, 0))],
        out_specs=[gspec] * 4,
        out_shape=[jax.ShapeDtypeStruct((B, K, S), jnp.float32)] * 4,
    )(ox, oy, oz, cnt)


def kernel(xyz, params):
    B, N, _ = xyz.shape

    sa1_layers = [_fold_layers(ls) for ls in params['sa1']]
    sa2_layers = [_fold_layers(ls) for ls in params['sa2']]
    sa3_layers = _fold_layers(params['sa3'])
    fp3_layers = _fold_layers(params['fp3'])
    fp2_layers = _fold_layers(params['fp2'])

    c1x, c1y, c1z, c2x, c2y, c2z = _fps(xyz)
    c1 = jnp.stack([c1x, c1y, c1z], axis=1)
    c2 = jnp.stack([c2x, c2y, c2z], axis=1)

    groups = _bq_sa1(xyz[:, :, 0], xyz[:, :, 1], xyz[:, :, 2],
                     c1x, c1y, c1z)
    outs1 = []
    for (ox, oy, oz, cnt), (r, ns, kp), layers in zip(groups, _SA1_RAD,
                                                      sa1_layers):
        gx, gy, gz, valid = _bq_fix(ox, oy, oz, cnt, ns)
        outs1.append(_sa1_branch(gx, gy, gz, valid, c1, layers))
    l1_points = jnp.concatenate(outs1, axis=1)

    src2 = jnp.concatenate([c1, l1_points], axis=1)
    groups2 = _bq_sa2(c1x, c1y, c1z, c2x, c2y, c2z)
    c2t = jnp.transpose(c2, (0, 2, 1))
    outs2 = []
    for (idx, cnt), (r, ns, kp), layers in zip(groups2, _SA2_RAD,
                                               sa2_layers):
        (W1, b1) = layers[0]
        A = _mlp(src2, [(W1, b1)], relus=(False,))
        A2 = jnp.transpose(A, (0, 2, 1)).reshape(B * 512, 128)
        A4 = _sc_gather(A2, idx, cnt, ns, 512)
        outs2.append(_sa2_branch_s(A4, cnt.reshape(B, 128, 1), c2t,
                                   W1[:, :3], layers[1:]))
    l2_points = jnp.concatenate(outs2, axis=1)

    g3 = jnp.concatenate([c2, l2_points], axis=1)
    l3 = _mlp(g3, sa3_layers, pool=True)

    interp3 = jnp.broadcast_to(l3, (B, 1024, 128))
    f3 = jnp.concatenate([interp3, l2_points], axis=1)
    l2f = _mlp(f3, fp3_layers)

    interp2 = _interp3(c2x, c2y, c2z, c1x, c1y, c1z, l2f)
    f2 = jnp.concatenate([interp2, l1_points], axis=1)
    l1f = _mlp(f2, fp2_layers)

    p = params['conv1']
    s1 = p['g1'] * _BN_S
    W1 = p['W1'] * s1[:, None]
    b1 = (p['b1'] * s1 + p['be1'])[:, None]
    W2 = p['W2']
    b2 = p['b2'][:, None]
    seg, obj, bck = _head(l1f, W1, b1, W2, b2)

    final_seg = _interp3(c1x, c1y, c1z,
                         xyz[:, :, 0], xyz[:, :, 1], xyz[:, :, 2],
                         seg, unit_w=True)

    return (seg, l1f, jnp.squeeze(obj, -1), jnp.squeeze(bck, -1), final_seg)


# speedup vs baseline: 30.4605x; 1.0499x over previous
import functools

import jax
import jax.numpy as jnp
import numpy as np
from jax import lax
from jax.experimental import pallas as pl
from jax.experimental.pallas import tpu as pltpu
from jax.experimental.pallas import tpu_sc as plsc

_BN_S = 1.0 / np.sqrt(1.0 + 1e-5)


def _fold_layers(layers):
    out = []
    for (W, b, g, be) in layers:
        s = g * _BN_S
        out.append((W * s[:, None], (b * s + be)[:, None]))
    return out


def _mlp_body(x_ref, *refs, nlayers, relus, pool):
    out_ref = refs[-1]
    x = x_ref[0]
    for i in range(nlayers):
        W = refs[2 * i][...]
        b = refs[2 * i + 1][...]
        x = jnp.dot(W, x, preferred_element_type=jnp.float32) + b
        if relus[i]:
            x = jnp.maximum(x, 0.0)
    if pool:
        out_ref[0] = jnp.max(x, axis=-1, keepdims=True)
    else:
        out_ref[0] = x


def _mlp(x, layers, relus=None, pool=False):
    B, Cin, S = x.shape
    nlayers = len(layers)
    if relus is None:
        relus = (True,) * nlayers
    Cout = layers[-1][0].shape[0]
    Sout = 1 if pool else S
    wargs = []
    wspecs = []
    for (W, b) in layers:
        wargs += [W, b]
        wspecs += [pl.BlockSpec(W.shape, lambda b_: (0, 0)),
                   pl.BlockSpec(b.shape, lambda b_: (0, 0))]
    return pl.pallas_call(
        functools.partial(_mlp_body, nlayers=nlayers, relus=tuple(relus),
                          pool=pool),
        grid=(B,),
        in_specs=[pl.BlockSpec((1, Cin, S), lambda b_: (b_, 0, 0))] + wspecs,
        out_specs=pl.BlockSpec((1, Cout, Sout), lambda b_: (b_, 0, 0)),
        out_shape=jax.ShapeDtypeStruct((B, Cout, Sout), jnp.float32),
    )(x, *wargs)


def _sa1_body(gx_ref, gy_ref, gz_ref, v_ref, c_ref, *refs, nlayers, K):
    out_ref = refs[-1]
    c3 = c_ref[0]
    cx, cy, cz = c3[0:1], c3[1:2], c3[2:3]
    Cout = out_ref.shape[1]
    S = out_ref.shape[2]

    def body(k, m):
        xk = gx_ref[0, pl.ds(k, 1), :]
        yk = gy_ref[0, pl.ds(k, 1), :]
        zk = gz_ref[0, pl.ds(k, 1), :]
        vk = v_ref[0, pl.ds(k, 1), :]
        x = jnp.concatenate([xk - cx, yk - cy, zk - cz, xk, yk, zk], axis=0)
        for i in range(nlayers):
            W = refs[2 * i][...]
            b = refs[2 * i + 1][...]
            x = jnp.maximum(jnp.dot(W, x, preferred_element_type=jnp.float32)
                            + b, 0.0)
        return jnp.maximum(m, x * vk)

    out_ref[0] = jax.lax.fori_loop(0, K, body, jnp.zeros((Cout, S),
                                                         jnp.float32),
                                   unroll=8)


def _sa1_branch(gx, gy, gz, valid, centers, layers):
    B, K, S = gx.shape
    nlayers = len(layers)
    Cout = layers[-1][0].shape[0]
    wargs = []
    wspecs = []
    for (W, b) in layers:
        wargs += [W, b]
        wspecs += [pl.BlockSpec(W.shape, lambda b_: (0, 0)),
                   pl.BlockSpec(b.shape, lambda b_: (0, 0))]
    gspec = pl.BlockSpec((1, K, S), lambda b_: (b_, 0, 0))
    return pl.pallas_call(
        functools.partial(_sa1_body, nlayers=nlayers, K=K),
        grid=(B,),
        in_specs=[gspec, gspec, gspec, gspec,
                  pl.BlockSpec((1, 3, S), lambda b_: (b_, 0, 0))] + wspecs,
        out_specs=pl.BlockSpec((1, Cout, S), lambda b_: (b_, 0, 0)),
        out_shape=jax.ShapeDtypeStruct((B, Cout, S), jnp.float32),
    )(gx, gy, gz, valid, centers, *wargs)


def _head_body(x_ref, w1_ref, b1_ref, w2_ref, b2_ref, seg_ref, obj_ref,
               bck_ref):
    x = x_ref[0]
    h = jnp.maximum(jnp.dot(w1_ref[...], x,
                            preferred_element_type=jnp.float32)
                    + b1_ref[...], 0.0)
    z = jnp.dot(w2_ref[...], h, preferred_element_type=jnp.float32) \
        + b2_ref[...]
    s = 1.0 / (1.0 + jnp.exp(-z))
    seg_ref[0] = s
    obj_ref[0] = jnp.max(s * x, axis=-1, keepdims=True)
    bck_ref[0] = jnp.max((1.0 - s) * x, axis=-1, keepdims=True)


def _head(x, W1, b1, W2, b2):
    B, C, S = x.shape
    return pl.pallas_call(
        _head_body,
        grid=(B,),
        in_specs=[pl.BlockSpec((1, C, S), lambda b_: (b_, 0, 0)),
                  pl.BlockSpec(W1.shape, lambda b_: (0, 0)),
                  pl.BlockSpec(b1.shape, lambda b_: (0, 0)),
                  pl.BlockSpec(W2.shape, lambda b_: (0, 0)),
                  pl.BlockSpec(b2.shape, lambda b_: (0, 0))],
        out_specs=[pl.BlockSpec((1, 1, S), lambda b_: (b_, 0, 0)),
                   pl.BlockSpec((1, C, 1), lambda b_: (b_, 0, 0)),
                   pl.BlockSpec((1, C, 1), lambda b_: (b_, 0, 0))],
        out_shape=[jax.ShapeDtypeStruct((B, 1, S), jnp.float32),
                   jax.ShapeDtypeStruct((B, C, 1), jnp.float32),
                   jax.ShapeDtypeStruct((B, C, 1), jnp.float32)],
    )(x, W1, b1, W2, b2)


def _fps_level(x, y, z, S, cx_ref, cy_ref, cz_ref):
    B, N = x.shape
    iota_n = jax.lax.broadcasted_iota(jnp.int32, (B, N), 1)
    iota_s = jax.lax.broadcasted_iota(jnp.int32, (B, S), 1)
    cx_ref[...] = jnp.zeros((B, S), jnp.float32)
    cy_ref[...] = jnp.zeros((B, S), jnp.float32)
    cz_ref[...] = jnp.zeros((B, S), jnp.float32)

    def body(t, carry):
        dist, far = carry
        sel = (iota_n == far).astype(jnp.float32)
        cx = jnp.sum(x * sel, -1, keepdims=True)
        cy = jnp.sum(y * sel, -1, keepdims=True)
        cz = jnp.sum(z * sel, -1, keepdims=True)
        oh = (iota_s == t).astype(jnp.float32)
        cx_ref[...] += cx * oh
        cy_ref[...] += cy * oh
        cz_ref[...] += cz * oh
        d = (x - cx) ** 2 + (y - cy) ** 2 + (z - cz) ** 2
        dist = jnp.minimum(dist, d)
        m = jnp.max(dist, -1, keepdims=True)
        far = jnp.min(jnp.where(dist == m, iota_n, N), -1, keepdims=True)
        return dist, far

    jax.lax.fori_loop(
        0, S, body,
        (jnp.full((B, N), 1e10, jnp.float32),
         jnp.zeros((B, 1), jnp.int32)))


def _fps_body(x_ref, y_ref, z_ref, cx_ref, cy_ref, cz_ref, *, S):
    _fps_level(x_ref[...], y_ref[...], z_ref[...], S,
               cx_ref, cy_ref, cz_ref)


def _fps_stage(x, y, z, S):
    B, N = x.shape
    full = lambda s: pl.BlockSpec(s, lambda: tuple(0 for _ in s))
    return pl.pallas_call(
        functools.partial(_fps_body, S=S),
        in_specs=[full((B, N))] * 3,
        out_specs=[full((B, S))] * 3,
        out_shape=[jax.ShapeDtypeStruct((B, S), jnp.float32)] * 3,
    )(x, y, z)


def _fps(xyz):
    x = xyz[:, :, 0]
    y = xyz[:, :, 1]
    z = xyz[:, :, 2]
    c1x, c1y, c1z = _fps_stage(x, y, z, 512)
    c2x, c2y, c2z = _fps_stage(c1x, c1y, c1z, 128)
    return c1x, c1y, c1z, c2x, c2y, c2z


def _interp3_body(kx_ref, ky_ref, kz_ref, ux_ref, uy_ref, uz_ref, f_ref,
                  out_ref, *, unit_w):
    kx = kx_ref[0]
    ky = ky_ref[0]
    kz = kz_ref[0]
    ux = ux_ref[0]
    uy = uy_ref[0]
    uz = uz_ref[0]
    Sk = kx.shape[0]
    Su = ux.shape[1]
    d2 = (kx - ux) ** 2 + (ky - uy) ** 2 + (kz - uz) ** 2
    iota_k = jax.lax.broadcasted_iota(jnp.int32, (Sk, Su), 0)
    E = jnp.zeros((Sk, Su), jnp.float32)
    ws = []
    idxs = []
    for _ in range(3):
        m = jnp.min(d2, axis=0, keepdims=True)
        i = jnp.min(jnp.where(d2 == m, iota_k, Sk), axis=0, keepdims=True)
        idxs.append(i)
        ws.append(1.0 / (jnp.sqrt(jnp.maximum(m, 0.0)) + 1e-8))
        d2 = jnp.where(iota_k == i, jnp.float32(3.0e38), d2)
    if unit_w:
        for i in idxs:
            E += (iota_k == i).astype(jnp.float32)
    else:
        wsum = ws[0] + ws[1] + ws[2]
        for w, i in zip(ws, idxs):
            E += jnp.where(iota_k == i, w / wsum, 0.0)
    out_ref[0] = jnp.dot(f_ref[0], E, preferred_element_type=jnp.float32)


def _interp3(kx, ky, kz, ux, uy, uz, feats, unit_w=False):
    B, Sk = kx.shape
    Su = ux.shape[1]
    C = feats.shape[1]
    kcol = lambda a: a.reshape(B, Sk, 1)
    urow = lambda a: a.reshape(B, 1, Su)
    return pl.pallas_call(
        functools.partial(_interp3_body, unit_w=unit_w),
        grid=(B,),
        in_specs=[pl.BlockSpec((1, Sk, 1), lambda b_: (b_, 0, 0))] * 3
        + [pl.BlockSpec((1, 1, Su), lambda b_: (b_, 0, 0))] * 3
        + [pl.BlockSpec((1, C, Sk), lambda b_: (b_, 0, 0))],
        out_specs=pl.BlockSpec((1, C, Su), lambda b_: (b_, 0, 0)),
        out_shape=jax.ShapeDtypeStruct((B, C, Su), jnp.float32),
    )(kcol(kx), kcol(ky), kcol(kz), urow(ux), urow(uy), urow(uz), feats)


_SA1_RAD = ((0.1, 32, 48), (0.2, 64, 80), (0.4, 128, 144))


def _bq_sa1(x, y, z, cx, cy, cz):
    B, N = x.shape
    S = cx.shape[1]
    NW = 32
    RPW = B * S // NW
    WPB = NW // B
    SPW = S // WPB
    NJ = N // 16
    mesh = plsc.VectorSubcoreMesh(core_axis_name="c", subcore_axis_name="s")
    out_type = []
    for (_, ns, kp) in _SA1_RAD:
        out_type += [jax.ShapeDtypeStruct((B * S * kp,), jnp.float32)] * 3
        out_type += [jax.ShapeDtypeStruct((B * S,), jnp.int32)]
    scratch = ([pltpu.VMEM((N,), jnp.float32)] * 3
               + [pltpu.VMEM((RPW,), jnp.float32)] * 3)
    for (_, ns, kp) in _SA1_RAD:
        scratch += [pltpu.VMEM((16 * kp,), jnp.int32)]
        scratch += [pltpu.VMEM((16 * kp,), jnp.float32)] * 3
        scratch += [pltpu.VMEM((16,), jnp.int32)]
    scratch += [pltpu.SemaphoreType.DMA]

    @functools.partial(
        pl.kernel, mesh=mesh, out_type=out_type, scratch_types=scratch,
        compiler_params=pltpu.CompilerParams(needs_layout_passes=False))
    def k(xh, yh, zh, cxh, cyh, czh, *refs):
        outs = refs[:12]
        xv, yv, zv, ccx, ccy, ccz = refs[12:18]
        bufs = refs[18:-1]
        dsem = refs[-1]
        wid = lax.axis_index("s") * 2 + lax.axis_index("c")
        b = wid // WPB
        s0 = pl.multiple_of((wid % WPB) * SPW, SPW)
        pltpu.sync_copy(xh.at[b], xv)
        pltpu.sync_copy(yh.at[b], yv)
        pltpu.sync_copy(zh.at[b], zv)
        pltpu.sync_copy(cxh.at[b, pl.ds(s0, SPW)], ccx)
        pltpu.sync_copy(cyh.at[b, pl.ds(s0, SPW)], ccy)
        pltpu.sync_copy(czh.at[b, pl.ds(s0, SPW)], ccz)
        lane = lax.iota(jnp.int32, 16)

        kp_base = [lane * kp for (_, ns, kp) in _SA1_RAD]
        zero16 = jnp.zeros((16,), jnp.int32)

        def group_body(g, _):
            g16 = pl.multiple_of(g * 16, 16)
            cxg = ccx[pl.ds(g16, 16)]
            cyg = ccy[pl.ds(g16, 16)]
            czg = ccz[pl.ds(g16, 16)]

            def pt_body(j, cnts):
                pxv = xv[pl.ds(j * 16, 16)]
                pyv = yv[pl.ds(j * 16, 16)]
                pzv = zv[pl.ds(j * 16, 16)]
                new = list(cnts)
                for l in range(16):
                    pxb = jnp.full((16,), pxv[l], jnp.float32)
                    pyb = jnp.full((16,), pyv[l], jnp.float32)
                    pzb = jnp.full((16,), pzv[l], jnp.float32)
                    dx = cxg - pxb
                    dy = cyg - pyb
                    dz = czg - pzb
                    d2 = dx * dx + dy * dy + dz * dz
                    pib = jnp.full((16,), j * 16 + l, jnp.int32)
                    for ri, (r, ns, kp) in enumerate(_SA1_RAD):
                        m = d2 <= r * r
                        tgt = kp_base[ri] + jnp.minimum(new[ri], ns)
                        plsc.store_scatter(bufs[5 * ri], [tgt], pib,
                                           mask=m)
                        new[ri] = new[ri] + m.astype(jnp.int32)
                return tuple(new)

            cnts = lax.fori_loop(0, NJ, pt_body, (zero16, zero16, zero16))

            row0 = pl.multiple_of(wid * RPW + g16, 16)
            handles = []
            for ri, (r, ns, kp) in enumerate(_SA1_RAD):
                ib = bufs[5 * ri]
                bx = bufs[5 * ri + 1]
                by = bufs[5 * ri + 2]
                bz = bufs[5 * ri + 3]
                cb = bufs[5 * ri + 4]

                def flush_body(w, _, ib=ib, bx=bx, by=by, bz=bz):
                    iv = ib[pl.ds(w * 16, 16)]
                    ivc = jnp.minimum(jnp.maximum(iv, 0), N - 1)
                    bx[pl.ds(w * 16, 16)] = plsc.load_gather(xv, [ivc])
                    by[pl.ds(w * 16, 16)] = plsc.load_gather(yv, [ivc])
                    bz[pl.ds(w * 16, 16)] = plsc.load_gather(zv, [ivc])
                    return 0

                lax.fori_loop(0, kp, flush_body, 0)
                cb[...] = jnp.minimum(cnts[ri], ns)
                for ch, buf in enumerate((bx, by, bz)):
                    handles.append(pltpu.async_copy(
                        buf,
                        outs[4 * ri + ch].at[
                            pl.ds(pl.multiple_of(row0 * kp, 16 * kp),
                                  16 * kp)], dsem))
                handles.append(pltpu.async_copy(
                    cb, outs[4 * ri + 3].at[pl.ds(row0, 16)], dsem))
            for h in handles:
                h.wait()
            return 0

        lax.fori_loop(0, RPW // 16, group_body, 0)

    res = k(x, y, z, cx, cy, cz)
    groups = []
    for ri, (r, ns, kp) in enumerate(_SA1_RAD):
        ox, oy, oz, cnt = res[4 * ri:4 * ri + 4]
        groups.append((ox.reshape(B, S, kp), oy.reshape(B, S, kp),
                       oz.reshape(B, S, kp), cnt.reshape(B, 1, S)))
    return groups


_SA2_RAD = ((0.4, 64, 80), (0.8, 128, 144))


def _bq_sa2(x, y, z, cx, cy, cz):
    B, N = x.shape
    S = cx.shape[1]
    NW = 32
    RPW = B * S // NW
    WPB = NW // B
    SPW = S // WPB
    NJ = N // 16
    mesh = plsc.VectorSubcoreMesh(core_axis_name="c", subcore_axis_name="s")
    out_type = []
    for (_, ns, kp) in _SA2_RAD:
        out_type += [jax.ShapeDtypeStruct((B * S * kp,), jnp.int32),
                     jax.ShapeDtypeStruct((B * S,), jnp.int32)]
    scratch = ([pltpu.VMEM((N,), jnp.float32)] * 3
               + [pltpu.VMEM((SPW,), jnp.float32)] * 3)
    for (_, ns, kp) in _SA2_RAD:
        scratch += [pltpu.VMEM((16 * kp,), jnp.int32),
                    pltpu.VMEM((16,), jnp.int32)]
    scratch += [pltpu.SemaphoreType.DMA]

    @functools.partial(
        pl.kernel, mesh=mesh, out_type=out_type, scratch_types=scratch,
        compiler_params=pltpu.CompilerParams(needs_layout_passes=False))
    def k(xh, yh, zh, cxh, cyh, czh, *refs):
        outs = refs[:4]
        xv, yv, zv, ccx, ccy, ccz = refs[4:10]
        bufs = refs[10:-1]
        dsem = refs[-1]
        wid = lax.axis_index("s") * 2 + lax.axis_index("c")
        b = wid // WPB
        s0 = pl.multiple_of((wid % WPB) * SPW, SPW)
        pltpu.sync_copy(xh.at[b], xv)
        pltpu.sync_copy(yh.at[b], yv)
        pltpu.sync_copy(zh.at[b], zv)
        pltpu.sync_copy(cxh.at[b, pl.ds(s0, SPW)], ccx)
        pltpu.sync_copy(cyh.at[b, pl.ds(s0, SPW)], ccy)
        pltpu.sync_copy(czh.at[b, pl.ds(s0, SPW)], ccz)
        lane = lax.iota(jnp.int32, 16)

        kp_base = [lane * kp for (_, ns, kp) in _SA2_RAD]
        zero16 = jnp.zeros((16,), jnp.int32)

        def group_body(g, _):
            g16 = pl.multiple_of(g * 16, 16)
            cxg = ccx[pl.ds(g16, 16)]
            cyg = ccy[pl.ds(g16, 16)]
            czg = ccz[pl.ds(g16, 16)]

            def pt_body(j, cnts):
                pxv = xv[pl.ds(j * 16, 16)]
                pyv = yv[pl.ds(j * 16, 16)]
                pzv = zv[pl.ds(j * 16, 16)]
                new = list(cnts)
                for l in range(16):
                    pxb = jnp.full((16,), pxv[l], jnp.float32)
                    pyb = jnp.full((16,), pyv[l], jnp.float32)
                    pzb = jnp.full((16,), pzv[l], jnp.float32)
                    dx = cxg - pxb
                    dy = cyg - pyb
                    dz = czg - pzb
                    d2 = dx * dx + dy * dy + dz * dz
                    pib = jnp.full((16,), j * 16 + l, jnp.int32)
                    for ri, (r, ns, kp) in enumerate(_SA2_RAD):
                        m = d2 <= r * r
                        tgt = kp_base[ri] + jnp.minimum(new[ri], ns)
                        plsc.store_scatter(bufs[2 * ri], [tgt], pib,
                                           mask=m)
                        new[ri] = new[ri] + m.astype(jnp.int32)
                return tuple(new)

            cnts = lax.fori_loop(0, NJ, pt_body, (zero16, zero16))

            row0 = pl.multiple_of(wid * RPW + g16, 16)
            handles = []
            for ri, (r, ns, kp) in enumerate(_SA2_RAD):
                cb = bufs[2 * ri + 1]
                cb[...] = jnp.minimum(cnts[ri], ns)
                handles.append(pltpu.async_copy(
                    bufs[2 * ri],
                    outs[2 * ri].at[
                        pl.ds(pl.multiple_of(row0 * kp, 16 * kp), 16 * kp)],
                    dsem))
                handles.append(pltpu.async_copy(
                    cb, outs[2 * ri + 1].at[pl.ds(row0, 16)], dsem))
            for h in handles:
                h.wait()
            return 0

        lax.fori_loop(0, RPW // 16, group_body, 0)

    res = k(x, y, z, cx, cy, cz)
    groups = []
    for ri, (r, ns, kp) in enumerate(_SA2_RAD):
        idx, cnt = res[2 * ri:2 * ri + 2]
        groups.append((idx.reshape(B, S, kp), cnt.reshape(B, S)))
    return groups


def _sc_gather(A2, idx, cnt, ns, Np):
    BNp, C = A2.shape
    B, S, KP = idx.shape
    NW = 32
    WPB = 4
    KPW = ns // WPB
    NJ = S // 16
    mesh = plsc.VectorSubcoreMesh(core_axis_name="c", subcore_axis_name="s")
    idx_f = idx.reshape(B, S * KP)
    out_type = jax.ShapeDtypeStruct((B, ns, S, C), jnp.float32)
    scratch = [pltpu.VMEM((S * KP,), jnp.int32),
               pltpu.VMEM((S,), jnp.int32),
               pltpu.VMEM((S,), jnp.int32),
               pltpu.VMEM((S, C), jnp.float32),
               pltpu.SemaphoreType.DMA]

    @functools.partial(
        pl.kernel, mesh=mesh, out_type=out_type, scratch_types=scratch,
        compiler_params=pltpu.CompilerParams(needs_layout_passes=False))
    def k(ah, ih, ch, oh, iv, cv, gidx, rows, sem):
        wid = lax.axis_index("s") * 2 + lax.axis_index("c")
        b = wid // WPB
        k0 = (wid % WPB) * KPW
        pltpu.sync_copy(ih.at[b], iv)
        pltpu.sync_copy(ch.at[b], cv)
        lane = lax.iota(jnp.int32, 16)
        base = b * Np

        def k_body(kk, _):
            kq = k0 + kk
            for jj in range(NJ):
                offs = (jj * 16 + lane) * KP + kq
                raw = plsc.load_gather(iv, [offs])
                cchunk = cv[pl.ds(jj * 16, 16)]
                safe = jnp.minimum(jnp.maximum(raw, 0), Np - 1)
                sel = jnp.where(kq < cchunk, safe, 0) + base
                gidx[pl.ds(jj * 16, 16)] = sel
            pltpu.async_copy(ah.at[gidx], rows, sem).wait()
            pltpu.sync_copy(rows, oh.at[b, kq])
            return 0

        lax.fori_loop(0, KPW, k_body, 0)

    return k(A2, idx_f, cnt)


def _sa2s_body(a_ref, cnt_ref, c_ref, w1xt_ref, *refs, nlayers, K):
    out_ref = refs[-1]
    corr = jnp.dot(c_ref[0], w1xt_ref[...],
                   preferred_element_type=jnp.float32)
    cntcol = cnt_ref[0]
    S = out_ref.shape[2]
    Cout = out_ref.shape[1]

    def body(kq, m):
        ak = a_ref[0, pl.ds(kq, 1)][0]
        vk = (cntcol > kq).astype(jnp.float32)
        x = jnp.maximum(ak - corr, 0.0)
        for i in range(nlayers):
            Wt = refs[2 * i][...]
            bt = refs[2 * i + 1][...]
            x = jnp.maximum(jnp.dot(x, Wt,
                                    preferred_element_type=jnp.float32)
                            + bt, 0.0)
        return jnp.maximum(m, x * vk)

    m = jax.lax.fori_loop(0, K, body, jnp.zeros((S, Cout), jnp.float32),
                          unroll=8)
    out_ref[0] = jnp.transpose(m)


def _sa2_branch_s(A4, cnt, centers_t, W1x, layers):
    B, K, S, C1 = A4.shape
    nlayers = len(layers)
    Cout = layers[-1][0].shape[0]
    wargs = []
    wspecs = []
    for (W, b) in layers:
        Wt = jnp.transpose(W)
        bt = jnp.transpose(b)
        wargs += [Wt, bt]
        wspecs += [pl.BlockSpec(Wt.shape, lambda b_: (0, 0)),
                   pl.BlockSpec(bt.shape, lambda b_: (0, 0))]
    W1xt = jnp.transpose(W1x)
    return pl.pallas_call(
        functools.partial(_sa2s_body, nlayers=nlayers, K=K),
        grid=(B,),
        in_specs=[pl.BlockSpec((1, K, S, C1), lambda b_: (b_, 0, 0, 0)),
                  pl.BlockSpec((1, S, 1), lambda b_: (b_, 0, 0)),
                  pl.BlockSpec((1, S, 3), lambda b_: (b_, 0, 0)),
                  pl.BlockSpec(W1xt.shape, lambda b_: (0, 0))] + wspecs,
        out_specs=pl.BlockSpec((1, Cout, S), lambda b_: (b_, 0, 0)),
        out_shape=jax.ShapeDtypeStruct((B, Cout, S), jnp.float32),
    )(A4, cnt, centers_t, W1xt, *wargs)


def _bq_fix_body(ox_ref, oy_ref, oz_ref, cnt_ref,
                 gx_ref, gy_ref, gz_ref, v_ref, *, K):
    S = cnt_ref.shape[2]
    cnt = cnt_ref[0]
    iota_k = lax.broadcasted_iota(jnp.int32, (K, S), 0)
    vm = iota_k < cnt
    for src, dst in ((ox_ref, gx_ref), (oy_ref, gy_ref), (oz_ref, gz_ref)):
        t = jnp.transpose(src[0])[:K]
        dst[0] = jnp.where(vm, t, 0.0)
    v_ref[0] = vm.astype(jnp.float32)


def _bq_fix(ox, oy, oz, cnt, K):
    B, S, KP = ox.shape
    ospec = pl.BlockSpec((1, S, KP), lambda b_: (b_, 0, 0))
    gspec = pl.BlockSpec((1, K, S), lambda b_: (b_, 0, 0))
    return pl.pallas_call(
        functools.partial(_bq_fix_body, K=K),
        grid=(B,),
        in_specs=[ospec] * 3 + [pl.BlockSpec((1, 1, S), lambda b_: (b_, 0, 0))],
        out_specs=[gspec] * 4,
        out_shape=[jax.ShapeDtypeStruct((B, K, S), jnp.float32)] * 4,
    )(ox, oy, oz, cnt)


def kernel(xyz, params):
    B, N, _ = xyz.shape

    sa1_layers = [_fold_layers(ls) for ls in params['sa1']]
    sa2_layers = [_fold_layers(ls) for ls in params['sa2']]
    sa3_layers = _fold_layers(params['sa3'])
    fp3_layers = _fold_layers(params['fp3'])
    fp2_layers = _fold_layers(params['fp2'])

    c1x, c1y, c1z, c2x, c2y, c2z = _fps(xyz)
    c1 = jnp.stack([c1x, c1y, c1z], axis=1)
    c2 = jnp.stack([c2x, c2y, c2z], axis=1)

    groups = _bq_sa1(xyz[:, :, 0], xyz[:, :, 1], xyz[:, :, 2],
                     c1x, c1y, c1z)
    outs1 = []
    for (ox, oy, oz, cnt), (r, ns, kp), layers in zip(groups, _SA1_RAD,
                                                      sa1_layers):
        gx, gy, gz, valid = _bq_fix(ox, oy, oz, cnt, ns)
        outs1.append(_sa1_branch(gx, gy, gz, valid, c1, layers))
    l1_points = jnp.concatenate(outs1, axis=1)

    src2 = jnp.concatenate([c1, l1_points], axis=1)
    groups2 = _bq_sa2(c1x, c1y, c1z, c2x, c2y, c2z)
    c2t = jnp.transpose(c2, (0, 2, 1))
    outs2 = []
    for (idx, cnt), (r, ns, kp), layers in zip(groups2, _SA2_RAD,
                                               sa2_layers):
        (W1, b1) = layers[0]
        A = _mlp(src2, [(W1, b1)], relus=(False,))
        A2 = jnp.transpose(A, (0, 2, 1)).reshape(B * 512, 128)
        A4 = _sc_gather(A2, idx, cnt, ns, 512)
        outs2.append(_sa2_branch_s(A4, cnt.reshape(B, 128, 1), c2t,
                                   W1[:, :3], layers[1:]))
    l2_points = jnp.concatenate(outs2, axis=1)

    g3 = jnp.concatenate([c2, l2_points], axis=1)
    l3 = _mlp(g3, sa3_layers, pool=True)

    interp3 = jnp.broadcast_to(l3, (B, 1024, 128))
    f3 = jnp.concatenate([interp3, l2_points], axis=1)
    l2f = _mlp(f3, fp3_layers)

    interp2 = _interp3(c2x, c2y, c2z, c1x, c1y, c1z, l2f)
    f2 = jnp.concatenate([interp2, l1_points], axis=1)
    l1f = _mlp(f2, fp2_layers)

    p = params['conv1']
    s1 = p['g1'] * _BN_S
    W1 = p['W1'] * s1[:, None]
    b1 = (p['b1'] * s1 + p['be1'])[:, None]
    W2 = p['W2']
    b2 = p['b2'][:, None]
    seg, obj, bck = _head(l1f, W1, b1, W2, b2)

    final_seg = _interp3(c1x, c1y, c1z,
                         xyz[:, :, 0], xyz[:, :, 1], xyz[:, :, 2],
                         seg, unit_w=True)

    return (seg, l1f, jnp.squeeze(obj, -1), jnp.squeeze(bck, -1), final_seg)


# speedup vs baseline: 30.5533x; 1.0030x over previous
import functools

import jax
import jax.numpy as jnp
import numpy as np
from jax import lax
from jax.experimental import pallas as pl
from jax.experimental.pallas import tpu as pltpu
from jax.experimental.pallas import tpu_sc as plsc

_BN_S = 1.0 / np.sqrt(1.0 + 1e-5)


def _fold_layers(layers):
    out = []
    for (W, b, g, be) in layers:
        s = g * _BN_S
        out.append((W * s[:, None], (b * s + be)[:, None]))
    return out


def _mlp_body(x_ref, *refs, nlayers, relus, pool):
    out_ref = refs[-1]
    x = x_ref[0]
    for i in range(nlayers):
        W = refs[2 * i][...]
        b = refs[2 * i + 1][...]
        x = jnp.dot(W, x, preferred_element_type=jnp.float32) + b
        if relus[i]:
            x = jnp.maximum(x, 0.0)
    if pool:
        out_ref[0] = jnp.max(x, axis=-1, keepdims=True)
    else:
        out_ref[0] = x


def _mlp(x, layers, relus=None, pool=False):
    B, Cin, S = x.shape
    nlayers = len(layers)
    if relus is None:
        relus = (True,) * nlayers
    Cout = layers[-1][0].shape[0]
    Sout = 1 if pool else S
    wargs = []
    wspecs = []
    for (W, b) in layers:
        wargs += [W, b]
        wspecs += [pl.BlockSpec(W.shape, lambda b_: (0, 0)),
                   pl.BlockSpec(b.shape, lambda b_: (0, 0))]
    return pl.pallas_call(
        functools.partial(_mlp_body, nlayers=nlayers, relus=tuple(relus),
                          pool=pool),
        grid=(B,),
        in_specs=[pl.BlockSpec((1, Cin, S), lambda b_: (b_, 0, 0))] + wspecs,
        out_specs=pl.BlockSpec((1, Cout, Sout), lambda b_: (b_, 0, 0)),
        out_shape=jax.ShapeDtypeStruct((B, Cout, Sout), jnp.float32),
    )(x, *wargs)


def _sa1_body(gx_ref, gy_ref, gz_ref, v_ref, c_ref, *refs, nlayers, K):
    out_ref = refs[-1]
    c3 = c_ref[0]
    cx, cy, cz = c3[0:1], c3[1:2], c3[2:3]
    Cout = out_ref.shape[1]
    S = out_ref.shape[2]

    def body(k, m):
        xk = gx_ref[0, pl.ds(k, 1), :]
        yk = gy_ref[0, pl.ds(k, 1), :]
        zk = gz_ref[0, pl.ds(k, 1), :]
        vk = v_ref[0, pl.ds(k, 1), :]
        x = jnp.concatenate([xk - cx, yk - cy, zk - cz, xk, yk, zk], axis=0)
        for i in range(nlayers):
            W = refs[2 * i][...]
            b = refs[2 * i + 1][...]
            x = jnp.maximum(jnp.dot(W, x, preferred_element_type=jnp.float32)
                            + b, 0.0)
        return jnp.maximum(m, x * vk)

    out_ref[0] = jax.lax.fori_loop(0, K, body, jnp.zeros((Cout, S),
                                                         jnp.float32),
                                   unroll=8)


def _sa1_branch(gx, gy, gz, valid, centers, layers):
    B, K, S = gx.shape
    nlayers = len(layers)
    Cout = layers[-1][0].shape[0]
    wargs = []
    wspecs = []
    for (W, b) in layers:
        wargs += [W, b]
        wspecs += [pl.BlockSpec(W.shape, lambda b_: (0, 0)),
                   pl.BlockSpec(b.shape, lambda b_: (0, 0))]
    gspec = pl.BlockSpec((1, K, S), lambda b_: (b_, 0, 0))
    return pl.pallas_call(
        functools.partial(_sa1_body, nlayers=nlayers, K=K),
        grid=(B,),
        in_specs=[gspec, gspec, gspec, gspec,
                  pl.BlockSpec((1, 3, S), lambda b_: (b_, 0, 0))] + wspecs,
        out_specs=pl.BlockSpec((1, Cout, S), lambda b_: (b_, 0, 0)),
        out_shape=jax.ShapeDtypeStruct((B, Cout, S), jnp.float32),
    )(gx, gy, gz, valid, centers, *wargs)


def _head_body(x_ref, w1_ref, b1_ref, w2_ref, b2_ref, seg_ref, obj_ref,
               bck_ref):
    x = x_ref[0]
    h = jnp.maximum(jnp.dot(w1_ref[...], x,
                            preferred_element_type=jnp.float32)
                    + b1_ref[...], 0.0)
    z = jnp.dot(w2_ref[...], h, preferred_element_type=jnp.float32) \
        + b2_ref[...]
    s = 1.0 / (1.0 + jnp.exp(-z))
    seg_ref[0] = s
    obj_ref[0] = jnp.max(s * x, axis=-1, keepdims=True)
    bck_ref[0] = jnp.max((1.0 - s) * x, axis=-1, keepdims=True)


def _head(x, W1, b1, W2, b2):
    B, C, S = x.shape
    return pl.pallas_call(
        _head_body,
        grid=(B,),
        in_specs=[pl.BlockSpec((1, C, S), lambda b_: (b_, 0, 0)),
                  pl.BlockSpec(W1.shape, lambda b_: (0, 0)),
                  pl.BlockSpec(b1.shape, lambda b_: (0, 0)),
                  pl.BlockSpec(W2.shape, lambda b_: (0, 0)),
                  pl.BlockSpec(b2.shape, lambda b_: (0, 0))],
        out_specs=[pl.BlockSpec((1, 1, S), lambda b_: (b_, 0, 0)),
                   pl.BlockSpec((1, C, 1), lambda b_: (b_, 0, 0)),
                   pl.BlockSpec((1, C, 1), lambda b_: (b_, 0, 0))],
        out_shape=[jax.ShapeDtypeStruct((B, 1, S), jnp.float32),
                   jax.ShapeDtypeStruct((B, C, 1), jnp.float32),
                   jax.ShapeDtypeStruct((B, C, 1), jnp.float32)],
    )(x, W1, b1, W2, b2)


def _fps_level(x, y, z, S, cx_ref, cy_ref, cz_ref):
    B, N = x.shape
    iota_n = jax.lax.broadcasted_iota(jnp.int32, (B, N), 1)
    iota_s = jax.lax.broadcasted_iota(jnp.int32, (B, S), 1)
    cx_ref[...] = jnp.zeros((B, S), jnp.float32)
    cy_ref[...] = jnp.zeros((B, S), jnp.float32)
    cz_ref[...] = jnp.zeros((B, S), jnp.float32)

    def body(t, carry):
        dist, far = carry
        sel = (iota_n == far).astype(jnp.float32)
        cx = jnp.sum(x * sel, -1, keepdims=True)
        cy = jnp.sum(y * sel, -1, keepdims=True)
        cz = jnp.sum(z * sel, -1, keepdims=True)
        oh = (iota_s == t).astype(jnp.float32)
        cx_ref[...] += cx * oh
        cy_ref[...] += cy * oh
        cz_ref[...] += cz * oh
        d = (x - cx) ** 2 + (y - cy) ** 2 + (z - cz) ** 2
        dist = jnp.minimum(dist, d)
        m = jnp.max(dist, -1, keepdims=True)
        far = jnp.min(jnp.where(dist == m, iota_n, N), -1, keepdims=True)
        return dist, far

    jax.lax.fori_loop(
        0, S, body,
        (jnp.full((B, N), 1e10, jnp.float32),
         jnp.zeros((B, 1), jnp.int32)))


def _fps_body(x_ref, y_ref, z_ref, cx_ref, cy_ref, cz_ref, *, S):
    _fps_level(x_ref[...], y_ref[...], z_ref[...], S,
               cx_ref, cy_ref, cz_ref)


def _fps_stage(x, y, z, S):
    B, N = x.shape
    full = lambda s: pl.BlockSpec(s, lambda: tuple(0 for _ in s))
    return pl.pallas_call(
        functools.partial(_fps_body, S=S),
        in_specs=[full((B, N))] * 3,
        out_specs=[full((B, S))] * 3,
        out_shape=[jax.ShapeDtypeStruct((B, S), jnp.float32)] * 3,
    )(x, y, z)


def _fps(xyz):
    x = xyz[:, :, 0]
    y = xyz[:, :, 1]
    z = xyz[:, :, 2]
    c1x, c1y, c1z = _fps_stage(x, y, z, 512)
    c2x, c2y, c2z = _fps_stage(c1x, c1y, c1z, 128)
    return c1x, c1y, c1z, c2x, c2y, c2z


def _interp3_body(kx_ref, ky_ref, kz_ref, ux_ref, uy_ref, uz_ref, f_ref,
                  out_ref, *, unit_w):
    kx = kx_ref[0]
    ky = ky_ref[0]
    kz = kz_ref[0]
    ux = ux_ref[0]
    uy = uy_ref[0]
    uz = uz_ref[0]
    Sk = kx.shape[0]
    Su = ux.shape[1]
    d2 = (kx - ux) ** 2 + (ky - uy) ** 2 + (kz - uz) ** 2
    iota_k = jax.lax.broadcasted_iota(jnp.int32, (Sk, Su), 0)
    E = jnp.zeros((Sk, Su), jnp.float32)
    ws = []
    idxs = []
    for _ in range(3):
        m = jnp.min(d2, axis=0, keepdims=True)
        i = jnp.min(jnp.where(d2 == m, iota_k, Sk), axis=0, keepdims=True)
        idxs.append(i)
        ws.append(1.0 / (jnp.sqrt(jnp.maximum(m, 0.0)) + 1e-8))
        d2 = jnp.where(iota_k == i, jnp.float32(3.0e38), d2)
    if unit_w:
        for i in idxs:
            E += (iota_k == i).astype(jnp.float32)
    else:
        wsum = ws[0] + ws[1] + ws[2]
        for w, i in zip(ws, idxs):
            E += jnp.where(iota_k == i, w / wsum, 0.0)
    out_ref[0] = jnp.dot(f_ref[0], E, preferred_element_type=jnp.float32)


def _interp3(kx, ky, kz, ux, uy, uz, feats, unit_w=False):
    B, Sk = kx.shape
    Su = ux.shape[1]
    C = feats.shape[1]
    kcol = lambda a: a.reshape(B, Sk, 1)
    urow = lambda a: a.reshape(B, 1, Su)
    return pl.pallas_call(
        functools.partial(_interp3_body, unit_w=unit_w),
        grid=(B,),
        in_specs=[pl.BlockSpec((1, Sk, 1), lambda b_: (b_, 0, 0))] * 3
        + [pl.BlockSpec((1, 1, Su), lambda b_: (b_, 0, 0))] * 3
        + [pl.BlockSpec((1, C, Sk), lambda b_: (b_, 0, 0))],
        out_specs=pl.BlockSpec((1, C, Su), lambda b_: (b_, 0, 0)),
        out_shape=jax.ShapeDtypeStruct((B, C, Su), jnp.float32),
    )(kcol(kx), kcol(ky), kcol(kz), urow(ux), urow(uy), urow(uz), feats)


_SA1_RAD = ((0.1, 32, 48), (0.2, 64, 80), (0.4, 128, 144))


def _bq_sa1(x, y, z, cx, cy, cz):
    B, N = x.shape
    S = cx.shape[1]
    NW = 32
    RPW = B * S // NW
    WPB = NW // B
    SPW = S // WPB
    NJ = N // 16
    mesh = plsc.VectorSubcoreMesh(core_axis_name="c", subcore_axis_name="s")
    out_type = []
    for (_, ns, kp) in _SA1_RAD:
        out_type += [jax.ShapeDtypeStruct((B * S * kp,), jnp.float32)] * 3
        out_type += [jax.ShapeDtypeStruct((B * S,), jnp.int32)]
    scratch = ([pltpu.VMEM((N,), jnp.float32)] * 3
               + [pltpu.VMEM((RPW,), jnp.float32)] * 3)
    for (_, ns, kp) in _SA1_RAD:
        scratch += [pltpu.VMEM((16 * kp,), jnp.int32)]
        scratch += [pltpu.VMEM((16 * kp,), jnp.float32)] * 3
        scratch += [pltpu.VMEM((16,), jnp.int32)]
    scratch += [pltpu.SemaphoreType.DMA]

    @functools.partial(
        pl.kernel, mesh=mesh, out_type=out_type, scratch_types=scratch,
        compiler_params=pltpu.CompilerParams(needs_layout_passes=False))
    def k(xh, yh, zh, cxh, cyh, czh, *refs):
        outs = refs[:12]
        xv, yv, zv, ccx, ccy, ccz = refs[12:18]
        bufs = refs[18:-1]
        dsem = refs[-1]
        wid = lax.axis_index("s") * 2 + lax.axis_index("c")
        b = wid // WPB
        s0 = pl.multiple_of((wid % WPB) * SPW, SPW)
        pltpu.sync_copy(xh.at[b], xv)
        pltpu.sync_copy(yh.at[b], yv)
        pltpu.sync_copy(zh.at[b], zv)
        pltpu.sync_copy(cxh.at[b, pl.ds(s0, SPW)], ccx)
        pltpu.sync_copy(cyh.at[b, pl.ds(s0, SPW)], ccy)
        pltpu.sync_copy(czh.at[b, pl.ds(s0, SPW)], ccz)
        lane = lax.iota(jnp.int32, 16)

        kp_base = [lane * kp for (_, ns, kp) in _SA1_RAD]
        zero16 = jnp.zeros((16,), jnp.int32)

        def group_body(g, _):
            g16 = pl.multiple_of(g * 16, 16)
            cxg = ccx[pl.ds(g16, 16)]
            cyg = ccy[pl.ds(g16, 16)]
            czg = ccz[pl.ds(g16, 16)]

            def pt_body(j, cnts):
                pxv = xv[pl.ds(j * 16, 16)]
                pyv = yv[pl.ds(j * 16, 16)]
                pzv = zv[pl.ds(j * 16, 16)]
                new = list(cnts)
                for l in range(16):
                    pxb = jnp.full((16,), pxv[l], jnp.float32)
                    pyb = jnp.full((16,), pyv[l], jnp.float32)
                    pzb = jnp.full((16,), pzv[l], jnp.float32)
                    dx = cxg - pxb
                    dy = cyg - pyb
                    dz = czg - pzb
                    d2 = dx * dx + dy * dy + dz * dz
                    pib = jnp.full((16,), j * 16 + l, jnp.int32)
                    for ri, (r, ns, kp) in enumerate(_SA1_RAD):
                        m = d2 <= r * r
                        tgt = kp_base[ri] + jnp.minimum(new[ri], ns)
                        plsc.store_scatter(bufs[5 * ri], [tgt], pib,
                                           mask=m)
                        new[ri] = new[ri] + m.astype(jnp.int32)
                return tuple(new)

            cnts = lax.fori_loop(0, NJ, pt_body, (zero16, zero16, zero16))

            row0 = pl.multiple_of(wid * RPW + g16, 16)
            handles = []
            for ri, (r, ns, kp) in enumerate(_SA1_RAD):
                ib = bufs[5 * ri]
                bx = bufs[5 * ri + 1]
                by = bufs[5 * ri + 2]
                bz = bufs[5 * ri + 3]
                cb = bufs[5 * ri + 4]

                def flush_body(w, _, ib=ib, bx=bx, by=by, bz=bz):
                    iv = ib[pl.ds(w * 16, 16)]
                    ivc = jnp.minimum(jnp.maximum(iv, 0), N - 1)
                    bx[pl.ds(w * 16, 16)] = plsc.load_gather(xv, [ivc])
                    by[pl.ds(w * 16, 16)] = plsc.load_gather(yv, [ivc])
                    bz[pl.ds(w * 16, 16)] = plsc.load_gather(zv, [ivc])
                    return 0

                lax.fori_loop(0, kp, flush_body, 0)
                cb[...] = jnp.minimum(cnts[ri], ns)
                for ch, buf in enumerate((bx, by, bz)):
                    handles.append(pltpu.async_copy(
                        buf,
                        outs[4 * ri + ch].at[
                            pl.ds(pl.multiple_of(row0 * kp, 16 * kp),
                                  16 * kp)], dsem))
                handles.append(pltpu.async_copy(
                    cb, outs[4 * ri + 3].at[pl.ds(row0, 16)], dsem))
            for h in handles:
                h.wait()
            return 0

        lax.fori_loop(0, RPW // 16, group_body, 0)

    res = k(x, y, z, cx, cy, cz)
    groups = []
    for ri, (r, ns, kp) in enumerate(_SA1_RAD):
        ox, oy, oz, cnt = res[4 * ri:4 * ri + 4]
        groups.append((ox.reshape(B, S, kp), oy.reshape(B, S, kp),
                       oz.reshape(B, S, kp), cnt.reshape(B, 1, S)))
    return groups


_SA2_RAD = ((0.4, 64, 80), (0.8, 128, 144))


def _bq_sa2(x, y, z, cx, cy, cz):
    B, N = x.shape
    S = cx.shape[1]
    NW = 32
    RPW = B * S // NW
    WPB = NW // B
    SPW = S // WPB
    NJ = N // 16
    mesh = plsc.VectorSubcoreMesh(core_axis_name="c", subcore_axis_name="s")
    out_type = []
    for (_, ns, kp) in _SA2_RAD:
        out_type += [jax.ShapeDtypeStruct((B * S * kp,), jnp.int32),
                     jax.ShapeDtypeStruct((B * S,), jnp.int32)]
    scratch = ([pltpu.VMEM((N,), jnp.float32)] * 3
               + [pltpu.VMEM((SPW,), jnp.float32)] * 3)
    for (_, ns, kp) in _SA2_RAD:
        scratch += [pltpu.VMEM((16 * kp,), jnp.int32),
                    pltpu.VMEM((16,), jnp.int32)]
    scratch += [pltpu.SemaphoreType.DMA]

    @functools.partial(
        pl.kernel, mesh=mesh, out_type=out_type, scratch_types=scratch,
        compiler_params=pltpu.CompilerParams(needs_layout_passes=False))
    def k(xh, yh, zh, cxh, cyh, czh, *refs):
        outs = refs[:4]
        xv, yv, zv, ccx, ccy, ccz = refs[4:10]
        bufs = refs[10:-1]
        dsem = refs[-1]
        wid = lax.axis_index("s") * 2 + lax.axis_index("c")
        b = wid // WPB
        s0 = pl.multiple_of((wid % WPB) * SPW, SPW)
        pltpu.sync_copy(xh.at[b], xv)
        pltpu.sync_copy(yh.at[b], yv)
        pltpu.sync_copy(zh.at[b], zv)
        pltpu.sync_copy(cxh.at[b, pl.ds(s0, SPW)], ccx)
        pltpu.sync_copy(cyh.at[b, pl.ds(s0, SPW)], ccy)
        pltpu.sync_copy(czh.at[b, pl.ds(s0, SPW)], ccz)
        lane = lax.iota(jnp.int32, 16)

        kp_base = [lane * kp for (_, ns, kp) in _SA2_RAD]
        zero16 = jnp.zeros((16,), jnp.int32)

        def group_body(g, _):
            g16 = pl.multiple_of(g * 16, 16)
            cxg = ccx[pl.ds(g16, 16)]
            cyg = ccy[pl.ds(g16, 16)]
            czg = ccz[pl.ds(g16, 16)]

            def pt_body(j, cnts):
                pxv = xv[pl.ds(j * 16, 16)]
                pyv = yv[pl.ds(j * 16, 16)]
                pzv = zv[pl.ds(j * 16, 16)]
                new = list(cnts)
                for l in range(16):
                    pxb = jnp.full((16,), pxv[l], jnp.float32)
                    pyb = jnp.full((16,), pyv[l], jnp.float32)
                    pzb = jnp.full((16,), pzv[l], jnp.float32)
                    dx = cxg - pxb
                    dy = cyg - pyb
                    dz = czg - pzb
                    d2 = dx * dx + dy * dy + dz * dz
                    pib = jnp.full((16,), j * 16 + l, jnp.int32)
                    for ri, (r, ns, kp) in enumerate(_SA2_RAD):
                        m = d2 <= r * r
                        tgt = kp_base[ri] + jnp.minimum(new[ri], ns)
                        plsc.store_scatter(bufs[2 * ri], [tgt], pib,
                                           mask=m)
                        new[ri] = new[ri] + m.astype(jnp.int32)
                return tuple(new)

            cnts = lax.fori_loop(0, NJ, pt_body, (zero16, zero16))

            row0 = pl.multiple_of(wid * RPW + g16, 16)
            handles = []
            for ri, (r, ns, kp) in enumerate(_SA2_RAD):
                cb = bufs[2 * ri + 1]
                cb[...] = jnp.minimum(cnts[ri], ns)
                handles.append(pltpu.async_copy(
                    bufs[2 * ri],
                    outs[2 * ri].at[
                        pl.ds(pl.multiple_of(row0 * kp, 16 * kp), 16 * kp)],
                    dsem))
                handles.append(pltpu.async_copy(
                    cb, outs[2 * ri + 1].at[pl.ds(row0, 16)], dsem))
            for h in handles:
                h.wait()
            return 0

        lax.fori_loop(0, RPW // 16, group_body, 0)

    res = k(x, y, z, cx, cy, cz)
    groups = []
    for ri, (r, ns, kp) in enumerate(_SA2_RAD):
        idx, cnt = res[2 * ri:2 * ri + 2]
        groups.append((idx.reshape(B, S, kp), cnt.reshape(B, S)))
    return groups


def _sc_gather(A2, idx, cnt, ns, Np):
    BNp, C = A2.shape
    B, S, KP = idx.shape
    NW = 32
    WPB = 4
    KPW = ns // WPB
    NJ = S // 16
    mesh = plsc.VectorSubcoreMesh(core_axis_name="c", subcore_axis_name="s")
    idx_f = idx.reshape(B, S * KP)
    out_type = jax.ShapeDtypeStruct((B, ns, S, C), jnp.float32)
    scratch = [pltpu.VMEM((S * KP,), jnp.int32),
               pltpu.VMEM((S,), jnp.int32),
               pltpu.VMEM((S,), jnp.int32),
               pltpu.VMEM((S,), jnp.int32),
               pltpu.VMEM((S, C), jnp.float32),
               pltpu.VMEM((S, C), jnp.float32),
               pltpu.SemaphoreType.DMA,
               pltpu.SemaphoreType.DMA,
               pltpu.SemaphoreType.DMA]

    @functools.partial(
        pl.kernel, mesh=mesh, out_type=out_type, scratch_types=scratch,
        compiler_params=pltpu.CompilerParams(needs_layout_passes=False))
    def k(ah, ih, ch, oh, iv, cv, g0, g1, r0, r1, sg, so0, so1):
        wid = lax.axis_index("s") * 2 + lax.axis_index("c")
        b = wid // WPB
        k0 = (wid % WPB) * KPW
        pltpu.sync_copy(ih.at[b], iv)
        pltpu.sync_copy(ch.at[b], cv)
        lane = lax.iota(jnp.int32, 16)
        base = b * Np

        bufs2 = ((g0, r0, so0), (g1, r1, so1))
        pending = [None, None]
        for kk in range(KPW):
            kq = k0 + kk
            gb, rb, so = bufs2[kk % 2]
            if pending[kk % 2] is not None:
                pending[kk % 2].wait()
            for jj in range(NJ):
                offs = (jj * 16 + lane) * KP + kq
                raw = plsc.load_gather(iv, [offs])
                cchunk = cv[pl.ds(jj * 16, 16)]
                safe = jnp.minimum(jnp.maximum(raw, 0), Np - 1)
                sel = jnp.where(kq < cchunk, safe, 0) + base
                gb[pl.ds(jj * 16, 16)] = sel
            pltpu.async_copy(ah.at[gb], rb, sg).wait()
            pending[kk % 2] = pltpu.async_copy(rb, oh.at[b, kq], so)
        for h in pending:
            if h is not None:
                h.wait()

    return k(A2, idx_f, cnt)


def _sa2s_body(a_ref, cnt_ref, c_ref, w1xt_ref, *refs, nlayers, K):
    out_ref = refs[-1]
    corr = jnp.dot(c_ref[0], w1xt_ref[...],
                   preferred_element_type=jnp.float32)
    cntcol = cnt_ref[0]
    S = out_ref.shape[2]
    Cout = out_ref.shape[1]

    def body(kq, m):
        ak = a_ref[0, pl.ds(kq, 1)][0]
        vk = (cntcol > kq).astype(jnp.float32)
        x = jnp.maximum(ak - corr, 0.0)
        for i in range(nlayers):
            Wt = refs[2 * i][...]
            bt = refs[2 * i + 1][...]
            x = jnp.maximum(jnp.dot(x, Wt,
                                    preferred_element_type=jnp.float32)
                            + bt, 0.0)
        return jnp.maximum(m, x * vk)

    m = jax.lax.fori_loop(0, K, body, jnp.zeros((S, Cout), jnp.float32),
                          unroll=8)
    out_ref[0] = jnp.transpose(m)


def _sa2_branch_s(A4, cnt, centers_t, W1x, layers):
    B, K, S, C1 = A4.shape
    nlayers = len(layers)
    Cout = layers[-1][0].shape[0]
    wargs = []
    wspecs = []
    for (W, b) in layers:
        Wt = jnp.transpose(W)
        bt = jnp.transpose(b)
        wargs += [Wt, bt]
        wspecs += [pl.BlockSpec(Wt.shape, lambda b_: (0, 0)),
                   pl.BlockSpec(bt.shape, lambda b_: (0, 0))]
    W1xt = jnp.transpose(W1x)
    return pl.pallas_call(
        functools.partial(_sa2s_body, nlayers=nlayers, K=K),
        grid=(B,),
        in_specs=[pl.BlockSpec((1, K, S, C1), lambda b_: (b_, 0, 0, 0)),
                  pl.BlockSpec((1, S, 1), lambda b_: (b_, 0, 0)),
                  pl.BlockSpec((1, S, 3), lambda b_: (b_, 0, 0)),
                  pl.BlockSpec(W1xt.shape, lambda b_: (0, 0))] + wspecs,
        out_specs=pl.BlockSpec((1, Cout, S), lambda b_: (b_, 0, 0)),
        out_shape=jax.ShapeDtypeStruct((B, Cout, S), jnp.float32),
    )(A4, cnt, centers_t, W1xt, *wargs)


def _bq_fix_body(ox_ref, oy_ref, oz_ref, cnt_ref,
                 gx_ref, gy_ref, gz_ref, v_ref, *, K):
    S = cnt_ref.shape[2]
    cnt = cnt_ref[0]
    iota_k = lax.broadcasted_iota(jnp.int32, (K, S), 0)
    vm = iota_k < cnt
    for src, dst in ((ox_ref, gx_ref), (oy_ref, gy_ref), (oz_ref, gz_ref)):
        t = jnp.transpose(src[0])[:K]
        dst[0] = jnp.where(vm, t, 0.0)
    v_ref[0] = vm.astype(jnp.float32)


def _bq_fix(ox, oy, oz, cnt, K):
    B, S, KP = ox.shape
    ospec = pl.BlockSpec((1, S, KP), lambda b_: (b_, 0, 0))
    gspec = pl.BlockSpec((1, K, S), lambda b_: (b_, 0, 0))
    return pl.pallas_call(
        functools.partial(_bq_fix_body, K=K),
        grid=(B,),
        in_specs=[ospec] * 3 + [pl.BlockSpec((1, 1, S), lambda b_: (b_, 0, 0))],
        out_specs=[gspec] * 4,
        out_shape=[jax.ShapeDtypeStruct((B, K, S), jnp.float32)] * 4,
    )(ox, oy, oz, cnt)


def kernel(xyz, params):
    B, N, _ = xyz.shape

    sa1_layers = [_fold_layers(ls) for ls in params['sa1']]
    sa2_layers = [_fold_layers(ls) for ls in params['sa2']]
    sa3_layers = _fold_layers(params['sa3'])
    fp3_layers = _fold_layers(params['fp3'])
    fp2_layers = _fold_layers(params['fp2'])

    c1x, c1y, c1z, c2x, c2y, c2z = _fps(xyz)
    c1 = jnp.stack([c1x, c1y, c1z], axis=1)
    c2 = jnp.stack([c2x, c2y, c2z], axis=1)

    groups = _bq_sa1(xyz[:, :, 0], xyz[:, :, 1], xyz[:, :, 2],
                     c1x, c1y, c1z)
    outs1 = []
    for (ox, oy, oz, cnt), (r, ns, kp), layers in zip(groups, _SA1_RAD,
                                                      sa1_layers):
        gx, gy, gz, valid = _bq_fix(ox, oy, oz, cnt, ns)
        outs1.append(_sa1_branch(gx, gy, gz, valid, c1, layers))
    l1_points = jnp.concatenate(outs1, axis=1)

    src2 = jnp.concatenate([c1, l1_points], axis=1)
    groups2 = _bq_sa2(c1x, c1y, c1z, c2x, c2y, c2z)
    c2t = jnp.transpose(c2, (0, 2, 1))
    outs2 = []
    for (idx, cnt), (r, ns, kp), layers in zip(groups2, _SA2_RAD,
                                               sa2_layers):
        (W1, b1) = layers[0]
        A = _mlp(src2, [(W1, b1)], relus=(False,))
        A2 = jnp.transpose(A, (0, 2, 1)).reshape(B * 512, 128)
        A4 = _sc_gather(A2, idx, cnt, ns, 512)
        outs2.append(_sa2_branch_s(A4, cnt.reshape(B, 128, 1), c2t,
                                   W1[:, :3], layers[1:]))
    l2_points = jnp.concatenate(outs2, axis=1)

    g3 = jnp.concatenate([c2, l2_points], axis=1)
    l3 = _mlp(g3, sa3_layers, pool=True)

    interp3 = jnp.broadcast_to(l3, (B, 1024, 128))
    f3 = jnp.concatenate([interp3, l2_points], axis=1)
    l2f = _mlp(f3, fp3_layers)

    interp2 = _interp3(c2x, c2y, c2z, c1x, c1y, c1z, l2f)
    f2 = jnp.concatenate([interp2, l1_points], axis=1)
    l1f = _mlp(f2, fp2_layers)

    p = params['conv1']
    s1 = p['g1'] * _BN_S
    W1 = p['W1'] * s1[:, None]
    b1 = (p['b1'] * s1 + p['be1'])[:, None]
    W2 = p['W2']
    b2 = p['b2'][:, None]
    seg, obj, bck = _head(l1f, W1, b1, W2, b2)

    final_seg = _interp3(c1x, c1y, c1z,
                         xyz[:, :, 0], xyz[:, :, 1], xyz[:, :, 2],
                         seg, unit_w=True)

    return (seg, l1f, jnp.squeeze(obj, -1), jnp.squeeze(bck, -1), final_seg)
